# Initial kernel scaffold; baseline (speedup 1.0000x reference)
import jax
import jax.numpy as jnp
from jax.experimental import pallas as pl


def kernel(x, edge_index, W_emb, b_emb, W1, b1, W2, b2, Wg, a_src, a_dst, bg):
    raise NotImplementedError("write your pallas kernel here")


# speedup vs baseline: 39.5091x; 39.5091x over previous
import functools

import jax
import jax.numpy as jnp
from jax import lax
from jax.experimental import pallas as pl
from jax.experimental.pallas import tpu as pltpu
from jax.experimental.pallas import tpu_sc as plsc

N = 10000
D = 128
NPAD = 10240
NC = 2
NS = 16
L = 16
EB = 128
NB = 82
EPW = NB * EB
ETOT = NC * NS * EPW
RPT = NPAD // NS
BLK = 1024
GRID = NPAD // BLK
NEG = -1e30

_MESH = plsc.VectorSubcoreMesh(
    core_axis_name="c", subcore_axis_name="s", num_cores=NC, num_subcores=NS)


def _fill(ref, n, value):
  vec = jnp.full((L,), value, ref.dtype)
  def body(i, _):
    ref[pl.ds(i * L, L)] = vec
    return 0
  lax.fori_loop(0, n // L, body, 0)


def _zero_shared_slice(acc_sh, zbuf, s):
  nz = zbuf.shape[0]
  for i in range(RPT // nz):
    pltpu.sync_copy(zbuf, acc_sh.at[pl.ds(s * RPT + i * nz, nz)])


def _copyout_shared_slice(acc_sh, out_hbm, bounce, c, s):
  nz = bounce.shape[0]
  for i in range(RPT // nz):
    sl = pl.ds(s * RPT + i * nz, nz)
    pltpu.sync_copy(acc_sh.at[sl], bounce)
    pltpu.sync_copy(bounce, out_hbm.at[c, sl])


@functools.partial(
    pl.kernel,
    out_type=jax.ShapeDtypeStruct((NC, NPAD), jnp.float32),
    mesh=_MESH,
    scratch_types=[
        pltpu.VMEM((NB, EB), jnp.int32),
        pltpu.VMEM((EB,), jnp.float32),
        pltpu.VMEM((RPT,), jnp.float32),
        pltpu.VMEM_SHARED((NPAD,), jnp.float32),
    ],
)
def _deg_sc(dst_hbm, out_hbm, idx_v, ones_v, row_v, acc_sh):
  c = lax.axis_index("c")
  s = lax.axis_index("s")
  _fill(ones_v, EB, 1.0)
  _fill(row_v, RPT, 0.0)
  pltpu.sync_copy(row_v, acc_sh.at[pl.ds(s * RPT, RPT)])
  pltpu.sync_copy(dst_hbm.at[c, s], idx_v)
  plsc.subcore_barrier()

  def body(j, _):
    pltpu.sync_copy(ones_v, acc_sh.at[idx_v.at[j]], add=True)
    return 0
  lax.fori_loop(0, NB, body, 0)

  plsc.subcore_barrier()
  sl = pl.ds(s * RPT, RPT)
  pltpu.sync_copy(acc_sh.at[sl], row_v)
  pltpu.sync_copy(row_v, out_hbm.at[c, sl])


@functools.partial(
    pl.kernel,
    out_type=jax.ShapeDtypeStruct((NC, NPAD, D), jnp.float32),
    mesh=_MESH,
    scratch_types=[
        pltpu.VMEM((2, EB), jnp.int32),
        pltpu.VMEM((2, EB), jnp.int32),
        pltpu.VMEM((EB, D), jnp.float32),
        pltpu.VMEM((EB, D), jnp.float32),
        pltpu.VMEM_SHARED((NPAD, D), jnp.float32),
        pltpu.SemaphoreType.DMA,
        pltpu.SemaphoreType.DMA,
        pltpu.SemaphoreType.DMA,
    ],
)
def _spmm_sc(p_hbm, src_hbm, dst_hbm, out_hbm,
             srcb, dstb, bufa, bufb, acc_sh, sga, sgb, ssc):
  c = lax.axis_index("c")
  s = lax.axis_index("s")
  zvec = jnp.zeros((L,), jnp.float32)
  def zrow(r, _):
    for k in range(D // L):
      bufa[r, pl.ds(k * L, L)] = zvec
    return 0
  lax.fori_loop(0, EB, zrow, 0)
  _zero_shared_slice(acc_sh, bufa, s)
  plsc.subcore_barrier()

  pltpu.sync_copy(src_hbm.at[c, s, 0], srcb.at[0])
  pltpu.sync_copy(dst_hbm.at[c, s, 0], dstb.at[0])
  pltpu.async_copy(p_hbm.at[srcb.at[0]], bufa, sga)

  def body(j, _):
    ja = 2 * j
    jb = 2 * j + 1
    pltpu.sync_copy(src_hbm.at[c, s, jb], srcb.at[1])
    pltpu.sync_copy(dst_hbm.at[c, s, jb], dstb.at[1])
    gb = pltpu.async_copy(p_hbm.at[srcb.at[1]], bufb, sgb)
    pltpu.make_async_copy(p_hbm.at[srcb.at[0]], bufa, sga).wait()
    pltpu.async_copy(bufa, acc_sh.at[dstb.at[0]], ssc, add=True).wait()

    @pl.when(jb + 1 < NB)
    def _():
      pltpu.sync_copy(src_hbm.at[c, s, jb + 1], srcb.at[0])
      pltpu.sync_copy(dst_hbm.at[c, s, jb + 1], dstb.at[0])
      pltpu.async_copy(p_hbm.at[srcb.at[0]], bufa, sga)
    gb.wait()
    pltpu.async_copy(bufb, acc_sh.at[dstb.at[1]], ssc, add=True).wait()
    return 0

  lax.fori_loop(0, NB // 2, body, 0)

  plsc.subcore_barrier()
  _copyout_shared_slice(acc_sh, out_hbm, bufa, c, s)


def _edge_e(s_v, d_v, src_v, dst_v, j, k):
  si = src_v[j, pl.ds(k * L, L)]
  di = dst_v[j, pl.ds(k * L, L)]
  sv = plsc.load_gather(s_v, [si])
  dv = plsc.load_gather(d_v, [di])
  z = sv + dv
  return di, jnp.where(z >= 0, z, 0.2 * z)


@functools.partial(
    pl.kernel,
    out_type=jax.ShapeDtypeStruct((NC, NPAD), jnp.float32),
    mesh=_MESH,
    scratch_types=[
        pltpu.VMEM((NPAD,), jnp.float32),
        pltpu.VMEM((NPAD,), jnp.float32),
        pltpu.VMEM((NPAD,), jnp.float32),
        pltpu.VMEM((NB, EB), jnp.int32),
        pltpu.VMEM((NB, EB), jnp.int32),
        pltpu.VMEM((RPT,), jnp.float32),
        pltpu.VMEM((RPT,), jnp.float32),
        pltpu.VMEM_SHARED((NS, NPAD), jnp.float32),
    ],
    compiler_params=pltpu.CompilerParams(needs_layout_passes=False),
)
def _gat_max_sc(s_hbm, d_hbm, src_hbm, dst_hbm, out_hbm,
                s_v, d_v, m_v, src_v, dst_v, acc_v, tmp_v, stage_sh):
  c = lax.axis_index("c")
  s = lax.axis_index("s")
  pltpu.sync_copy(s_hbm, s_v)
  pltpu.sync_copy(d_hbm, d_v)
  pltpu.sync_copy(src_hbm.at[c, s], src_v)
  pltpu.sync_copy(dst_hbm.at[c, s], dst_v)
  _fill(m_v, NPAD, NEG)

  def batch(j, _):
    for k in range(EB // L):
      di, e = _edge_e(s_v, d_v, src_v, dst_v, j, k)
      cur = plsc.load_gather(m_v, [di])
      need = e > cur

      def w_body(need):
        plsc.store_scatter(m_v, [di], e, mask=need)
        cur = plsc.load_gather(m_v, [di])
        return e > cur

      lax.while_loop(jnp.any, w_body, need)
    return 0

  lax.fori_loop(0, NB, batch, 0)

  pltpu.sync_copy(m_v, stage_sh.at[s])
  plsc.subcore_barrier()
  sl = pl.ds(s * RPT, RPT)
  pltpu.sync_copy(stage_sh.at[0, sl], acc_v)
  for i in range(1, NS):
    pltpu.sync_copy(stage_sh.at[i, sl], tmp_v)
    def red(t, _):
      ds = pl.ds(t * L, L)
      acc_v[ds] = jnp.maximum(acc_v[ds], tmp_v[ds])
      return 0
    lax.fori_loop(0, RPT // L, red, 0)
  pltpu.sync_copy(acc_v, out_hbm.at[c, sl])


@functools.partial(
    pl.kernel,
    out_type=jax.ShapeDtypeStruct((NC, NPAD), jnp.float32),
    mesh=_MESH,
    scratch_types=[
        pltpu.VMEM((NPAD,), jnp.float32),
        pltpu.VMEM((NPAD,), jnp.float32),
        pltpu.VMEM((NPAD,), jnp.float32),
        pltpu.VMEM((NPAD,), jnp.float32),
        pltpu.VMEM((NB, EB), jnp.int32),
        pltpu.VMEM((NB, EB), jnp.int32),
        pltpu.VMEM((EB,), jnp.float32),
        pltpu.VMEM((RPT,), jnp.float32),
        pltpu.VMEM_SHARED((NPAD,), jnp.float32),
    ],
    compiler_params=pltpu.CompilerParams(needs_layout_passes=False),
)
def _gat_denom_sc(s_hbm, d_hbm, m_hbm, src_hbm, dst_hbm, out_hbm,
                  s_v, d_v, m_v, t_v, src_v, dst_v, val_v, row_v, acc_sh):
  c = lax.axis_index("c")
  s = lax.axis_index("s")
  pltpu.sync_copy(s_hbm, s_v)
  pltpu.sync_copy(d_hbm, d_v)
  pltpu.sync_copy(m_hbm.at[0], m_v)
  pltpu.sync_copy(m_hbm.at[1], t_v)
  def mx(t, _):
    ds = pl.ds(t * L, L)
    m_v[ds] = jnp.maximum(m_v[ds], t_v[ds])
    return 0
  lax.fori_loop(0, NPAD // L, mx, 0)
  pltpu.sync_copy(src_hbm.at[c, s], src_v)
  pltpu.sync_copy(dst_hbm.at[c, s], dst_v)
  _fill(row_v, RPT, 0.0)
  pltpu.sync_copy(row_v, acc_sh.at[pl.ds(s * RPT, RPT)])
  plsc.subcore_barrier()

  def batch(j, _):
    for k in range(EB // L):
      di, e = _edge_e(s_v, d_v, src_v, dst_v, j, k)
      mv = plsc.load_gather(m_v, [di])
      val_v[pl.ds(k * L, L)] = jnp.exp(e - mv)
    pltpu.sync_copy(val_v, acc_sh.at[dst_v.at[j]], add=True)
    return 0

  lax.fori_loop(0, NB, batch, 0)

  plsc.subcore_barrier()
  sl = pl.ds(s * RPT, RPT)
  pltpu.sync_copy(acc_sh.at[sl], row_v)
  pltpu.sync_copy(row_v, out_hbm.at[c, sl])


@functools.partial(
    pl.kernel,
    out_type=jax.ShapeDtypeStruct((NC, NPAD), jnp.float32),
    mesh=_MESH,
    scratch_types=[
        pltpu.VMEM((NPAD,), jnp.float32),
        pltpu.VMEM((NPAD,), jnp.float32),
        pltpu.VMEM((NPAD,), jnp.float32),
        pltpu.VMEM((NPAD,), jnp.float32),
        pltpu.VMEM((NPAD,), jnp.float32),
        pltpu.VMEM((NB, EB), jnp.int32),
        pltpu.VMEM((NB, EB), jnp.int32),
        pltpu.VMEM((EB,), jnp.float32),
        pltpu.VMEM((RPT,), jnp.float32),
        pltpu.VMEM_SHARED((NPAD,), jnp.float32),
    ],
    compiler_params=pltpu.CompilerParams(needs_layout_passes=False),
)
def _gat_w_sc(s_hbm, d_hbm, m_hbm, den_hbm, src_hbm, dst_hbm, out_hbm,
              s_v, d_v, m_v, den_v, t_v, src_v, dst_v, val_v, row_v, acc_sh):
  c = lax.axis_index("c")
  s = lax.axis_index("s")
  pltpu.sync_copy(s_hbm, s_v)
  pltpu.sync_copy(d_hbm, d_v)
  pltpu.sync_copy(m_hbm.at[0], m_v)
  pltpu.sync_copy(m_hbm.at[1], t_v)
  def mx(t, _):
    ds = pl.ds(t * L, L)
    m_v[ds] = jnp.maximum(m_v[ds], t_v[ds])
    return 0
  lax.fori_loop(0, NPAD // L, mx, 0)
  pltpu.sync_copy(den_hbm.at[0], den_v)
  pltpu.sync_copy(den_hbm.at[1], t_v)
  def ad(t, _):
    ds = pl.ds(t * L, L)
    den_v[ds] = den_v[ds] + t_v[ds] + 1e-16
    return 0
  lax.fori_loop(0, NPAD // L, ad, 0)
  pltpu.sync_copy(src_hbm.at[c, s], src_v)
  pltpu.sync_copy(dst_hbm.at[c, s], dst_v)
  _fill(row_v, RPT, 0.0)
  pltpu.sync_copy(row_v, acc_sh.at[pl.ds(s * RPT, RPT)])
  plsc.subcore_barrier()

  def batch(j, _):
    for k in range(EB // L):
      di, e = _edge_e(s_v, d_v, src_v, dst_v, j, k)
      mv = plsc.load_gather(m_v, [di])
      dv = plsc.load_gather(den_v, [di])
      val_v[pl.ds(k * L, L)] = jnp.exp(e - mv) / dv
    pltpu.sync_copy(val_v, acc_sh.at[src_v.at[j]], add=True)
    return 0

  lax.fori_loop(0, NB, batch, 0)

  plsc.subcore_barrier()
  sl = pl.ds(s * RPT, RPT)
  pltpu.sync_copy(acc_sh.at[sl], row_v)
  pltpu.sync_copy(row_v, out_hbm.at[c, sl])


def _dis_of(deg2):
  deg = deg2[0] + deg2[1]
  return jnp.where(deg > 0, lax.rsqrt(jnp.maximum(deg, 1e-12)), 0.0)


def _t0_body(x_ref, w_ref, b_ref, o_ref):
  o_ref[...] = jax.nn.relu(
      jnp.dot(x_ref[...], w_ref[...], preferred_element_type=jnp.float32)
      + b_ref[...])


def _t1_body(h_ref, w_ref, deg_ref, o_ref):
  dis = _dis_of(deg_ref[...])
  o_ref[...] = dis[:, None] * jnp.dot(
      h_ref[...], w_ref[...], preferred_element_type=jnp.float32)


def _t2_body(q_ref, deg_ref, b_ref, w_ref, o_ref):
  dis = _dis_of(deg_ref[...])
  h = jax.nn.relu(dis[:, None] * (q_ref[0] + q_ref[1]) + b_ref[...])
  o_ref[...] = dis[:, None] * jnp.dot(
      h, w_ref[...], preferred_element_type=jnp.float32)


def _t3_body(q_ref, deg_ref, b_ref, wg_ref, asrc_ref, adst_ref,
             hg_ref, s_ref, d_ref):
  dis = _dis_of(deg_ref[...])
  h = jax.nn.relu(dis[:, None] * (q_ref[0] + q_ref[1]) + b_ref[...])
  hg = jnp.dot(h, wg_ref[...], preferred_element_type=jnp.float32)
  hg_ref[...] = hg
  s_ref[...] = jnp.dot(hg, asrc_ref[...],
                       preferred_element_type=jnp.float32).T
  d_ref[...] = jnp.dot(hg, adst_ref[...],
                       preferred_element_type=jnp.float32).T


def _t4_body(w_ref, hg_ref, bg_ref, o_ref):
  i = pl.program_id(0)
  rows = i * BLK + lax.broadcasted_iota(jnp.int32, (BLK,), 0)
  w = jnp.where(rows < N, w_ref[0] + w_ref[1], 0.0)
  part = jnp.dot(w[None, :], hg_ref[...], preferred_element_type=jnp.float32)

  @pl.when(i == 0)
  def _():
    o_ref[...] = jnp.zeros_like(o_ref)
  o_ref[...] += part

  @pl.when(i == GRID - 1)
  def _():
    o_ref[...] = o_ref[...] * (1.0 / N) + bg_ref[...]


def _row_spec(block=None):
  return pl.BlockSpec((BLK, D) if block is None else block, lambda i: (0, 0))


_SPEC_ROWS = pl.BlockSpec((BLK, D), lambda i: (i, 0))
_SPEC_W = pl.BlockSpec((D, D), lambda i: (0, 0))
_SPEC_B = pl.BlockSpec((1, D), lambda i: (0, 0))
_SPEC_DEG = pl.BlockSpec((NC, BLK), lambda i: (0, i))
_SPEC_Q = pl.BlockSpec((NC, BLK, D), lambda i: (0, i, 0))
_SPEC_VEC = pl.BlockSpec((D, 1), lambda i: (0, 0))
_SPEC_SD = pl.BlockSpec((1, BLK), lambda i: (0, i))


def _t0(xp, W_emb, b_emb):
  return pl.pallas_call(
      _t0_body, grid=(GRID,),
      in_specs=[_SPEC_ROWS, _SPEC_W, _SPEC_B],
      out_specs=_SPEC_ROWS,
      out_shape=jax.ShapeDtypeStruct((NPAD, D), jnp.float32),
  )(xp, W_emb, b_emb)


def _t1(h0, W1, deg2):
  return pl.pallas_call(
      _t1_body, grid=(GRID,),
      in_specs=[_SPEC_ROWS, _SPEC_W, _SPEC_DEG],
      out_specs=_SPEC_ROWS,
      out_shape=jax.ShapeDtypeStruct((NPAD, D), jnp.float32),
  )(h0, W1, deg2)


def _t2(q2, deg2, b, W):
  return pl.pallas_call(
      _t2_body, grid=(GRID,),
      in_specs=[_SPEC_Q, _SPEC_DEG, _SPEC_B, _SPEC_W],
      out_specs=_SPEC_ROWS,
      out_shape=jax.ShapeDtypeStruct((NPAD, D), jnp.float32),
  )(q2, deg2, b, W)


def _t3(q2, deg2, b, Wg, a_src, a_dst):
  return pl.pallas_call(
      _t3_body, grid=(GRID,),
      in_specs=[_SPEC_Q, _SPEC_DEG, _SPEC_B, _SPEC_W, _SPEC_VEC, _SPEC_VEC],
      out_specs=[_SPEC_ROWS, _SPEC_SD, _SPEC_SD],
      out_shape=[
          jax.ShapeDtypeStruct((NPAD, D), jnp.float32),
          jax.ShapeDtypeStruct((1, NPAD), jnp.float32),
          jax.ShapeDtypeStruct((1, NPAD), jnp.float32),
      ],
  )(q2, deg2, b, Wg, a_src, a_dst)


def _t4(w2, hg, bg):
  return pl.pallas_call(
      _t4_body, grid=(GRID,),
      in_specs=[_SPEC_DEG, _SPEC_ROWS, _SPEC_B],
      out_specs=_SPEC_B,
      out_shape=jax.ShapeDtypeStruct((1, D), jnp.float32),
  )(w2, hg, bg)


def kernel(x, edge_index, W_emb, b_emb, W1, b1, W2, b2, Wg, a_src, a_dst, bg):
  ei = edge_index.astype(jnp.int32)
  loop = jnp.arange(N, dtype=jnp.int32)
  npad_e = ETOT - (ei.shape[1] + N)
  padidx = N + 200 + (jnp.arange(npad_e, dtype=jnp.int32) % 8)
  src4 = jnp.concatenate([ei[0], loop, padidx]).reshape(NC, NS, NB, EB)
  dst4 = jnp.concatenate([ei[1], loop, padidx]).reshape(NC, NS, NB, EB)
  xp = jnp.zeros((NPAD, D), jnp.float32).at[:N].set(x)
  b_emb2 = b_emb.reshape(1, D)
  b12 = b1.reshape(1, D)
  b22 = b2.reshape(1, D)
  bg2 = bg.reshape(1, D)
  a_src2 = a_src.reshape(D, 1)
  a_dst2 = a_dst.reshape(D, 1)

  deg2 = _deg_sc(dst4)
  h0 = _t0(xp, W_emb, b_emb2)
  p1 = _t1(h0, W1, deg2)
  q1 = _spmm_sc(p1, src4, dst4)
  p2 = _t2(q1, deg2, b12, W2)
  q2 = _spmm_sc(p2, src4, dst4)
  hg, s2, d2 = _t3(q2, deg2, b22, Wg, a_src2, a_dst2)
  s1 = s2.reshape(NPAD)
  d1 = d2.reshape(NPAD)
  m2 = _gat_max_sc(s1, d1, src4, dst4)
  den2 = _gat_denom_sc(s1, d1, m2, src4, dst4)
  w2 = _gat_w_sc(s1, d1, m2, den2, src4, dst4)
  return _t4(w2, hg, bg2)


# speedup vs baseline: 41.2780x; 1.0448x over previous
import functools

import jax
import jax.numpy as jnp
from jax import lax
from jax.experimental import pallas as pl
from jax.experimental.pallas import tpu as pltpu
from jax.experimental.pallas import tpu_sc as plsc

N = 10000
D = 128
NPAD = 10240
NC = 2
NS = 16
L = 16
EB = 128
NB = 82
NCH = NB // 2
EPW = NB * EB
ETOT = NC * NS * EPW
RPT = NPAD // NS
BLK = 1024
GRID = NPAD // BLK
NEG = -1e30

_MESH = plsc.VectorSubcoreMesh(
    core_axis_name="c", subcore_axis_name="s", num_cores=NC, num_subcores=NS)


def _fill(ref, n, value):
  vec = jnp.full((L,), value, ref.dtype)
  def body(i, _):
    ref[pl.ds(i * L, L)] = vec
    return 0
  lax.fori_loop(0, n // L, body, 0)


def _zero_shared_slice(acc_sh, zbuf, s):
  nz = zbuf.shape[0]
  for i in range(RPT // nz):
    pltpu.sync_copy(zbuf, acc_sh.at[pl.ds(s * RPT + i * nz, nz)])


def _copyout_shared_slice(acc_sh, out_hbm, bounce, c, s):
  nz = bounce.shape[0]
  for i in range(RPT // nz):
    sl = pl.ds(s * RPT + i * nz, nz)
    pltpu.sync_copy(acc_sh.at[sl], bounce)
    pltpu.sync_copy(bounce, out_hbm.at[c, sl])


@functools.partial(
    pl.kernel,
    out_type=jax.ShapeDtypeStruct((NC, NPAD), jnp.float32),
    mesh=_MESH,
    scratch_types=[
        pltpu.VMEM((NB, EB), jnp.int32),
        pltpu.VMEM((EB,), jnp.float32),
        pltpu.VMEM((RPT,), jnp.float32),
        pltpu.VMEM_SHARED((NPAD,), jnp.float32),
        pltpu.SemaphoreType.DMA,
    ],
)
def _deg_sc(dst_hbm, out_hbm, idx_v, ones_v, row_v, acc_sh, sadd):
  c = lax.axis_index("c")
  s = lax.axis_index("s")
  _fill(ones_v, EB, 1.0)
  _fill(row_v, RPT, 0.0)
  pltpu.sync_copy(row_v, acc_sh.at[pl.ds(s * RPT, RPT)])
  pltpu.sync_copy(dst_hbm.at[c, s], idx_v)
  plsc.subcore_barrier()

  def body(j, _):
    pltpu.async_copy(ones_v, acc_sh.at[idx_v.at[j]], sadd, add=True)
    return 0
  lax.fori_loop(0, NB, body, 0)

  def drain(j, _):
    pltpu.make_async_copy(ones_v, acc_sh.at[idx_v.at[j]], sadd).wait()
    return 0
  lax.fori_loop(0, NB, drain, 0)

  plsc.subcore_barrier()
  sl = pl.ds(s * RPT, RPT)
  pltpu.sync_copy(acc_sh.at[sl], row_v)
  pltpu.sync_copy(row_v, out_hbm.at[c, sl])


@functools.partial(
    pl.kernel,
    out_type=jax.ShapeDtypeStruct((NC, NPAD, D), jnp.float32),
    mesh=_MESH,
    scratch_types=[
        pltpu.VMEM((3, 2, EB), jnp.int32),
        pltpu.VMEM((3, 2, EB), jnp.int32),
        pltpu.VMEM((EB, D), jnp.float32),
        pltpu.VMEM((EB, D), jnp.float32),
        pltpu.VMEM_SHARED((NPAD, D), jnp.float32),
        pltpu.SemaphoreType.DMA,
        pltpu.SemaphoreType.DMA,
        pltpu.SemaphoreType.DMA,
        pltpu.SemaphoreType.DMA,
        pltpu.SemaphoreType.DMA,
    ],
)
def _spmm_sc(p_hbm, src_hbm, dst_hbm, out_hbm,
             srcb, dstb, bufa, bufb, acc_sh, sga, sgb, ssa, ssb, sidx):
  c = lax.axis_index("c")
  s = lax.axis_index("s")
  zvec = jnp.zeros((L,), jnp.float32)
  def zrow(r, _):
    for k in range(D // L):
      bufa[r, pl.ds(k * L, L)] = zvec
    return 0
  lax.fori_loop(0, EB, zrow, 0)
  _zero_shared_slice(acc_sh, bufa, s)
  plsc.subcore_barrier()

  pltpu.sync_copy(src_hbm.at[c, s, 0], srcb.at[0])
  pltpu.sync_copy(dst_hbm.at[c, s, 0], dstb.at[0])
  pltpu.async_copy(src_hbm.at[c, s, 1], srcb.at[1], sidx)
  pltpu.async_copy(dst_hbm.at[c, s, 1], dstb.at[1], sidx)
  pltpu.async_copy(p_hbm.at[srcb.at[0, 0]], bufa, sga)
  pltpu.async_copy(p_hbm.at[srcb.at[0, 1]], bufb, sgb)

  def body(t, _):
    sl = lax.rem(t, 3)
    sl1 = lax.rem(t + 1, 3)
    sl2 = lax.rem(t + 2, 3)
    pltpu.make_async_copy(p_hbm.at[srcb.at[sl, 0]], bufa, sga).wait()
    pltpu.async_copy(bufa, acc_sh.at[dstb.at[sl, 0]], ssa, add=True)
    pltpu.make_async_copy(p_hbm.at[srcb.at[sl, 1]], bufb, sgb).wait()
    pltpu.async_copy(bufb, acc_sh.at[dstb.at[sl, 1]], ssb, add=True)

    @pl.when(t < NCH - 1)
    def _():
      pltpu.make_async_copy(src_hbm.at[c, s, t + 1], srcb.at[sl1], sidx).wait()
      pltpu.make_async_copy(dst_hbm.at[c, s, t + 1], dstb.at[sl1], sidx).wait()

      @pl.when(t < NCH - 2)
      def _():
        pltpu.async_copy(src_hbm.at[c, s, t + 2], srcb.at[sl2], sidx)
        pltpu.async_copy(dst_hbm.at[c, s, t + 2], dstb.at[sl2], sidx)
      pltpu.make_async_copy(bufa, acc_sh.at[dstb.at[sl, 0]], ssa).wait()
      pltpu.async_copy(p_hbm.at[srcb.at[sl1, 0]], bufa, sga)
      pltpu.make_async_copy(bufb, acc_sh.at[dstb.at[sl, 1]], ssb).wait()
      pltpu.async_copy(p_hbm.at[srcb.at[sl1, 1]], bufb, sgb)
    return 0

  lax.fori_loop(0, NCH, body, 0)

  lsl = (NCH - 1) % 3
  pltpu.make_async_copy(bufa, acc_sh.at[dstb.at[lsl, 0]], ssa).wait()
  pltpu.make_async_copy(bufb, acc_sh.at[dstb.at[lsl, 1]], ssb).wait()

  plsc.subcore_barrier()
  _copyout_shared_slice(acc_sh, out_hbm, bufa, c, s)


def _edge_e(s_v, d_v, src_v, dst_v, j, k):
  si = src_v[j, pl.ds(k * L, L)]
  di = dst_v[j, pl.ds(k * L, L)]
  sv = plsc.load_gather(s_v, [si])
  dv = plsc.load_gather(d_v, [di])
  z = sv + dv
  return di, jnp.where(z >= 0, z, 0.2 * z)


@functools.partial(
    pl.kernel,
    out_type=jax.ShapeDtypeStruct((NC, NPAD), jnp.float32),
    mesh=_MESH,
    scratch_types=[
        pltpu.VMEM((NPAD,), jnp.float32),
        pltpu.VMEM((NPAD,), jnp.float32),
        pltpu.VMEM((NPAD,), jnp.float32),
        pltpu.VMEM((NB, EB), jnp.int32),
        pltpu.VMEM((NB, EB), jnp.int32),
        pltpu.VMEM((RPT,), jnp.float32),
        pltpu.VMEM((RPT,), jnp.float32),
        pltpu.VMEM_SHARED((NS, NPAD), jnp.float32),
    ],
    compiler_params=pltpu.CompilerParams(needs_layout_passes=False),
)
def _gat_max_sc(s_hbm, d_hbm, src_hbm, dst_hbm, out_hbm,
                s_v, d_v, m_v, src_v, dst_v, acc_v, tmp_v, stage_sh):
  c = lax.axis_index("c")
  s = lax.axis_index("s")
  pltpu.sync_copy(s_hbm, s_v)
  pltpu.sync_copy(d_hbm, d_v)
  pltpu.sync_copy(src_hbm.at[c, s], src_v)
  pltpu.sync_copy(dst_hbm.at[c, s], dst_v)
  _fill(m_v, NPAD, NEG)

  def batch(j, _):
    for k in range(EB // L):
      di, e = _edge_e(s_v, d_v, src_v, dst_v, j, k)
      cur = plsc.load_gather(m_v, [di])
      need = e > cur

      def w_body(need):
        plsc.store_scatter(m_v, [di], e, mask=need)
        cur = plsc.load_gather(m_v, [di])
        return e > cur

      lax.while_loop(jnp.any, w_body, need)
    return 0

  lax.fori_loop(0, NB, batch, 0)

  pltpu.sync_copy(m_v, stage_sh.at[s])
  plsc.subcore_barrier()
  sl = pl.ds(s * RPT, RPT)
  pltpu.sync_copy(stage_sh.at[0, sl], acc_v)
  for i in range(1, NS):
    pltpu.sync_copy(stage_sh.at[i, sl], tmp_v)
    def red(t, _):
      ds = pl.ds(t * L, L)
      acc_v[ds] = jnp.maximum(acc_v[ds], tmp_v[ds])
      return 0
    lax.fori_loop(0, RPT // L, red, 0)
  pltpu.sync_copy(acc_v, out_hbm.at[c, sl])


@functools.partial(
    pl.kernel,
    out_type=jax.ShapeDtypeStruct((NC, NPAD), jnp.float32),
    mesh=_MESH,
    scratch_types=[
        pltpu.VMEM((NPAD,), jnp.float32),
        pltpu.VMEM((NPAD,), jnp.float32),
        pltpu.VMEM((NPAD,), jnp.float32),
        pltpu.VMEM((NPAD,), jnp.float32),
        pltpu.VMEM((NB, EB), jnp.int32),
        pltpu.VMEM((NB, EB), jnp.int32),
        pltpu.VMEM((2, EB), jnp.float32),
        pltpu.VMEM((RPT,), jnp.float32),
        pltpu.VMEM_SHARED((NPAD,), jnp.float32),
        pltpu.SemaphoreType.DMA,
        pltpu.SemaphoreType.DMA,
    ],
    compiler_params=pltpu.CompilerParams(needs_layout_passes=False),
)
def _gat_denom_sc(s_hbm, d_hbm, m_hbm, src_hbm, dst_hbm, out_hbm,
                  s_v, d_v, m_v, t_v, src_v, dst_v, val_v, row_v, acc_sh,
                  ssa, ssb):
  c = lax.axis_index("c")
  s = lax.axis_index("s")
  pltpu.sync_copy(s_hbm, s_v)
  pltpu.sync_copy(d_hbm, d_v)
  pltpu.sync_copy(m_hbm.at[0], m_v)
  pltpu.sync_copy(m_hbm.at[1], t_v)
  def mx(t, _):
    ds = pl.ds(t * L, L)
    m_v[ds] = jnp.maximum(m_v[ds], t_v[ds])
    return 0
  lax.fori_loop(0, NPAD // L, mx, 0)
  pltpu.sync_copy(src_hbm.at[c, s], src_v)
  pltpu.sync_copy(dst_hbm.at[c, s], dst_v)
  _fill(row_v, RPT, 0.0)
  pltpu.sync_copy(row_v, acc_sh.at[pl.ds(s * RPT, RPT)])
  plsc.subcore_barrier()

  def fill_val(slot, j):
    for k in range(EB // L):
      di, e = _edge_e(s_v, d_v, src_v, dst_v, j, k)
      mv = plsc.load_gather(m_v, [di])
      val_v[slot, pl.ds(k * L, L)] = jnp.exp(e - mv)

  def pair(u, _):
    ja = 2 * u
    jb = 2 * u + 1

    @pl.when(u >= 1)
    def _():
      pltpu.make_async_copy(val_v.at[0], acc_sh.at[dst_v.at[ja]], ssa).wait()
    fill_val(0, ja)
    pltpu.async_copy(val_v.at[0], acc_sh.at[dst_v.at[ja]], ssa, add=True)

    @pl.when(u >= 1)
    def _():
      pltpu.make_async_copy(val_v.at[1], acc_sh.at[dst_v.at[jb]], ssb).wait()
    fill_val(1, jb)
    pltpu.async_copy(val_v.at[1], acc_sh.at[dst_v.at[jb]], ssb, add=True)
    return 0

  lax.fori_loop(0, NB // 2, pair, 0)
  pltpu.make_async_copy(val_v.at[0], acc_sh.at[dst_v.at[NB - 2]], ssa).wait()
  pltpu.make_async_copy(val_v.at[1], acc_sh.at[dst_v.at[NB - 1]], ssb).wait()

  plsc.subcore_barrier()
  sl = pl.ds(s * RPT, RPT)
  pltpu.sync_copy(acc_sh.at[sl], row_v)
  pltpu.sync_copy(row_v, out_hbm.at[c, sl])


@functools.partial(
    pl.kernel,
    out_type=jax.ShapeDtypeStruct((NC, NPAD), jnp.float32),
    mesh=_MESH,
    scratch_types=[
        pltpu.VMEM((NPAD,), jnp.float32),
        pltpu.VMEM((NPAD,), jnp.float32),
        pltpu.VMEM((NPAD,), jnp.float32),
        pltpu.VMEM((NPAD,), jnp.float32),
        pltpu.VMEM((NPAD,), jnp.float32),
        pltpu.VMEM((NB, EB), jnp.int32),
        pltpu.VMEM((NB, EB), jnp.int32),
        pltpu.VMEM((2, EB), jnp.float32),
        pltpu.VMEM((RPT,), jnp.float32),
        pltpu.VMEM_SHARED((NPAD,), jnp.float32),
        pltpu.SemaphoreType.DMA,
        pltpu.SemaphoreType.DMA,
    ],
    compiler_params=pltpu.CompilerParams(needs_layout_passes=False),
)
def _gat_w_sc(s_hbm, d_hbm, m_hbm, den_hbm, src_hbm, dst_hbm, out_hbm,
              s_v, d_v, m_v, den_v, t_v, src_v, dst_v, val_v, row_v, acc_sh,
              ssa, ssb):
  c = lax.axis_index("c")
  s = lax.axis_index("s")
  pltpu.sync_copy(s_hbm, s_v)
  pltpu.sync_copy(d_hbm, d_v)
  pltpu.sync_copy(m_hbm.at[0], m_v)
  pltpu.sync_copy(m_hbm.at[1], t_v)
  def mx(t, _):
    ds = pl.ds(t * L, L)
    m_v[ds] = jnp.maximum(m_v[ds], t_v[ds])
    return 0
  lax.fori_loop(0, NPAD // L, mx, 0)
  pltpu.sync_copy(den_hbm.at[0], den_v)
  pltpu.sync_copy(den_hbm.at[1], t_v)
  def ad(t, _):
    ds = pl.ds(t * L, L)
    den_v[ds] = den_v[ds] + t_v[ds] + 1e-16
    return 0
  lax.fori_loop(0, NPAD // L, ad, 0)
  pltpu.sync_copy(src_hbm.at[c, s], src_v)
  pltpu.sync_copy(dst_hbm.at[c, s], dst_v)
  _fill(row_v, RPT, 0.0)
  pltpu.sync_copy(row_v, acc_sh.at[pl.ds(s * RPT, RPT)])
  plsc.subcore_barrier()

  def fill_val(slot, j):
    for k in range(EB // L):
      di, e = _edge_e(s_v, d_v, src_v, dst_v, j, k)
      mv = plsc.load_gather(m_v, [di])
      dv = plsc.load_gather(den_v, [di])
      val_v[slot, pl.ds(k * L, L)] = jnp.exp(e - mv) / dv

  def pair(u, _):
    ja = 2 * u
    jb = 2 * u + 1

    @pl.when(u >= 1)
    def _():
      pltpu.make_async_copy(val_v.at[0], acc_sh.at[src_v.at[ja]], ssa).wait()
    fill_val(0, ja)
    pltpu.async_copy(val_v.at[0], acc_sh.at[src_v.at[ja]], ssa, add=True)

    @pl.when(u >= 1)
    def _():
      pltpu.make_async_copy(val_v.at[1], acc_sh.at[src_v.at[jb]], ssb).wait()
    fill_val(1, jb)
    pltpu.async_copy(val_v.at[1], acc_sh.at[src_v.at[jb]], ssb, add=True)
    return 0

  lax.fori_loop(0, NB // 2, pair, 0)
  pltpu.make_async_copy(val_v.at[0], acc_sh.at[src_v.at[NB - 2]], ssa).wait()
  pltpu.make_async_copy(val_v.at[1], acc_sh.at[src_v.at[NB - 1]], ssb).wait()

  plsc.subcore_barrier()
  sl = pl.ds(s * RPT, RPT)
  pltpu.sync_copy(acc_sh.at[sl], row_v)
  pltpu.sync_copy(row_v, out_hbm.at[c, sl])


def _dis_of(deg2):
  deg = deg2[0] + deg2[1]
  return jnp.where(deg > 0, lax.rsqrt(jnp.maximum(deg, 1e-12)), 0.0)


def _t0_body(x_ref, w_ref, b_ref, o_ref):
  o_ref[...] = jax.nn.relu(
      jnp.dot(x_ref[...], w_ref[...], preferred_element_type=jnp.float32)
      + b_ref[...])


def _t1_body(h_ref, w_ref, deg_ref, o_ref):
  dis = _dis_of(deg_ref[...])
  o_ref[...] = dis[:, None] * jnp.dot(
      h_ref[...], w_ref[...], preferred_element_type=jnp.float32)


def _t2_body(q_ref, deg_ref, b_ref, w_ref, o_ref):
  dis = _dis_of(deg_ref[...])
  h = jax.nn.relu(dis[:, None] * (q_ref[0] + q_ref[1]) + b_ref[...])
  o_ref[...] = dis[:, None] * jnp.dot(
      h, w_ref[...], preferred_element_type=jnp.float32)


def _t3_body(q_ref, deg_ref, b_ref, wg_ref, asrc_ref, adst_ref,
             hg_ref, s_ref, d_ref):
  dis = _dis_of(deg_ref[...])
  h = jax.nn.relu(dis[:, None] * (q_ref[0] + q_ref[1]) + b_ref[...])
  hg = jnp.dot(h, wg_ref[...], preferred_element_type=jnp.float32)
  hg_ref[...] = hg
  s_ref[...] = jnp.dot(hg, asrc_ref[...],
                       preferred_element_type=jnp.float32).T
  d_ref[...] = jnp.dot(hg, adst_ref[...],
                       preferred_element_type=jnp.float32).T


def _t4_body(w_ref, hg_ref, bg_ref, o_ref):
  i = pl.program_id(0)
  rows = i * BLK + lax.broadcasted_iota(jnp.int32, (BLK,), 0)
  w = jnp.where(rows < N, w_ref[0] + w_ref[1], 0.0)
  part = jnp.dot(w[None, :], hg_ref[...], preferred_element_type=jnp.float32)

  @pl.when(i == 0)
  def _():
    o_ref[...] = jnp.zeros_like(o_ref)
  o_ref[...] += part

  @pl.when(i == GRID - 1)
  def _():
    o_ref[...] = o_ref[...] * (1.0 / N) + bg_ref[...]


def _row_spec(block=None):
  return pl.BlockSpec((BLK, D) if block is None else block, lambda i: (0, 0))


_SPEC_ROWS = pl.BlockSpec((BLK, D), lambda i: (i, 0))
_SPEC_W = pl.BlockSpec((D, D), lambda i: (0, 0))
_SPEC_B = pl.BlockSpec((1, D), lambda i: (0, 0))
_SPEC_DEG = pl.BlockSpec((NC, BLK), lambda i: (0, i))
_SPEC_Q = pl.BlockSpec((NC, BLK, D), lambda i: (0, i, 0))
_SPEC_VEC = pl.BlockSpec((D, 1), lambda i: (0, 0))
_SPEC_SD = pl.BlockSpec((1, BLK), lambda i: (0, i))


def _t0(xp, W_emb, b_emb):
  return pl.pallas_call(
      _t0_body, grid=(GRID,),
      in_specs=[_SPEC_ROWS, _SPEC_W, _SPEC_B],
      out_specs=_SPEC_ROWS,
      out_shape=jax.ShapeDtypeStruct((NPAD, D), jnp.float32),
  )(xp, W_emb, b_emb)


def _t1(h0, W1, deg2):
  return pl.pallas_call(
      _t1_body, grid=(GRID,),
      in_specs=[_SPEC_ROWS, _SPEC_W, _SPEC_DEG],
      out_specs=_SPEC_ROWS,
      out_shape=jax.ShapeDtypeStruct((NPAD, D), jnp.float32),
  )(h0, W1, deg2)


def _t2(q2, deg2, b, W):
  return pl.pallas_call(
      _t2_body, grid=(GRID,),
      in_specs=[_SPEC_Q, _SPEC_DEG, _SPEC_B, _SPEC_W],
      out_specs=_SPEC_ROWS,
      out_shape=jax.ShapeDtypeStruct((NPAD, D), jnp.float32),
  )(q2, deg2, b, W)


def _t3(q2, deg2, b, Wg, a_src, a_dst):
  return pl.pallas_call(
      _t3_body, grid=(GRID,),
      in_specs=[_SPEC_Q, _SPEC_DEG, _SPEC_B, _SPEC_W, _SPEC_VEC, _SPEC_VEC],
      out_specs=[_SPEC_ROWS, _SPEC_SD, _SPEC_SD],
      out_shape=[
          jax.ShapeDtypeStruct((NPAD, D), jnp.float32),
          jax.ShapeDtypeStruct((1, NPAD), jnp.float32),
          jax.ShapeDtypeStruct((1, NPAD), jnp.float32),
      ],
  )(q2, deg2, b, Wg, a_src, a_dst)


def _t4(w2, hg, bg):
  return pl.pallas_call(
      _t4_body, grid=(GRID,),
      in_specs=[_SPEC_DEG, _SPEC_ROWS, _SPEC_B],
      out_specs=_SPEC_B,
      out_shape=jax.ShapeDtypeStruct((1, D), jnp.float32),
  )(w2, hg, bg)


def kernel(x, edge_index, W_emb, b_emb, W1, b1, W2, b2, Wg, a_src, a_dst, bg):
  ei = edge_index.astype(jnp.int32)
  loop = jnp.arange(N, dtype=jnp.int32)
  npad_e = ETOT - (ei.shape[1] + N)
  padidx = N + 200 + (jnp.arange(npad_e, dtype=jnp.int32) % 8)
  src4 = jnp.concatenate([ei[0], loop, padidx]).reshape(NC, NS, NB, EB)
  dst4 = jnp.concatenate([ei[1], loop, padidx]).reshape(NC, NS, NB, EB)
  xp = jnp.zeros((NPAD, D), jnp.float32).at[:N].set(x)
  b_emb2 = b_emb.reshape(1, D)
  b12 = b1.reshape(1, D)
  b22 = b2.reshape(1, D)
  bg2 = bg.reshape(1, D)
  a_src2 = a_src.reshape(D, 1)
  a_dst2 = a_dst.reshape(D, 1)

  src5 = src4.reshape(NC, NS, NCH, 2, EB)
  dst5 = dst4.reshape(NC, NS, NCH, 2, EB)

  deg2 = _deg_sc(dst4)
  h0 = _t0(xp, W_emb, b_emb2)
  p1 = _t1(h0, W1, deg2)
  q1 = _spmm_sc(p1, src5, dst5)
  p2 = _t2(q1, deg2, b12, W2)
  q2 = _spmm_sc(p2, src5, dst5)
  hg, s2, d2 = _t3(q2, deg2, b22, Wg, a_src2, a_dst2)
  s1 = s2.reshape(NPAD)
  d1 = d2.reshape(NPAD)
  m2 = _gat_max_sc(s1, d1, src4, dst4)
  den2 = _gat_denom_sc(s1, d1, m2, src4, dst4)
  w2 = _gat_w_sc(s1, d1, m2, den2, src4, dst4)
  return _t4(w2, hg, bg2)


# speedup vs baseline: 49.4699x; 1.1985x over previous
import functools

import jax
import jax.numpy as jnp
from jax import lax
from jax.experimental import pallas as pl
from jax.experimental.pallas import tpu as pltpu
from jax.experimental.pallas import tpu_sc as plsc

N = 10000
D = 128
NPAD = 10240
NC = 2
NS = 16
L = 16
EB = 128
NB = 82
SB = 96
NBT = 108
ETOT_S = NC * NS * NBT * SB
EPW = NB * EB
ETOT = NC * NS * EPW
RPT = NPAD // NS
BLK = 1024
GRID = NPAD // BLK
NEG = -1e30

_MESH = plsc.VectorSubcoreMesh(
    core_axis_name="c", subcore_axis_name="s", num_cores=NC, num_subcores=NS)


def _fill(ref, n, value):
  vec = jnp.full((L,), value, ref.dtype)
  def body(i, _):
    ref[pl.ds(i * L, L)] = vec
    return 0
  lax.fori_loop(0, n // L, body, 0)


def _zero_shared_slice(acc_sh, zbuf, s):
  nz = zbuf.shape[0]
  for i in range(RPT // nz):
    pltpu.sync_copy(zbuf, acc_sh.at[pl.ds(s * RPT + i * nz, nz)])
  rem = RPT % nz
  if rem:
    pltpu.sync_copy(zbuf.at[pl.ds(0, rem)],
                    acc_sh.at[pl.ds(s * RPT + (RPT // nz) * nz, rem)])


def _copyout_shared_slice(acc_sh, out_hbm, bounce, c, s):
  nz = bounce.shape[0]
  for i in range(RPT // nz):
    sl = pl.ds(s * RPT + i * nz, nz)
    pltpu.sync_copy(acc_sh.at[sl], bounce)
    pltpu.sync_copy(bounce, out_hbm.at[c, sl])
  rem = RPT % nz
  if rem:
    sl = pl.ds(s * RPT + (RPT // nz) * nz, rem)
    pltpu.sync_copy(acc_sh.at[sl], bounce.at[pl.ds(0, rem)])
    pltpu.sync_copy(bounce.at[pl.ds(0, rem)], out_hbm.at[c, sl])


@functools.partial(
    pl.kernel,
    out_type=jax.ShapeDtypeStruct((NC, NPAD), jnp.float32),
    mesh=_MESH,
    scratch_types=[
        pltpu.VMEM((NB, EB), jnp.int32),
        pltpu.VMEM((EB,), jnp.float32),
        pltpu.VMEM((RPT,), jnp.float32),
        pltpu.VMEM_SHARED((NPAD,), jnp.float32),
        pltpu.SemaphoreType.DMA,
    ],
)
def _deg_sc(dst_hbm, out_hbm, idx_v, ones_v, row_v, acc_sh, sadd):
  c = lax.axis_index("c")
  s = lax.axis_index("s")
  _fill(ones_v, EB, 1.0)
  _fill(row_v, RPT, 0.0)
  pltpu.sync_copy(row_v, acc_sh.at[pl.ds(s * RPT, RPT)])
  pltpu.sync_copy(dst_hbm.at[c, s], idx_v)
  plsc.subcore_barrier()

  def body(j, _):
    pltpu.async_copy(ones_v, acc_sh.at[idx_v.at[j]], sadd, add=True)
    return 0
  lax.fori_loop(0, NB, body, 0)

  def drain(j, _):
    pltpu.make_async_copy(ones_v, acc_sh.at[idx_v.at[j]], sadd).wait()
    return 0
  lax.fori_loop(0, NB, drain, 0)

  plsc.subcore_barrier()
  sl = pl.ds(s * RPT, RPT)
  pltpu.sync_copy(acc_sh.at[sl], row_v)
  pltpu.sync_copy(row_v, out_hbm.at[c, sl])


@functools.partial(
    pl.kernel,
    out_type=jax.ShapeDtypeStruct((NC, NPAD, D), jnp.float32),
    mesh=_MESH,
    scratch_types=[
        pltpu.VMEM((4, SB), jnp.int32),
        pltpu.VMEM((4, SB), jnp.int32),
        pltpu.VMEM((SB, D), jnp.float32),
        pltpu.VMEM((SB, D), jnp.float32),
        pltpu.VMEM((SB, D), jnp.float32),
        pltpu.VMEM_SHARED((NPAD, D), jnp.float32),
        pltpu.SemaphoreType.DMA,
        pltpu.SemaphoreType.DMA,
        pltpu.SemaphoreType.DMA,
        pltpu.SemaphoreType.DMA,
        pltpu.SemaphoreType.DMA,
        pltpu.SemaphoreType.DMA,
        pltpu.SemaphoreType.DMA,
    ],
)
def _spmm_sc(p_hbm, src_hbm, dst_hbm, out_hbm,
             srcb, dstb, buf0, buf1, buf2, acc_sh,
             sg0, sg1, sg2, ss0, ss1, ss2, sidx):
  c = lax.axis_index("c")
  s = lax.axis_index("s")
  bufs = (buf0, buf1, buf2)
  sgs = (sg0, sg1, sg2)
  sss = (ss0, ss1, ss2)
  zvec = jnp.zeros((L,), jnp.float32)
  def zrow(r, _):
    for k in range(D // L):
      buf0[r, pl.ds(k * L, L)] = zvec
    return 0
  lax.fori_loop(0, SB, zrow, 0)
  _zero_shared_slice(acc_sh, buf0, s)
  plsc.subcore_barrier()

  pltpu.sync_copy(src_hbm.at[c, s, 0], srcb.at[0])
  pltpu.sync_copy(dst_hbm.at[c, s, 0], dstb.at[0])
  pltpu.sync_copy(src_hbm.at[c, s, 1], srcb.at[1])
  pltpu.sync_copy(dst_hbm.at[c, s, 1], dstb.at[1])
  pltpu.async_copy(src_hbm.at[c, s, 2], srcb.at[2], sidx)
  pltpu.async_copy(dst_hbm.at[c, s, 2], dstb.at[2], sidx)
  pltpu.async_copy(p_hbm.at[srcb.at[0]], buf0, sg0)
  pltpu.async_copy(p_hbm.at[srcb.at[1]], buf1, sg1)

  def stage(t, b):
    bp = (b + 2) % 3
    i0 = lax.rem(t, 4)
    i2 = lax.rem(t + 2, 4)
    i3 = lax.rem(t + 3, 4)
    pltpu.make_async_copy(p_hbm.at[srcb.at[i0]], bufs[b], sgs[b]).wait()
    pltpu.async_copy(bufs[b], acc_sh.at[dstb.at[i0]], sss[b], add=True)

    @pl.when(t >= 1)
    def _():
      pltpu.make_async_copy(bufs[bp], acc_sh.at[dstb.at[i3]], sss[bp]).wait()

    @pl.when(t + 2 < NBT)
    def _():
      pltpu.make_async_copy(src_hbm.at[c, s, t + 2], srcb.at[i2], sidx).wait()
      pltpu.make_async_copy(dst_hbm.at[c, s, t + 2], dstb.at[i2], sidx).wait()

      @pl.when(t + 3 < NBT)
      def _():
        pltpu.async_copy(src_hbm.at[c, s, t + 3], srcb.at[i3], sidx)
        pltpu.async_copy(dst_hbm.at[c, s, t + 3], dstb.at[i3], sidx)
      pltpu.async_copy(p_hbm.at[srcb.at[i2]], bufs[bp], sgs[bp])

  def body(u, _):
    stage(3 * u, 0)
    stage(3 * u + 1, 1)
    stage(3 * u + 2, 2)
    return 0

  lax.fori_loop(0, NBT // 3, body, 0)

  lb = (NBT - 1) % 3
  pltpu.make_async_copy(
      bufs[lb], acc_sh.at[dstb.at[(NBT - 1) % 4]], sss[lb]).wait()

  plsc.subcore_barrier()
  _copyout_shared_slice(acc_sh, out_hbm, buf0, c, s)


def _edge_e(s_v, d_v, src_v, dst_v, j, k):
  si = src_v[j, pl.ds(k * L, L)]
  di = dst_v[j, pl.ds(k * L, L)]
  sv = plsc.load_gather(s_v, [si])
  dv = plsc.load_gather(d_v, [di])
  z = sv + dv
  return di, jnp.where(z >= 0, z, 0.2 * z)


@functools.partial(
    pl.kernel,
    out_type=jax.ShapeDtypeStruct((NC, NPAD), jnp.float32),
    mesh=_MESH,
    scratch_types=[
        pltpu.VMEM((NPAD,), jnp.float32),
        pltpu.VMEM((NPAD,), jnp.float32),
        pltpu.VMEM((NPAD,), jnp.float32),
        pltpu.VMEM((NB, EB), jnp.int32),
        pltpu.VMEM((NB, EB), jnp.int32),
        pltpu.VMEM((RPT,), jnp.float32),
        pltpu.VMEM((RPT,), jnp.float32),
        pltpu.VMEM_SHARED((NS, NPAD), jnp.float32),
    ],
    compiler_params=pltpu.CompilerParams(needs_layout_passes=False),
)
def _gat_max_sc(s_hbm, d_hbm, src_hbm, dst_hbm, out_hbm,
                s_v, d_v, m_v, src_v, dst_v, acc_v, tmp_v, stage_sh):
  c = lax.axis_index("c")
  s = lax.axis_index("s")
  pltpu.sync_copy(s_hbm, s_v)
  pltpu.sync_copy(d_hbm, d_v)
  pltpu.sync_copy(src_hbm.at[c, s], src_v)
  pltpu.sync_copy(dst_hbm.at[c, s], dst_v)
  _fill(m_v, NPAD, NEG)

  def batch(j, _):
    for k in range(EB // L):
      di, e = _edge_e(s_v, d_v, src_v, dst_v, j, k)
      cur = plsc.load_gather(m_v, [di])
      need = e > cur

      def w_body(need):
        plsc.store_scatter(m_v, [di], e, mask=need)
        cur = plsc.load_gather(m_v, [di])
        return e > cur

      lax.while_loop(jnp.any, w_body, need)
    return 0

  lax.fori_loop(0, NB, batch, 0)

  pltpu.sync_copy(m_v, stage_sh.at[s])
  plsc.subcore_barrier()
  sl = pl.ds(s * RPT, RPT)
  pltpu.sync_copy(stage_sh.at[0, sl], acc_v)
  for i in range(1, NS):
    pltpu.sync_copy(stage_sh.at[i, sl], tmp_v)
    def red(t, _):
      ds = pl.ds(t * L, L)
      acc_v[ds] = jnp.maximum(acc_v[ds], tmp_v[ds])
      return 0
    lax.fori_loop(0, RPT // L, red, 0)
  pltpu.sync_copy(acc_v, out_hbm.at[c, sl])


@functools.partial(
    pl.kernel,
    out_type=jax.ShapeDtypeStruct((NC, NPAD), jnp.float32),
    mesh=_MESH,
    scratch_types=[
        pltpu.VMEM((NPAD,), jnp.float32),
        pltpu.VMEM((NPAD,), jnp.float32),
        pltpu.VMEM((NPAD,), jnp.float32),
        pltpu.VMEM((NPAD,), jnp.float32),
        pltpu.VMEM((NB, EB), jnp.int32),
        pltpu.VMEM((NB, EB), jnp.int32),
        pltpu.VMEM((2, EB), jnp.float32),
        pltpu.VMEM((RPT,), jnp.float32),
        pltpu.VMEM_SHARED((NPAD,), jnp.float32),
        pltpu.SemaphoreType.DMA,
        pltpu.SemaphoreType.DMA,
    ],
    compiler_params=pltpu.CompilerParams(needs_layout_passes=False),
)
def _gat_denom_sc(s_hbm, d_hbm, m_hbm, src_hbm, dst_hbm, out_hbm,
                  s_v, d_v, m_v, t_v, src_v, dst_v, val_v, row_v, acc_sh,
                  ssa, ssb):
  c = lax.axis_index("c")
  s = lax.axis_index("s")
  pltpu.sync_copy(s_hbm, s_v)
  pltpu.sync_copy(d_hbm, d_v)
  pltpu.sync_copy(m_hbm.at[0], m_v)
  pltpu.sync_copy(m_hbm.at[1], t_v)
  def mx(t, _):
    ds = pl.ds(t * L, L)
    m_v[ds] = jnp.maximum(m_v[ds], t_v[ds])
    return 0
  lax.fori_loop(0, NPAD // L, mx, 0)
  pltpu.sync_copy(src_hbm.at[c, s], src_v)
  pltpu.sync_copy(dst_hbm.at[c, s], dst_v)
  _fill(row_v, RPT, 0.0)
  pltpu.sync_copy(row_v, acc_sh.at[pl.ds(s * RPT, RPT)])
  plsc.subcore_barrier()

  def fill_val(slot, j):
    for k in range(EB // L):
      di, e = _edge_e(s_v, d_v, src_v, dst_v, j, k)
      mv = plsc.load_gather(m_v, [di])
      val_v[slot, pl.ds(k * L, L)] = jnp.exp(e - mv)

  def pair(u, _):
    ja = 2 * u
    jb = 2 * u + 1

    @pl.when(u >= 1)
    def _():
      pltpu.make_async_copy(val_v.at[0], acc_sh.at[dst_v.at[ja]], ssa).wait()
    fill_val(0, ja)
    pltpu.async_copy(val_v.at[0], acc_sh.at[dst_v.at[ja]], ssa, add=True)

    @pl.when(u >= 1)
    def _():
      pltpu.make_async_copy(val_v.at[1], acc_sh.at[dst_v.at[jb]], ssb).wait()
    fill_val(1, jb)
    pltpu.async_copy(val_v.at[1], acc_sh.at[dst_v.at[jb]], ssb, add=True)
    return 0

  lax.fori_loop(0, NB // 2, pair, 0)
  pltpu.make_async_copy(val_v.at[0], acc_sh.at[dst_v.at[NB - 2]], ssa).wait()
  pltpu.make_async_copy(val_v.at[1], acc_sh.at[dst_v.at[NB - 1]], ssb).wait()

  plsc.subcore_barrier()
  sl = pl.ds(s * RPT, RPT)
  pltpu.sync_copy(acc_sh.at[sl], row_v)
  pltpu.sync_copy(row_v, out_hbm.at[c, sl])


@functools.partial(
    pl.kernel,
    out_type=jax.ShapeDtypeStruct((NC, NPAD), jnp.float32),
    mesh=_MESH,
    scratch_types=[
        pltpu.VMEM((NPAD,), jnp.float32),
        pltpu.VMEM((NPAD,), jnp.float32),
        pltpu.VMEM((NPAD,), jnp.float32),
        pltpu.VMEM((NPAD,), jnp.float32),
        pltpu.VMEM((NPAD,), jnp.float32),
        pltpu.VMEM((NB, EB), jnp.int32),
        pltpu.VMEM((NB, EB), jnp.int32),
        pltpu.VMEM((2, EB), jnp.float32),
        pltpu.VMEM((RPT,), jnp.float32),
        pltpu.VMEM_SHARED((NPAD,), jnp.float32),
        pltpu.SemaphoreType.DMA,
        pltpu.SemaphoreType.DMA,
    ],
    compiler_params=pltpu.CompilerParams(needs_layout_passes=False),
)
def _gat_w_sc(s_hbm, d_hbm, m_hbm, den_hbm, src_hbm, dst_hbm, out_hbm,
              s_v, d_v, m_v, den_v, t_v, src_v, dst_v, val_v, row_v, acc_sh,
              ssa, ssb):
  c = lax.axis_index("c")
  s = lax.axis_index("s")
  pltpu.sync_copy(s_hbm, s_v)
  pltpu.sync_copy(d_hbm, d_v)
  pltpu.sync_copy(m_hbm.at[0], m_v)
  pltpu.sync_copy(m_hbm.at[1], t_v)
  def mx(t, _):
    ds = pl.ds(t * L, L)
    m_v[ds] = jnp.maximum(m_v[ds], t_v[ds])
    return 0
  lax.fori_loop(0, NPAD // L, mx, 0)
  pltpu.sync_copy(den_hbm.at[0], den_v)
  pltpu.sync_copy(den_hbm.at[1], t_v)
  def ad(t, _):
    ds = pl.ds(t * L, L)
    den_v[ds] = den_v[ds] + t_v[ds] + 1e-16
    return 0
  lax.fori_loop(0, NPAD // L, ad, 0)
  pltpu.sync_copy(src_hbm.at[c, s], src_v)
  pltpu.sync_copy(dst_hbm.at[c, s], dst_v)
  _fill(row_v, RPT, 0.0)
  pltpu.sync_copy(row_v, acc_sh.at[pl.ds(s * RPT, RPT)])
  plsc.subcore_barrier()

  def fill_val(slot, j):
    for k in range(EB // L):
      di, e = _edge_e(s_v, d_v, src_v, dst_v, j, k)
      mv = plsc.load_gather(m_v, [di])
      dv = plsc.load_gather(den_v, [di])
      val_v[slot, pl.ds(k * L, L)] = jnp.exp(e - mv) / dv

  def pair(u, _):
    ja = 2 * u
    jb = 2 * u + 1

    @pl.when(u >= 1)
    def _():
      pltpu.make_async_copy(val_v.at[0], acc_sh.at[src_v.at[ja]], ssa).wait()
    fill_val(0, ja)
    pltpu.async_copy(val_v.at[0], acc_sh.at[src_v.at[ja]], ssa, add=True)

    @pl.when(u >= 1)
    def _():
      pltpu.make_async_copy(val_v.at[1], acc_sh.at[src_v.at[jb]], ssb).wait()
    fill_val(1, jb)
    pltpu.async_copy(val_v.at[1], acc_sh.at[src_v.at[jb]], ssb, add=True)
    return 0

  lax.fori_loop(0, NB // 2, pair, 0)
  pltpu.make_async_copy(val_v.at[0], acc_sh.at[src_v.at[NB - 2]], ssa).wait()
  pltpu.make_async_copy(val_v.at[1], acc_sh.at[src_v.at[NB - 1]], ssb).wait()

  plsc.subcore_barrier()
  sl = pl.ds(s * RPT, RPT)
  pltpu.sync_copy(acc_sh.at[sl], row_v)
  pltpu.sync_copy(row_v, out_hbm.at[c, sl])


def _dis_of(deg2):
  deg = deg2[0] + deg2[1]
  return jnp.where(deg > 0, lax.rsqrt(jnp.maximum(deg, 1e-12)), 0.0)


def _t0_body(x_ref, w_ref, b_ref, o_ref):
  o_ref[...] = jax.nn.relu(
      jnp.dot(x_ref[...], w_ref[...], preferred_element_type=jnp.float32)
      + b_ref[...])


def _t1_body(h_ref, w_ref, deg_ref, o_ref):
  dis = _dis_of(deg_ref[...])
  o_ref[...] = dis[:, None] * jnp.dot(
      h_ref[...], w_ref[...], preferred_element_type=jnp.float32)


def _t2_body(q_ref, deg_ref, b_ref, w_ref, o_ref):
  dis = _dis_of(deg_ref[...])
  h = jax.nn.relu(dis[:, None] * (q_ref[0] + q_ref[1]) + b_ref[...])
  o_ref[...] = dis[:, None] * jnp.dot(
      h, w_ref[...], preferred_element_type=jnp.float32)


def _t3_body(q_ref, deg_ref, b_ref, wg_ref, asrc_ref, adst_ref,
             hg_ref, s_ref, d_ref):
  dis = _dis_of(deg_ref[...])
  h = jax.nn.relu(dis[:, None] * (q_ref[0] + q_ref[1]) + b_ref[...])
  hg = jnp.dot(h, wg_ref[...], preferred_element_type=jnp.float32)
  hg_ref[...] = hg
  s_ref[...] = jnp.dot(hg, asrc_ref[...],
                       preferred_element_type=jnp.float32).T
  d_ref[...] = jnp.dot(hg, adst_ref[...],
                       preferred_element_type=jnp.float32).T


def _t4_body(w_ref, hg_ref, bg_ref, o_ref):
  i = pl.program_id(0)
  rows = i * BLK + lax.broadcasted_iota(jnp.int32, (BLK,), 0)
  w = jnp.where(rows < N, w_ref[0] + w_ref[1], 0.0)
  part = jnp.dot(w[None, :], hg_ref[...], preferred_element_type=jnp.float32)

  @pl.when(i == 0)
  def _():
    o_ref[...] = jnp.zeros_like(o_ref)
  o_ref[...] += part

  @pl.when(i == GRID - 1)
  def _():
    o_ref[...] = o_ref[...] * (1.0 / N) + bg_ref[...]


def _row_spec(block=None):
  return pl.BlockSpec((BLK, D) if block is None else block, lambda i: (0, 0))


_SPEC_ROWS = pl.BlockSpec((BLK, D), lambda i: (i, 0))
_SPEC_W = pl.BlockSpec((D, D), lambda i: (0, 0))
_SPEC_B = pl.BlockSpec((1, D), lambda i: (0, 0))
_SPEC_DEG = pl.BlockSpec((NC, BLK), lambda i: (0, i))
_SPEC_Q = pl.BlockSpec((NC, BLK, D), lambda i: (0, i, 0))
_SPEC_VEC = pl.BlockSpec((D, 1), lambda i: (0, 0))
_SPEC_SD = pl.BlockSpec((1, BLK), lambda i: (0, i))


def _t0(xp, W_emb, b_emb):
  return pl.pallas_call(
      _t0_body, grid=(GRID,),
      in_specs=[_SPEC_ROWS, _SPEC_W, _SPEC_B],
      out_specs=_SPEC_ROWS,
      out_shape=jax.ShapeDtypeStruct((NPAD, D), jnp.float32),
  )(xp, W_emb, b_emb)


def _t1(h0, W1, deg2):
  return pl.pallas_call(
      _t1_body, grid=(GRID,),
      in_specs=[_SPEC_ROWS, _SPEC_W, _SPEC_DEG],
      out_specs=_SPEC_ROWS,
      out_shape=jax.ShapeDtypeStruct((NPAD, D), jnp.float32),
  )(h0, W1, deg2)


def _t2(q2, deg2, b, W):
  return pl.pallas_call(
      _t2_body, grid=(GRID,),
      in_specs=[_SPEC_Q, _SPEC_DEG, _SPEC_B, _SPEC_W],
      out_specs=_SPEC_ROWS,
      out_shape=jax.ShapeDtypeStruct((NPAD, D), jnp.float32),
  )(q2, deg2, b, W)


def _t3(q2, deg2, b, Wg, a_src, a_dst):
  return pl.pallas_call(
      _t3_body, grid=(GRID,),
      in_specs=[_SPEC_Q, _SPEC_DEG, _SPEC_B, _SPEC_W, _SPEC_VEC, _SPEC_VEC],
      out_specs=[_SPEC_ROWS, _SPEC_SD, _SPEC_SD],
      out_shape=[
          jax.ShapeDtypeStruct((NPAD, D), jnp.float32),
          jax.ShapeDtypeStruct((1, NPAD), jnp.float32),
          jax.ShapeDtypeStruct((1, NPAD), jnp.float32),
      ],
  )(q2, deg2, b, Wg, a_src, a_dst)


def _t4(w2, hg, bg):
  return pl.pallas_call(
      _t4_body, grid=(GRID,),
      in_specs=[_SPEC_DEG, _SPEC_ROWS, _SPEC_B],
      out_specs=_SPEC_B,
      out_shape=jax.ShapeDtypeStruct((1, D), jnp.float32),
  )(w2, hg, bg)


def kernel(x, edge_index, W_emb, b_emb, W1, b1, W2, b2, Wg, a_src, a_dst, bg):
  ei = edge_index.astype(jnp.int32)
  loop = jnp.arange(N, dtype=jnp.int32)

  def padded(row, tot):
    npad_e = tot - (ei.shape[1] + N)
    padidx = N + 200 + (jnp.arange(npad_e, dtype=jnp.int32) % 8)
    return jnp.concatenate([row, loop, padidx])

  src4 = padded(ei[0], ETOT).reshape(NC, NS, NB, EB)
  dst4 = padded(ei[1], ETOT).reshape(NC, NS, NB, EB)
  src5 = padded(ei[0], ETOT_S).reshape(NC, NS, NBT, SB)
  dst5 = padded(ei[1], ETOT_S).reshape(NC, NS, NBT, SB)
  xp = jnp.zeros((NPAD, D), jnp.float32).at[:N].set(x)
  b_emb2 = b_emb.reshape(1, D)
  b12 = b1.reshape(1, D)
  b22 = b2.reshape(1, D)
  bg2 = bg.reshape(1, D)
  a_src2 = a_src.reshape(D, 1)
  a_dst2 = a_dst.reshape(D, 1)

  deg2 = _deg_sc(dst4)
  h0 = _t0(xp, W_emb, b_emb2)
  p1 = _t1(h0, W1, deg2)
  q1 = _spmm_sc(p1, src5, dst5)
  p2 = _t2(q1, deg2, b12, W2)
  q2 = _spmm_sc(p2, src5, dst5)
  hg, s2, d2 = _t3(q2, deg2, b22, Wg, a_src2, a_dst2)
  s1 = s2.reshape(NPAD)
  d1 = d2.reshape(NPAD)
  m2 = _gat_max_sc(s1, d1, src4, dst4)
  den2 = _gat_denom_sc(s1, d1, m2, src4, dst4)
  w2 = _gat_w_sc(s1, d1, m2, den2, src4, dst4)
  return _t4(w2, hg, bg2)


# speedup vs baseline: 51.4081x; 1.0392x over previous
import functools

import jax
import jax.numpy as jnp
from jax import lax
from jax.experimental import pallas as pl
from jax.experimental.pallas import tpu as pltpu
from jax.experimental.pallas import tpu_sc as plsc

N = 10000
D = 128
NPAD = 10240
NC = 2
NS = 16
L = 16
EB = 128
NB = 82
SB = 96
NBT = 108
ETOT_S = NC * NS * NBT * SB
EPW = NB * EB
ETOT = NC * NS * EPW
RPT = NPAD // NS
BLK = 1024
GRID = NPAD // BLK
NEG = -1e30

_MESH = plsc.VectorSubcoreMesh(
    core_axis_name="c", subcore_axis_name="s", num_cores=NC, num_subcores=NS)


def _fill(ref, n, value):
  vec = jnp.full((L,), value, ref.dtype)
  def body(i, _):
    ref[pl.ds(i * L, L)] = vec
    return 0
  lax.fori_loop(0, n // L, body, 0)


def _zero_shared_slice(acc_sh, zbuf, s):
  nz = zbuf.shape[0]
  for i in range(RPT // nz):
    pltpu.sync_copy(zbuf, acc_sh.at[pl.ds(s * RPT + i * nz, nz)])
  rem = RPT % nz
  if rem:
    pltpu.sync_copy(zbuf.at[pl.ds(0, rem)],
                    acc_sh.at[pl.ds(s * RPT + (RPT // nz) * nz, rem)])


def _copyout_shared_slice(acc_sh, out_hbm, bounce, c, s):
  nz = bounce.shape[0]
  for i in range(RPT // nz):
    sl = pl.ds(s * RPT + i * nz, nz)
    pltpu.sync_copy(acc_sh.at[sl], bounce)
    pltpu.sync_copy(bounce, out_hbm.at[c, sl])
  rem = RPT % nz
  if rem:
    sl = pl.ds(s * RPT + (RPT // nz) * nz, rem)
    pltpu.sync_copy(acc_sh.at[sl], bounce.at[pl.ds(0, rem)])
    pltpu.sync_copy(bounce.at[pl.ds(0, rem)], out_hbm.at[c, sl])


@functools.partial(
    pl.kernel,
    out_type=jax.ShapeDtypeStruct((NC, NPAD), jnp.float32),
    mesh=_MESH,
    scratch_types=[
        pltpu.VMEM((NB, EB), jnp.int32),
        pltpu.VMEM((EB,), jnp.float32),
        pltpu.VMEM((RPT,), jnp.float32),
        pltpu.VMEM_SHARED((NPAD,), jnp.float32),
        pltpu.SemaphoreType.DMA,
    ],
)
def _deg_sc(dst_hbm, out_hbm, idx_v, ones_v, row_v, acc_sh, sadd):
  c = lax.axis_index("c")
  s = lax.axis_index("s")
  _fill(ones_v, EB, 1.0)
  _fill(row_v, RPT, 0.0)
  pltpu.sync_copy(row_v, acc_sh.at[pl.ds(s * RPT, RPT)])
  pltpu.sync_copy(dst_hbm.at[c, s], idx_v)
  plsc.subcore_barrier()

  def body(j, _):
    pltpu.async_copy(ones_v, acc_sh.at[idx_v.at[j]], sadd, add=True)
    return 0
  lax.fori_loop(0, NB, body, 0)

  def drain(j, _):
    pltpu.make_async_copy(ones_v, acc_sh.at[idx_v.at[j]], sadd).wait()
    return 0
  lax.fori_loop(0, NB, drain, 0)

  plsc.subcore_barrier()
  sl = pl.ds(s * RPT, RPT)
  pltpu.sync_copy(acc_sh.at[sl], row_v)
  pltpu.sync_copy(row_v, out_hbm.at[c, sl])


@functools.partial(
    pl.kernel,
    out_type=jax.ShapeDtypeStruct((NC, NPAD, D), jnp.float32),
    mesh=_MESH,
    scratch_types=[
        pltpu.VMEM((4, SB), jnp.int32),
        pltpu.VMEM((4, SB), jnp.int32),
        pltpu.VMEM((SB, D), jnp.float32),
        pltpu.VMEM((SB, D), jnp.float32),
        pltpu.VMEM((SB, D), jnp.float32),
        pltpu.VMEM_SHARED((NPAD, D), jnp.float32),
        pltpu.SemaphoreType.DMA,
        pltpu.SemaphoreType.DMA,
        pltpu.SemaphoreType.DMA,
        pltpu.SemaphoreType.DMA,
        pltpu.SemaphoreType.DMA,
        pltpu.SemaphoreType.DMA,
        pltpu.SemaphoreType.DMA,
    ],
)
def _spmm_sc(p_hbm, src_hbm, dst_hbm, out_hbm,
             srcb, dstb, buf0, buf1, buf2, acc_sh,
             sg0, sg1, sg2, ss0, ss1, ss2, sidx):
  c = lax.axis_index("c")
  s = lax.axis_index("s")
  bufs = (buf0, buf1, buf2)
  sgs = (sg0, sg1, sg2)
  sss = (ss0, ss1, ss2)
  zvec = jnp.zeros((L,), jnp.float32)
  def zrow(r, _):
    for k in range(D // L):
      buf0[r, pl.ds(k * L, L)] = zvec
    return 0
  lax.fori_loop(0, SB, zrow, 0)
  _zero_shared_slice(acc_sh, buf0, s)
  plsc.subcore_barrier()

  pltpu.sync_copy(src_hbm.at[c, s, 0], srcb.at[0])
  pltpu.sync_copy(dst_hbm.at[c, s, 0], dstb.at[0])
  pltpu.sync_copy(src_hbm.at[c, s, 1], srcb.at[1])
  pltpu.sync_copy(dst_hbm.at[c, s, 1], dstb.at[1])
  pltpu.async_copy(src_hbm.at[c, s, 2], srcb.at[2], sidx)
  pltpu.async_copy(dst_hbm.at[c, s, 2], dstb.at[2], sidx)
  pltpu.async_copy(p_hbm.at[srcb.at[0]], buf0, sg0)
  pltpu.async_copy(p_hbm.at[srcb.at[1]], buf1, sg1)

  def stage(t, b):
    bp = (b + 2) % 3
    i0 = lax.rem(t, 4)
    i2 = lax.rem(t + 2, 4)
    i3 = lax.rem(t + 3, 4)
    pltpu.make_async_copy(p_hbm.at[srcb.at[i0]], bufs[b], sgs[b]).wait()
    pltpu.async_copy(bufs[b], acc_sh.at[dstb.at[i0]], sss[b], add=True)

    @pl.when(t >= 1)
    def _():
      pltpu.make_async_copy(bufs[bp], acc_sh.at[dstb.at[i3]], sss[bp]).wait()

    @pl.when(t + 2 < NBT)
    def _():
      pltpu.make_async_copy(src_hbm.at[c, s, t + 2], srcb.at[i2], sidx).wait()
      pltpu.make_async_copy(dst_hbm.at[c, s, t + 2], dstb.at[i2], sidx).wait()

      @pl.when(t + 3 < NBT)
      def _():
        pltpu.async_copy(src_hbm.at[c, s, t + 3], srcb.at[i3], sidx)
        pltpu.async_copy(dst_hbm.at[c, s, t + 3], dstb.at[i3], sidx)
      pltpu.async_copy(p_hbm.at[srcb.at[i2]], bufs[bp], sgs[bp])

  def body(u, _):
    stage(3 * u, 0)
    stage(3 * u + 1, 1)
    stage(3 * u + 2, 2)
    return 0

  lax.fori_loop(0, NBT // 3, body, 0)

  lb = (NBT - 1) % 3
  pltpu.make_async_copy(
      bufs[lb], acc_sh.at[dstb.at[(NBT - 1) % 4]], sss[lb]).wait()

  plsc.subcore_barrier()
  _copyout_shared_slice(acc_sh, out_hbm, buf0, c, s)


def _edge_e(s_v, d_v, src_v, dst_v, j, k):
  si = src_v[j, pl.ds(k * L, L)]
  di = dst_v[j, pl.ds(k * L, L)]
  sv = plsc.load_gather(s_v, [si])
  dv = plsc.load_gather(d_v, [di])
  z = sv + dv
  return di, jnp.where(z >= 0, z, 0.2 * z)


@functools.partial(
    pl.kernel,
    out_type=[
        jax.ShapeDtypeStruct((NC, NPAD), jnp.float32),
        jax.ShapeDtypeStruct((NC, NPAD), jnp.float32),
    ],
    mesh=_MESH,
    scratch_types=[
        pltpu.VMEM((NPAD,), jnp.float32),
        pltpu.VMEM((NPAD,), jnp.float32),
        pltpu.VMEM((NPAD,), jnp.float32),
        pltpu.VMEM((NB, EB), jnp.int32),
        pltpu.VMEM((NB, EB), jnp.int32),
        pltpu.VMEM((RPT,), jnp.float32),
        pltpu.VMEM((RPT,), jnp.float32),
        pltpu.VMEM((2, EB), jnp.float32),
        pltpu.VMEM_SHARED((NS, NPAD), jnp.float32),
        pltpu.VMEM_SHARED((NPAD,), jnp.float32),
        pltpu.SemaphoreType.DMA,
        pltpu.SemaphoreType.DMA,
    ],
    compiler_params=pltpu.CompilerParams(needs_layout_passes=False),
)
def _gat_maxdenom_sc(s_hbm, d_hbm, src_hbm, dst_hbm, m_hbm, den_hbm,
                     s_v, d_v, m_v, src_v, dst_v, acc_v, tmp_v, val_v,
                     stage_sh, den_sh, ssa, ssb):
  c = lax.axis_index("c")
  s = lax.axis_index("s")
  pltpu.sync_copy(s_hbm, s_v)
  pltpu.sync_copy(d_hbm, d_v)
  pltpu.sync_copy(src_hbm.at[c, s], src_v)
  pltpu.sync_copy(dst_hbm.at[c, s], dst_v)
  _fill(m_v, NPAD, NEG)

  def batch(j, _):
    for k in range(EB // L):
      di, e = _edge_e(s_v, d_v, src_v, dst_v, j, k)
      cur = plsc.load_gather(m_v, [di])
      need = e > cur

      def w_body(need):
        plsc.store_scatter(m_v, [di], e, mask=need)
        cur = plsc.load_gather(m_v, [di])
        return e > cur

      lax.while_loop(jnp.any, w_body, need)
    return 0

  lax.fori_loop(0, NB, batch, 0)

  pltpu.sync_copy(m_v, stage_sh.at[s])
  _fill(tmp_v, RPT, 0.0)
  sl = pl.ds(s * RPT, RPT)
  pltpu.sync_copy(tmp_v, den_sh.at[sl])
  plsc.subcore_barrier()
  pltpu.sync_copy(stage_sh.at[0, sl], acc_v)
  for i in range(1, NS):
    pltpu.sync_copy(stage_sh.at[i, sl], tmp_v)
    def red(t, _):
      ds = pl.ds(t * L, L)
      acc_v[ds] = jnp.maximum(acc_v[ds], tmp_v[ds])
      return 0
    lax.fori_loop(0, RPT // L, red, 0)
  pltpu.sync_copy(acc_v, m_hbm.at[c, sl])
  pltpu.sync_copy(acc_v, stage_sh.at[0, sl])
  plsc.subcore_barrier()
  pltpu.sync_copy(stage_sh.at[0], m_v)

  def fill_val(slot, j):
    for k in range(EB // L):
      di, e = _edge_e(s_v, d_v, src_v, dst_v, j, k)
      mv = plsc.load_gather(m_v, [di])
      val_v[slot, pl.ds(k * L, L)] = jnp.exp(e - mv)

  def pair(u, _):
    ja = 2 * u
    jb = 2 * u + 1

    @pl.when(u >= 1)
    def _():
      pltpu.make_async_copy(val_v.at[0], den_sh.at[dst_v.at[ja]], ssa).wait()
    fill_val(0, ja)
    pltpu.async_copy(val_v.at[0], den_sh.at[dst_v.at[ja]], ssa, add=True)

    @pl.when(u >= 1)
    def _():
      pltpu.make_async_copy(val_v.at[1], den_sh.at[dst_v.at[jb]], ssb).wait()
    fill_val(1, jb)
    pltpu.async_copy(val_v.at[1], den_sh.at[dst_v.at[jb]], ssb, add=True)
    return 0

  lax.fori_loop(0, NB // 2, pair, 0)
  pltpu.make_async_copy(val_v.at[0], den_sh.at[dst_v.at[NB - 2]], ssa).wait()
  pltpu.make_async_copy(val_v.at[1], den_sh.at[dst_v.at[NB - 1]], ssb).wait()

  plsc.subcore_barrier()
  pltpu.sync_copy(den_sh.at[sl], acc_v)
  pltpu.sync_copy(acc_v, den_hbm.at[c, sl])


@functools.partial(
    pl.kernel,
    out_type=jax.ShapeDtypeStruct((NC, NPAD), jnp.float32),
    mesh=_MESH,
    scratch_types=[
        pltpu.VMEM((NPAD,), jnp.float32),
        pltpu.VMEM((NPAD,), jnp.float32),
        pltpu.VMEM((NPAD,), jnp.float32),
        pltpu.VMEM((NPAD,), jnp.float32),
        pltpu.VMEM((NPAD,), jnp.float32),
        pltpu.VMEM((NPAD,), jnp.float32),
        pltpu.VMEM((NB, EB), jnp.int32),
        pltpu.VMEM((NB, EB), jnp.int32),
        pltpu.VMEM((2, EB), jnp.float32),
        pltpu.VMEM((RPT,), jnp.float32),
        pltpu.VMEM_SHARED((NPAD,), jnp.float32),
        pltpu.SemaphoreType.DMA,
        pltpu.SemaphoreType.DMA,
    ],
    compiler_params=pltpu.CompilerParams(needs_layout_passes=False),
)
def _gat_w_sc(s_hbm, d_hbm, m_hbm, den_hbm, src_hbm, dst_hbm, out_hbm,
              s_v, d_v, m_v, den_v, t_v, u_v, src_v, dst_v, val_v, row_v,
              acc_sh, ssa, ssb):
  c = lax.axis_index("c")
  s = lax.axis_index("s")
  pltpu.sync_copy(s_hbm, s_v)
  pltpu.sync_copy(d_hbm, d_v)
  pltpu.sync_copy(m_hbm.at[0], m_v)
  pltpu.sync_copy(m_hbm.at[1], t_v)
  pltpu.sync_copy(den_hbm.at[0], den_v)
  pltpu.sync_copy(den_hbm.at[1], u_v)

  def comb(t, _):
    ds = pl.ds(t * L, L)
    m0 = m_v[ds]
    m1 = t_v[ds]
    mm = jnp.maximum(m0, m1)
    den = (jnp.exp(m0 - mm) * den_v[ds] + jnp.exp(m1 - mm) * u_v[ds] + 1e-16)
    m_v[ds] = mm
    den_v[ds] = den
    return 0
  lax.fori_loop(0, NPAD // L, comb, 0)
  pltpu.sync_copy(src_hbm.at[c, s], src_v)
  pltpu.sync_copy(dst_hbm.at[c, s], dst_v)
  _fill(row_v, RPT, 0.0)
  pltpu.sync_copy(row_v, acc_sh.at[pl.ds(s * RPT, RPT)])
  plsc.subcore_barrier()

  def fill_val(slot, j):
    for k in range(EB // L):
      di, e = _edge_e(s_v, d_v, src_v, dst_v, j, k)
      mv = plsc.load_gather(m_v, [di])
      dv = plsc.load_gather(den_v, [di])
      val_v[slot, pl.ds(k * L, L)] = jnp.exp(e - mv) / dv

  def pair(u, _):
    ja = 2 * u
    jb = 2 * u + 1

    @pl.when(u >= 1)
    def _():
      pltpu.make_async_copy(val_v.at[0], acc_sh.at[src_v.at[ja]], ssa).wait()
    fill_val(0, ja)
    pltpu.async_copy(val_v.at[0], acc_sh.at[src_v.at[ja]], ssa, add=True)

    @pl.when(u >= 1)
    def _():
      pltpu.make_async_copy(val_v.at[1], acc_sh.at[src_v.at[jb]], ssb).wait()
    fill_val(1, jb)
    pltpu.async_copy(val_v.at[1], acc_sh.at[src_v.at[jb]], ssb, add=True)
    return 0

  lax.fori_loop(0, NB // 2, pair, 0)
  pltpu.make_async_copy(val_v.at[0], acc_sh.at[src_v.at[NB - 2]], ssa).wait()
  pltpu.make_async_copy(val_v.at[1], acc_sh.at[src_v.at[NB - 1]], ssb).wait()

  plsc.subcore_barrier()
  sl = pl.ds(s * RPT, RPT)
  pltpu.sync_copy(acc_sh.at[sl], row_v)
  pltpu.sync_copy(row_v, out_hbm.at[c, sl])


def _dis_of(deg2):
  deg = deg2[0] + deg2[1]
  return jnp.where(deg > 0, lax.rsqrt(jnp.maximum(deg, 1e-12)), 0.0)


def _t0_body(x_ref, w_ref, b_ref, o_ref):
  o_ref[...] = jax.nn.relu(
      jnp.dot(x_ref[...], w_ref[...], preferred_element_type=jnp.float32)
      + b_ref[...])


def _t1_body(h_ref, w_ref, deg_ref, o_ref):
  dis = _dis_of(deg_ref[...])
  o_ref[...] = dis[:, None] * jnp.dot(
      h_ref[...], w_ref[...], preferred_element_type=jnp.float32)


def _t2_body(q_ref, deg_ref, b_ref, w_ref, o_ref):
  dis = _dis_of(deg_ref[...])
  h = jax.nn.relu(dis[:, None] * (q_ref[0] + q_ref[1]) + b_ref[...])
  o_ref[...] = dis[:, None] * jnp.dot(
      h, w_ref[...], preferred_element_type=jnp.float32)


def _t3_body(q_ref, deg_ref, b_ref, wg_ref, asrc_ref, adst_ref,
             hg_ref, s_ref, d_ref):
  dis = _dis_of(deg_ref[...])
  h = jax.nn.relu(dis[:, None] * (q_ref[0] + q_ref[1]) + b_ref[...])
  hg = jnp.dot(h, wg_ref[...], preferred_element_type=jnp.float32)
  hg_ref[...] = hg
  s_ref[...] = jnp.dot(hg, asrc_ref[...],
                       preferred_element_type=jnp.float32).T
  d_ref[...] = jnp.dot(hg, adst_ref[...],
                       preferred_element_type=jnp.float32).T


def _t4_body(w_ref, hg_ref, bg_ref, o_ref):
  i = pl.program_id(0)
  rows = i * BLK + lax.broadcasted_iota(jnp.int32, (BLK,), 0)
  w = jnp.where(rows < N, w_ref[0] + w_ref[1], 0.0)
  part = jnp.dot(w[None, :], hg_ref[...], preferred_element_type=jnp.float32)

  @pl.when(i == 0)
  def _():
    o_ref[...] = jnp.zeros_like(o_ref)
  o_ref[...] += part

  @pl.when(i == GRID - 1)
  def _():
    o_ref[...] = o_ref[...] * (1.0 / N) + bg_ref[...]


def _row_spec(block=None):
  return pl.BlockSpec((BLK, D) if block is None else block, lambda i: (0, 0))


_SPEC_ROWS = pl.BlockSpec((BLK, D), lambda i: (i, 0))
_SPEC_W = pl.BlockSpec((D, D), lambda i: (0, 0))
_SPEC_B = pl.BlockSpec((1, D), lambda i: (0, 0))
_SPEC_DEG = pl.BlockSpec((NC, BLK), lambda i: (0, i))
_SPEC_Q = pl.BlockSpec((NC, BLK, D), lambda i: (0, i, 0))
_SPEC_VEC = pl.BlockSpec((D, 1), lambda i: (0, 0))
_SPEC_SD = pl.BlockSpec((1, BLK), lambda i: (0, i))


def _t0(xp, W_emb, b_emb):
  return pl.pallas_call(
      _t0_body, grid=(GRID,),
      in_specs=[_SPEC_ROWS, _SPEC_W, _SPEC_B],
      out_specs=_SPEC_ROWS,
      out_shape=jax.ShapeDtypeStruct((NPAD, D), jnp.float32),
  )(xp, W_emb, b_emb)


def _t1(h0, W1, deg2):
  return pl.pallas_call(
      _t1_body, grid=(GRID,),
      in_specs=[_SPEC_ROWS, _SPEC_W, _SPEC_DEG],
      out_specs=_SPEC_ROWS,
      out_shape=jax.ShapeDtypeStruct((NPAD, D), jnp.float32),
  )(h0, W1, deg2)


def _t2(q2, deg2, b, W):
  return pl.pallas_call(
      _t2_body, grid=(GRID,),
      in_specs=[_SPEC_Q, _SPEC_DEG, _SPEC_B, _SPEC_W],
      out_specs=_SPEC_ROWS,
      out_shape=jax.ShapeDtypeStruct((NPAD, D), jnp.float32),
  )(q2, deg2, b, W)


def _t3(q2, deg2, b, Wg, a_src, a_dst):
  return pl.pallas_call(
      _t3_body, grid=(GRID,),
      in_specs=[_SPEC_Q, _SPEC_DEG, _SPEC_B, _SPEC_W, _SPEC_VEC, _SPEC_VEC],
      out_specs=[_SPEC_ROWS, _SPEC_SD, _SPEC_SD],
      out_shape=[
          jax.ShapeDtypeStruct((NPAD, D), jnp.float32),
          jax.ShapeDtypeStruct((1, NPAD), jnp.float32),
          jax.ShapeDtypeStruct((1, NPAD), jnp.float32),
      ],
  )(q2, deg2, b, Wg, a_src, a_dst)


def _t4(w2, hg, bg):
  return pl.pallas_call(
      _t4_body, grid=(GRID,),
      in_specs=[_SPEC_DEG, _SPEC_ROWS, _SPEC_B],
      out_specs=_SPEC_B,
      out_shape=jax.ShapeDtypeStruct((1, D), jnp.float32),
  )(w2, hg, bg)


def kernel(x, edge_index, W_emb, b_emb, W1, b1, W2, b2, Wg, a_src, a_dst, bg):
  ei = edge_index.astype(jnp.int32)
  loop = jnp.arange(N, dtype=jnp.int32)

  def padded(row, tot):
    npad_e = tot - (ei.shape[1] + N)
    padidx = N + 200 + (jnp.arange(npad_e, dtype=jnp.int32) % 8)
    return jnp.concatenate([row, loop, padidx])

  src4 = padded(ei[0], ETOT).reshape(NC, NS, NB, EB)
  dst4 = padded(ei[1], ETOT).reshape(NC, NS, NB, EB)
  src5 = padded(ei[0], ETOT_S).reshape(NC, NS, NBT, SB)
  dst5 = padded(ei[1], ETOT_S).reshape(NC, NS, NBT, SB)
  xp = jnp.zeros((NPAD, D), jnp.float32).at[:N].set(x)
  b_emb2 = b_emb.reshape(1, D)
  b12 = b1.reshape(1, D)
  b22 = b2.reshape(1, D)
  bg2 = bg.reshape(1, D)
  a_src2 = a_src.reshape(D, 1)
  a_dst2 = a_dst.reshape(D, 1)

  deg2 = _deg_sc(dst4)
  h0 = _t0(xp, W_emb, b_emb2)
  p1 = _t1(h0, W1, deg2)
  q1 = _spmm_sc(p1, src5, dst5)
  p2 = _t2(q1, deg2, b12, W2)
  q2 = _spmm_sc(p2, src5, dst5)
  hg, s2, d2 = _t3(q2, deg2, b22, Wg, a_src2, a_dst2)
  s1 = s2.reshape(NPAD)
  d1 = d2.reshape(NPAD)
  m2, den2 = _gat_maxdenom_sc(s1, d1, src4, dst4)
  w2 = _gat_w_sc(s1, d1, m2, den2, src4, dst4)
  return _t4(w2, hg, bg2)


# speedup vs baseline: 53.2293x; 1.0354x over previous
import functools

import jax
import jax.numpy as jnp
from jax import lax
from jax.experimental import pallas as pl
from jax.experimental.pallas import tpu as pltpu
from jax.experimental.pallas import tpu_sc as plsc

N = 10000
D = 128
NPAD = 10240
NC = 2
NS = 16
L = 16
EB = 128
NB = 82
SB = 96
NBT = 105
ETOT_S = NC * NS * NBT * SB
EPW = NB * EB
ETOT = NC * NS * EPW
RPT = NPAD // NS
BLK = 1024
GRID = NPAD // BLK
NEG = -1e30

_MESH = plsc.VectorSubcoreMesh(
    core_axis_name="c", subcore_axis_name="s", num_cores=NC, num_subcores=NS)


def _fill(ref, n, value):
  vec = jnp.full((L,), value, ref.dtype)
  def body(i, _):
    ref[pl.ds(i * L, L)] = vec
    return 0
  lax.fori_loop(0, n // L, body, 0)


def _zero_shared_slice(acc_sh, zbuf, s):
  nz = zbuf.shape[0]
  for i in range(RPT // nz):
    pltpu.sync_copy(zbuf, acc_sh.at[pl.ds(s * RPT + i * nz, nz)])
  rem = RPT % nz
  if rem:
    pltpu.sync_copy(zbuf.at[pl.ds(0, rem)],
                    acc_sh.at[pl.ds(s * RPT + (RPT // nz) * nz, rem)])


def _copyout_shared_slice(acc_sh, out_hbm, bounce, c, s):
  nz = bounce.shape[0]
  for i in range(RPT // nz):
    sl = pl.ds(s * RPT + i * nz, nz)
    pltpu.sync_copy(acc_sh.at[sl], bounce)
    pltpu.sync_copy(bounce, out_hbm.at[c, sl])
  rem = RPT % nz
  if rem:
    sl = pl.ds(s * RPT + (RPT // nz) * nz, rem)
    pltpu.sync_copy(acc_sh.at[sl], bounce.at[pl.ds(0, rem)])
    pltpu.sync_copy(bounce.at[pl.ds(0, rem)], out_hbm.at[c, sl])


@functools.partial(
    pl.kernel,
    out_type=jax.ShapeDtypeStruct((NC, NPAD), jnp.float32),
    mesh=_MESH,
    scratch_types=[
        pltpu.VMEM((NB, EB), jnp.int32),
        pltpu.VMEM((EB,), jnp.float32),
        pltpu.VMEM((RPT,), jnp.float32),
        pltpu.VMEM_SHARED((NPAD,), jnp.float32),
        pltpu.SemaphoreType.DMA,
    ],
)
def _deg_sc(dst_hbm, out_hbm, idx_v, ones_v, row_v, acc_sh, sadd):
  c = lax.axis_index("c")
  s = lax.axis_index("s")
  _fill(ones_v, EB, 1.0)
  _fill(row_v, RPT, 0.0)
  pltpu.sync_copy(row_v, acc_sh.at[pl.ds(s * RPT, RPT)])
  pltpu.sync_copy(dst_hbm.at[c, s], idx_v)
  plsc.subcore_barrier()

  def body(j, _):
    pltpu.async_copy(ones_v, acc_sh.at[idx_v.at[j]], sadd, add=True)
    return 0
  lax.fori_loop(0, NB, body, 0)

  def drain(j, _):
    pltpu.make_async_copy(ones_v, acc_sh.at[idx_v.at[j]], sadd).wait()
    return 0
  lax.fori_loop(0, NB, drain, 0)

  plsc.subcore_barrier()
  sl = pl.ds(s * RPT, RPT)
  pltpu.sync_copy(acc_sh.at[sl], row_v)
  pltpu.sync_copy(row_v, out_hbm.at[c, sl])


@functools.partial(
    pl.kernel,
    out_type=jax.ShapeDtypeStruct((NC, NPAD, D), jnp.float32),
    mesh=_MESH,
    scratch_types=[
        pltpu.VMEM((4, SB), jnp.int32),
        pltpu.VMEM((4, SB), jnp.int32),
        pltpu.VMEM((SB, D), jnp.float32),
        pltpu.VMEM((SB, D), jnp.float32),
        pltpu.VMEM((SB, D), jnp.float32),
        pltpu.VMEM_SHARED((NPAD, D), jnp.float32),
        pltpu.SemaphoreType.DMA,
        pltpu.SemaphoreType.DMA,
        pltpu.SemaphoreType.DMA,
        pltpu.SemaphoreType.DMA,
        pltpu.SemaphoreType.DMA,
        pltpu.SemaphoreType.DMA,
        pltpu.SemaphoreType.DMA,
    ],
)
def _spmm_sc(p_hbm, src_hbm, dst_hbm, out_hbm,
             srcb, dstb, buf0, buf1, buf2, acc_sh,
             sg0, sg1, sg2, ss0, ss1, ss2, sidx):
  c = lax.axis_index("c")
  s = lax.axis_index("s")
  bufs = (buf0, buf1, buf2)
  sgs = (sg0, sg1, sg2)
  sss = (ss0, ss1, ss2)
  zvec = jnp.zeros((L,), jnp.float32)
  def zrow(r, _):
    for k in range(D // L):
      buf0[r, pl.ds(k * L, L)] = zvec
    return 0
  lax.fori_loop(0, SB, zrow, 0)
  _zero_shared_slice(acc_sh, buf0, s)
  plsc.subcore_barrier()

  pltpu.sync_copy(src_hbm.at[c, s, 0], srcb.at[0])
  pltpu.sync_copy(dst_hbm.at[c, s, 0], dstb.at[0])
  pltpu.sync_copy(src_hbm.at[c, s, 1], srcb.at[1])
  pltpu.sync_copy(dst_hbm.at[c, s, 1], dstb.at[1])
  pltpu.async_copy(src_hbm.at[c, s, 2], srcb.at[2], sidx)
  pltpu.async_copy(dst_hbm.at[c, s, 2], dstb.at[2], sidx)
  pltpu.async_copy(p_hbm.at[srcb.at[0]], buf0, sg0)
  pltpu.async_copy(p_hbm.at[srcb.at[1]], buf1, sg1)

  def stage(t, b):
    bp = (b + 2) % 3
    i0 = lax.rem(t, 4)
    i2 = lax.rem(t + 2, 4)
    i3 = lax.rem(t + 3, 4)
    pltpu.make_async_copy(p_hbm.at[srcb.at[i0]], bufs[b], sgs[b]).wait()
    pltpu.async_copy(bufs[b], acc_sh.at[dstb.at[i0]], sss[b], add=True)

    @pl.when(t >= 1)
    def _():
      pltpu.make_async_copy(bufs[bp], acc_sh.at[dstb.at[i3]], sss[bp]).wait()

    @pl.when(t + 2 < NBT)
    def _():
      pltpu.make_async_copy(src_hbm.at[c, s, t + 2], srcb.at[i2], sidx).wait()
      pltpu.make_async_copy(dst_hbm.at[c, s, t + 2], dstb.at[i2], sidx).wait()

      @pl.when(t + 3 < NBT)
      def _():
        pltpu.async_copy(src_hbm.at[c, s, t + 3], srcb.at[i3], sidx)
        pltpu.async_copy(dst_hbm.at[c, s, t + 3], dstb.at[i3], sidx)
      pltpu.async_copy(p_hbm.at[srcb.at[i2]], bufs[bp], sgs[bp])

  def body(u, _):
    stage(3 * u, 0)
    stage(3 * u + 1, 1)
    stage(3 * u + 2, 2)
    return 0

  assert NBT % 3 == 0
  lax.fori_loop(0, NBT // 3, body, 0)

  lb = (NBT - 1) % 3
  pltpu.make_async_copy(
      bufs[lb], acc_sh.at[dstb.at[(NBT - 1) % 4]], sss[lb]).wait()

  plsc.subcore_barrier()
  _copyout_shared_slice(acc_sh, out_hbm, buf0, c, s)


def _edge_e(s_v, d_v, src_v, dst_v, j, k):
  si = src_v[j, pl.ds(k * L, L)]
  di = dst_v[j, pl.ds(k * L, L)]
  sv = plsc.load_gather(s_v, [si])
  dv = plsc.load_gather(d_v, [di])
  z = sv + dv
  return di, jnp.where(z >= 0, z, 0.2 * z)


@functools.partial(
    pl.kernel,
    out_type=[
        jax.ShapeDtypeStruct((NC, NPAD), jnp.float32),
        jax.ShapeDtypeStruct((NC, NPAD), jnp.float32),
    ],
    mesh=_MESH,
    scratch_types=[
        pltpu.VMEM((NPAD,), jnp.float32),
        pltpu.VMEM((NPAD,), jnp.float32),
        pltpu.VMEM((NPAD,), jnp.float32),
        pltpu.VMEM((NB, EB), jnp.int32),
        pltpu.VMEM((NB, EB), jnp.int32),
        pltpu.VMEM((RPT,), jnp.float32),
        pltpu.VMEM((RPT,), jnp.float32),
        pltpu.VMEM((2, EB), jnp.float32),
        pltpu.VMEM((EPW,), jnp.float32),
        pltpu.VMEM_SHARED((NS, NPAD), jnp.float32),
        pltpu.VMEM_SHARED((NPAD,), jnp.float32),
        pltpu.SemaphoreType.DMA,
        pltpu.SemaphoreType.DMA,
    ],
    compiler_params=pltpu.CompilerParams(needs_layout_passes=False),
)
def _gat_maxdenom_sc(s_hbm, d_hbm, src_hbm, dst_hbm, m_hbm, den_hbm,
                     s_v, d_v, m_v, src_v, dst_v, acc_v, tmp_v, val_v,
                     e_v, stage_sh, den_sh, ssa, ssb):
  c = lax.axis_index("c")
  s = lax.axis_index("s")
  pltpu.sync_copy(s_hbm, s_v)
  pltpu.sync_copy(d_hbm, d_v)
  pltpu.sync_copy(src_hbm.at[c, s], src_v)
  pltpu.sync_copy(dst_hbm.at[c, s], dst_v)
  _fill(m_v, NPAD, NEG)

  def batch(j, _):
    for k in range(EB // L):
      di, e = _edge_e(s_v, d_v, src_v, dst_v, j, k)
      e_v[pl.ds(j * EB + k * L, L)] = e
      cur = plsc.load_gather(m_v, [di])
      need = e > cur

      def w_body(need):
        plsc.store_scatter(m_v, [di], e, mask=need)
        cur = plsc.load_gather(m_v, [di])
        return e > cur

      lax.while_loop(jnp.any, w_body, need)
    return 0

  lax.fori_loop(0, NB, batch, 0)

  pltpu.sync_copy(m_v, stage_sh.at[s])
  _fill(tmp_v, RPT, 0.0)
  sl = pl.ds(s * RPT, RPT)
  pltpu.sync_copy(tmp_v, den_sh.at[sl])
  plsc.subcore_barrier()
  pltpu.sync_copy(stage_sh.at[0, sl], acc_v)
  for i in range(1, NS):
    pltpu.sync_copy(stage_sh.at[i, sl], tmp_v)
    def red(t, _):
      ds = pl.ds(t * L, L)
      acc_v[ds] = jnp.maximum(acc_v[ds], tmp_v[ds])
      return 0
    lax.fori_loop(0, RPT // L, red, 0)
  pltpu.sync_copy(acc_v, m_hbm.at[c, sl])
  pltpu.sync_copy(acc_v, stage_sh.at[0, sl])
  plsc.subcore_barrier()
  pltpu.sync_copy(stage_sh.at[0], m_v)

  def fill_val(slot, j):
    for k in range(EB // L):
      di = dst_v[j, pl.ds(k * L, L)]
      e = e_v[pl.ds(j * EB + k * L, L)]
      mv = plsc.load_gather(m_v, [di])
      val_v[slot, pl.ds(k * L, L)] = jnp.exp(e - mv)

  def pair(u, _):
    ja = 2 * u
    jb = 2 * u + 1

    @pl.when(u >= 1)
    def _():
      pltpu.make_async_copy(val_v.at[0], den_sh.at[dst_v.at[ja]], ssa).wait()
    fill_val(0, ja)
    pltpu.async_copy(val_v.at[0], den_sh.at[dst_v.at[ja]], ssa, add=True)

    @pl.when(u >= 1)
    def _():
      pltpu.make_async_copy(val_v.at[1], den_sh.at[dst_v.at[jb]], ssb).wait()
    fill_val(1, jb)
    pltpu.async_copy(val_v.at[1], den_sh.at[dst_v.at[jb]], ssb, add=True)
    return 0

  lax.fori_loop(0, NB // 2, pair, 0)
  pltpu.make_async_copy(val_v.at[0], den_sh.at[dst_v.at[NB - 2]], ssa).wait()
  pltpu.make_async_copy(val_v.at[1], den_sh.at[dst_v.at[NB - 1]], ssb).wait()

  plsc.subcore_barrier()
  pltpu.sync_copy(den_sh.at[sl], acc_v)
  pltpu.sync_copy(acc_v, den_hbm.at[c, sl])


@functools.partial(
    pl.kernel,
    out_type=jax.ShapeDtypeStruct((NC, NPAD), jnp.float32),
    mesh=_MESH,
    scratch_types=[
        pltpu.VMEM((NPAD,), jnp.float32),
        pltpu.VMEM((NPAD,), jnp.float32),
        pltpu.VMEM((NPAD,), jnp.float32),
        pltpu.VMEM((NPAD,), jnp.float32),
        pltpu.VMEM((NPAD,), jnp.float32),
        pltpu.VMEM((NPAD,), jnp.float32),
        pltpu.VMEM((NB, EB), jnp.int32),
        pltpu.VMEM((NB, EB), jnp.int32),
        pltpu.VMEM((2, EB), jnp.float32),
        pltpu.VMEM((RPT,), jnp.float32),
        pltpu.VMEM_SHARED((NPAD,), jnp.float32),
        pltpu.SemaphoreType.DMA,
        pltpu.SemaphoreType.DMA,
    ],
    compiler_params=pltpu.CompilerParams(needs_layout_passes=False),
)
def _gat_w_sc(s_hbm, d_hbm, m_hbm, den_hbm, src_hbm, dst_hbm, out_hbm,
              s_v, d_v, m_v, den_v, t_v, u_v, src_v, dst_v, val_v, row_v,
              acc_sh, ssa, ssb):
  c = lax.axis_index("c")
  s = lax.axis_index("s")
  pltpu.sync_copy(s_hbm, s_v)
  pltpu.sync_copy(d_hbm, d_v)
  pltpu.sync_copy(m_hbm.at[0], m_v)
  pltpu.sync_copy(m_hbm.at[1], t_v)
  pltpu.sync_copy(den_hbm.at[0], den_v)
  pltpu.sync_copy(den_hbm.at[1], u_v)

  def comb(t, _):
    ds = pl.ds(t * L, L)
    m0 = m_v[ds]
    m1 = t_v[ds]
    mm = jnp.maximum(m0, m1)
    den = (jnp.exp(m0 - mm) * den_v[ds] + jnp.exp(m1 - mm) * u_v[ds] + 1e-16)
    m_v[ds] = mm
    den_v[ds] = den
    return 0
  lax.fori_loop(0, NPAD // L, comb, 0)
  pltpu.sync_copy(src_hbm.at[c, s], src_v)
  pltpu.sync_copy(dst_hbm.at[c, s], dst_v)
  _fill(row_v, RPT, 0.0)
  pltpu.sync_copy(row_v, acc_sh.at[pl.ds(s * RPT, RPT)])
  plsc.subcore_barrier()

  def fill_val(slot, j):
    for k in range(EB // L):
      di, e = _edge_e(s_v, d_v, src_v, dst_v, j, k)
      mv = plsc.load_gather(m_v, [di])
      dv = plsc.load_gather(den_v, [di])
      val_v[slot, pl.ds(k * L, L)] = jnp.exp(e - mv) / dv

  def pair(u, _):
    ja = 2 * u
    jb = 2 * u + 1

    @pl.when(u >= 1)
    def _():
      pltpu.make_async_copy(val_v.at[0], acc_sh.at[src_v.at[ja]], ssa).wait()
    fill_val(0, ja)
    pltpu.async_copy(val_v.at[0], acc_sh.at[src_v.at[ja]], ssa, add=True)

    @pl.when(u >= 1)
    def _():
      pltpu.make_async_copy(val_v.at[1], acc_sh.at[src_v.at[jb]], ssb).wait()
    fill_val(1, jb)
    pltpu.async_copy(val_v.at[1], acc_sh.at[src_v.at[jb]], ssb, add=True)
    return 0

  lax.fori_loop(0, NB // 2, pair, 0)
  pltpu.make_async_copy(val_v.at[0], acc_sh.at[src_v.at[NB - 2]], ssa).wait()
  pltpu.make_async_copy(val_v.at[1], acc_sh.at[src_v.at[NB - 1]], ssb).wait()

  plsc.subcore_barrier()
  sl = pl.ds(s * RPT, RPT)
  pltpu.sync_copy(acc_sh.at[sl], row_v)
  pltpu.sync_copy(row_v, out_hbm.at[c, sl])


def _dis_of(deg2):
  deg = deg2[0] + deg2[1]
  return jnp.where(deg > 0, lax.rsqrt(jnp.maximum(deg, 1e-12)), 0.0)


def _t01_body(x_ref, wemb_ref, bemb_ref, w1_ref, deg_ref, o_ref):
  h0 = jax.nn.relu(
      jnp.dot(x_ref[...], wemb_ref[...], preferred_element_type=jnp.float32)
      + bemb_ref[...])
  dis = _dis_of(deg_ref[...])
  o_ref[...] = dis[:, None] * jnp.dot(
      h0, w1_ref[...], preferred_element_type=jnp.float32)


def _t2_body(q_ref, p_ref, deg_ref, b_ref, w_ref, o_ref):
  dis = _dis_of(deg_ref[...])
  h = jax.nn.relu(
      dis[:, None] * (q_ref[0] + q_ref[1] + p_ref[...]) + b_ref[...])
  o_ref[...] = dis[:, None] * jnp.dot(
      h, w_ref[...], preferred_element_type=jnp.float32)


def _t3_body(q_ref, p_ref, deg_ref, b_ref, wg_ref, asrc_ref, adst_ref,
             hg_ref, s_ref, d_ref):
  dis = _dis_of(deg_ref[...])
  h = jax.nn.relu(
      dis[:, None] * (q_ref[0] + q_ref[1] + p_ref[...]) + b_ref[...])
  hg = jnp.dot(h, wg_ref[...], preferred_element_type=jnp.float32)
  hg_ref[...] = hg
  s_ref[...] = jnp.dot(hg, asrc_ref[...],
                       preferred_element_type=jnp.float32).T
  d_ref[...] = jnp.dot(hg, adst_ref[...],
                       preferred_element_type=jnp.float32).T


def _t4_body(w_ref, hg_ref, bg_ref, o_ref):
  i = pl.program_id(0)
  rows = i * BLK + lax.broadcasted_iota(jnp.int32, (BLK,), 0)
  w = jnp.where(rows < N, w_ref[0] + w_ref[1], 0.0)
  part = jnp.dot(w[None, :], hg_ref[...], preferred_element_type=jnp.float32)

  @pl.when(i == 0)
  def _():
    o_ref[...] = jnp.zeros_like(o_ref)
  o_ref[...] += part

  @pl.when(i == GRID - 1)
  def _():
    o_ref[...] = o_ref[...] * (1.0 / N) + bg_ref[...]


def _row_spec(block=None):
  return pl.BlockSpec((BLK, D) if block is None else block, lambda i: (0, 0))


_SPEC_ROWS = pl.BlockSpec((BLK, D), lambda i: (i, 0))
_SPEC_W = pl.BlockSpec((D, D), lambda i: (0, 0))
_SPEC_B = pl.BlockSpec((1, D), lambda i: (0, 0))
_SPEC_DEG = pl.BlockSpec((NC, BLK), lambda i: (0, i))
_SPEC_Q = pl.BlockSpec((NC, BLK, D), lambda i: (0, i, 0))
_SPEC_VEC = pl.BlockSpec((D, 1), lambda i: (0, 0))
_SPEC_SD = pl.BlockSpec((1, BLK), lambda i: (0, i))


def _t01(xp, W_emb, b_emb, W1, deg2):
  return pl.pallas_call(
      _t01_body, grid=(GRID,),
      in_specs=[_SPEC_ROWS, _SPEC_W, _SPEC_B, _SPEC_W, _SPEC_DEG],
      out_specs=_SPEC_ROWS,
      out_shape=jax.ShapeDtypeStruct((NPAD, D), jnp.float32),
  )(xp, W_emb, b_emb, W1, deg2)


def _t2(q2, p, deg2, b, W):
  return pl.pallas_call(
      _t2_body, grid=(GRID,),
      in_specs=[_SPEC_Q, _SPEC_ROWS, _SPEC_DEG, _SPEC_B, _SPEC_W],
      out_specs=_SPEC_ROWS,
      out_shape=jax.ShapeDtypeStruct((NPAD, D), jnp.float32),
  )(q2, p, deg2, b, W)


def _t3(q2, p, deg2, b, Wg, a_src, a_dst):
  return pl.pallas_call(
      _t3_body, grid=(GRID,),
      in_specs=[_SPEC_Q, _SPEC_ROWS, _SPEC_DEG, _SPEC_B, _SPEC_W,
                _SPEC_VEC, _SPEC_VEC],
      out_specs=[_SPEC_ROWS, _SPEC_SD, _SPEC_SD],
      out_shape=[
          jax.ShapeDtypeStruct((NPAD, D), jnp.float32),
          jax.ShapeDtypeStruct((1, NPAD), jnp.float32),
          jax.ShapeDtypeStruct((1, NPAD), jnp.float32),
      ],
  )(q2, p, deg2, b, Wg, a_src, a_dst)


def _t4(w2, hg, bg):
  return pl.pallas_call(
      _t4_body, grid=(GRID,),
      in_specs=[_SPEC_DEG, _SPEC_ROWS, _SPEC_B],
      out_specs=_SPEC_B,
      out_shape=jax.ShapeDtypeStruct((1, D), jnp.float32),
  )(w2, hg, bg)


def kernel(x, edge_index, W_emb, b_emb, W1, b1, W2, b2, Wg, a_src, a_dst, bg):
  ei = edge_index.astype(jnp.int32)
  loop = jnp.arange(N, dtype=jnp.int32)

  def padded(row, tot, with_loops):
    parts = [row, loop] if with_loops else [row]
    npad_e = tot - sum(p.shape[0] for p in parts)
    padidx = N + 200 + (jnp.arange(npad_e, dtype=jnp.int32) % 8)
    return jnp.concatenate(parts + [padidx])

  src4 = padded(ei[0], ETOT, True).reshape(NC, NS, NB, EB)
  dst4 = padded(ei[1], ETOT, True).reshape(NC, NS, NB, EB)
  src5 = padded(ei[0], ETOT_S, False).reshape(NC, NS, NBT, SB)
  dst5 = padded(ei[1], ETOT_S, False).reshape(NC, NS, NBT, SB)
  xp = jnp.zeros((NPAD, D), jnp.float32).at[:N].set(x)
  b_emb2 = b_emb.reshape(1, D)
  b12 = b1.reshape(1, D)
  b22 = b2.reshape(1, D)
  bg2 = bg.reshape(1, D)
  a_src2 = a_src.reshape(D, 1)
  a_dst2 = a_dst.reshape(D, 1)

  deg2 = _deg_sc(dst4)
  p1 = _t01(xp, W_emb, b_emb2, W1, deg2)
  q1 = _spmm_sc(p1, src5, dst5)
  p2 = _t2(q1, p1, deg2, b12, W2)
  q2 = _spmm_sc(p2, src5, dst5)
  hg, s2, d2 = _t3(q2, p2, deg2, b22, Wg, a_src2, a_dst2)
  s1 = s2.reshape(NPAD)
  d1 = d2.reshape(NPAD)
  m2, den2 = _gat_maxdenom_sc(s1, d1, src4, dst4)
  w2 = _gat_w_sc(s1, d1, m2, den2, src4, dst4)
  return _t4(w2, hg, bg2)


# speedup vs baseline: 54.4225x; 1.0224x over previous
import functools

import jax
import jax.numpy as jnp
from jax import lax
from jax.experimental import pallas as pl
from jax.experimental.pallas import tpu as pltpu
from jax.experimental.pallas import tpu_sc as plsc

N = 10000
D = 128
NPAD = 10240
NC = 2
NS = 16
L = 16
EB = 128
NB = 82
SB = 96
NBT = 105
ETOT_S = NC * NS * NBT * SB
EPW = NB * EB
ETOT = NC * NS * EPW
RPT = NPAD // NS
BLK = 2048
GRID = NPAD // BLK
NEG = -1e30

_MESH = plsc.VectorSubcoreMesh(
    core_axis_name="c", subcore_axis_name="s", num_cores=NC, num_subcores=NS)


def _fill(ref, n, value):
  vec = jnp.full((L,), value, ref.dtype)
  def body(i, _):
    ref[pl.ds(i * L, L)] = vec
    return 0
  lax.fori_loop(0, n // L, body, 0)


def _zero_shared_slice(acc_sh, zbuf, s):
  nz = zbuf.shape[0]
  for i in range(RPT // nz):
    pltpu.sync_copy(zbuf, acc_sh.at[pl.ds(s * RPT + i * nz, nz)])
  rem = RPT % nz
  if rem:
    pltpu.sync_copy(zbuf.at[pl.ds(0, rem)],
                    acc_sh.at[pl.ds(s * RPT + (RPT // nz) * nz, rem)])


def _copyout_shared_slice(acc_sh, out_hbm, bounce, c, s):
  nz = bounce.shape[0]
  for i in range(RPT // nz):
    sl = pl.ds(s * RPT + i * nz, nz)
    pltpu.sync_copy(acc_sh.at[sl], bounce)
    pltpu.sync_copy(bounce, out_hbm.at[c, sl])
  rem = RPT % nz
  if rem:
    sl = pl.ds(s * RPT + (RPT // nz) * nz, rem)
    pltpu.sync_copy(acc_sh.at[sl], bounce.at[pl.ds(0, rem)])
    pltpu.sync_copy(bounce.at[pl.ds(0, rem)], out_hbm.at[c, sl])


@functools.partial(
    pl.kernel,
    out_type=jax.ShapeDtypeStruct((NC, NPAD), jnp.float32),
    mesh=_MESH,
    scratch_types=[
        pltpu.VMEM((NB, EB), jnp.int32),
        pltpu.VMEM((EB,), jnp.float32),
        pltpu.VMEM((RPT,), jnp.float32),
        pltpu.VMEM_SHARED((NPAD,), jnp.float32),
        pltpu.SemaphoreType.DMA,
    ],
)
def _deg_sc(dst_hbm, out_hbm, idx_v, ones_v, row_v, acc_sh, sadd):
  c = lax.axis_index("c")
  s = lax.axis_index("s")
  _fill(ones_v, EB, 1.0)
  _fill(row_v, RPT, 0.0)
  pltpu.sync_copy(row_v, acc_sh.at[pl.ds(s * RPT, RPT)])
  pltpu.sync_copy(dst_hbm.at[c, s], idx_v)
  plsc.subcore_barrier()

  def body(j, _):
    pltpu.async_copy(ones_v, acc_sh.at[idx_v.at[j]], sadd, add=True)
    return 0
  lax.fori_loop(0, NB, body, 0)

  def drain(j, _):
    pltpu.make_async_copy(ones_v, acc_sh.at[idx_v.at[j]], sadd).wait()
    return 0
  lax.fori_loop(0, NB, drain, 0)

  plsc.subcore_barrier()
  sl = pl.ds(s * RPT, RPT)
  pltpu.sync_copy(acc_sh.at[sl], row_v)
  pltpu.sync_copy(row_v, out_hbm.at[c, sl])


@functools.partial(
    pl.kernel,
    out_type=jax.ShapeDtypeStruct((NC, NPAD, D), jnp.float32),
    mesh=_MESH,
    scratch_types=[
        pltpu.VMEM((4, SB), jnp.int32),
        pltpu.VMEM((4, SB), jnp.int32),
        pltpu.VMEM((SB, D), jnp.float32),
        pltpu.VMEM((SB, D), jnp.float32),
        pltpu.VMEM((SB, D), jnp.float32),
        pltpu.VMEM_SHARED((NPAD, D), jnp.float32),
        pltpu.SemaphoreType.DMA,
        pltpu.SemaphoreType.DMA,
        pltpu.SemaphoreType.DMA,
        pltpu.SemaphoreType.DMA,
        pltpu.SemaphoreType.DMA,
        pltpu.SemaphoreType.DMA,
        pltpu.SemaphoreType.DMA,
    ],
)
def _spmm_sc(p_hbm, src_hbm, dst_hbm, out_hbm,
             srcb, dstb, buf0, buf1, buf2, acc_sh,
             sg0, sg1, sg2, ss0, ss1, ss2, sidx):
  c = lax.axis_index("c")
  s = lax.axis_index("s")
  bufs = (buf0, buf1, buf2)
  sgs = (sg0, sg1, sg2)
  sss = (ss0, ss1, ss2)
  zvec = jnp.zeros((L,), jnp.float32)
  def zrow(r, _):
    for k in range(D // L):
      buf0[r, pl.ds(k * L, L)] = zvec
    return 0
  lax.fori_loop(0, SB, zrow, 0)
  _zero_shared_slice(acc_sh, buf0, s)
  plsc.subcore_barrier()

  pltpu.sync_copy(src_hbm.at[c, s, 0], srcb.at[0])
  pltpu.sync_copy(dst_hbm.at[c, s, 0], dstb.at[0])
  pltpu.sync_copy(src_hbm.at[c, s, 1], srcb.at[1])
  pltpu.sync_copy(dst_hbm.at[c, s, 1], dstb.at[1])
  pltpu.async_copy(src_hbm.at[c, s, 2], srcb.at[2], sidx)
  pltpu.async_copy(dst_hbm.at[c, s, 2], dstb.at[2], sidx)
  pltpu.async_copy(p_hbm.at[srcb.at[0]], buf0, sg0)
  pltpu.async_copy(p_hbm.at[srcb.at[1]], buf1, sg1)

  def stage(t, b):
    bp = (b + 2) % 3
    i0 = lax.rem(t, 4)
    i2 = lax.rem(t + 2, 4)
    i3 = lax.rem(t + 3, 4)
    pltpu.make_async_copy(p_hbm.at[srcb.at[i0]], bufs[b], sgs[b]).wait()
    pltpu.async_copy(bufs[b], acc_sh.at[dstb.at[i0]], sss[b], add=True)

    @pl.when(t >= 1)
    def _():
      pltpu.make_async_copy(bufs[bp], acc_sh.at[dstb.at[i3]], sss[bp]).wait()

    @pl.when(t + 2 < NBT)
    def _():
      pltpu.make_async_copy(src_hbm.at[c, s, t + 2], srcb.at[i2], sidx).wait()
      pltpu.make_async_copy(dst_hbm.at[c, s, t + 2], dstb.at[i2], sidx).wait()

      @pl.when(t + 3 < NBT)
      def _():
        pltpu.async_copy(src_hbm.at[c, s, t + 3], srcb.at[i3], sidx)
        pltpu.async_copy(dst_hbm.at[c, s, t + 3], dstb.at[i3], sidx)
      pltpu.async_copy(p_hbm.at[srcb.at[i2]], bufs[bp], sgs[bp])

  def body(u, _):
    stage(3 * u, 0)
    stage(3 * u + 1, 1)
    stage(3 * u + 2, 2)
    return 0

  assert NBT % 3 == 0
  lax.fori_loop(0, NBT // 3, body, 0)

  lb = (NBT - 1) % 3
  pltpu.make_async_copy(
      bufs[lb], acc_sh.at[dstb.at[(NBT - 1) % 4]], sss[lb]).wait()

  plsc.subcore_barrier()
  _copyout_shared_slice(acc_sh, out_hbm, buf0, c, s)


def _edge_e(s_v, d_v, src_v, dst_v, j, k):
  si = src_v[j, pl.ds(k * L, L)]
  di = dst_v[j, pl.ds(k * L, L)]
  sv = plsc.load_gather(s_v, [si])
  dv = plsc.load_gather(d_v, [di])
  z = sv + dv
  return di, jnp.where(z >= 0, z, 0.2 * z)


@functools.partial(
    pl.kernel,
    out_type=[
        jax.ShapeDtypeStruct((NC, NPAD), jnp.float32),
        jax.ShapeDtypeStruct((NC, NPAD), jnp.float32),
    ],
    mesh=_MESH,
    scratch_types=[
        pltpu.VMEM((NPAD,), jnp.float32),
        pltpu.VMEM((NPAD,), jnp.float32),
        pltpu.VMEM((NPAD,), jnp.float32),
        pltpu.VMEM((NB, EB), jnp.int32),
        pltpu.VMEM((NB, EB), jnp.int32),
        pltpu.VMEM((RPT,), jnp.float32),
        pltpu.VMEM((RPT,), jnp.float32),
        pltpu.VMEM((2, EB), jnp.float32),
        pltpu.VMEM((EPW,), jnp.float32),
        pltpu.VMEM_SHARED((NS, NPAD), jnp.float32),
        pltpu.VMEM_SHARED((NPAD,), jnp.float32),
        pltpu.SemaphoreType.DMA,
        pltpu.SemaphoreType.DMA,
    ],
    compiler_params=pltpu.CompilerParams(needs_layout_passes=False),
)
def _gat_maxdenom_sc(s_hbm, d_hbm, src_hbm, dst_hbm, m_hbm, den_hbm,
                     s_v, d_v, m_v, src_v, dst_v, acc_v, tmp_v, val_v,
                     e_v, stage_sh, den_sh, ssa, ssb):
  c = lax.axis_index("c")
  s = lax.axis_index("s")
  pltpu.sync_copy(s_hbm, s_v)
  pltpu.sync_copy(d_hbm, d_v)
  pltpu.sync_copy(src_hbm.at[c, s], src_v)
  pltpu.sync_copy(dst_hbm.at[c, s], dst_v)
  _fill(m_v, NPAD, NEG)

  def batch(j, _):
    for k in range(EB // L):
      di, e = _edge_e(s_v, d_v, src_v, dst_v, j, k)
      e_v[pl.ds(j * EB + k * L, L)] = e
      cur = plsc.load_gather(m_v, [di])
      need = e > cur

      def w_body(need):
        plsc.store_scatter(m_v, [di], e, mask=need)
        cur = plsc.load_gather(m_v, [di])
        return e > cur

      lax.while_loop(jnp.any, w_body, need)
    return 0

  lax.fori_loop(0, NB, batch, 0)

  pltpu.sync_copy(m_v, stage_sh.at[s])
  _fill(tmp_v, RPT, 0.0)
  sl = pl.ds(s * RPT, RPT)
  pltpu.sync_copy(tmp_v, den_sh.at[sl])
  plsc.subcore_barrier()
  pltpu.sync_copy(stage_sh.at[0, sl], acc_v)
  for i in range(1, NS):
    pltpu.sync_copy(stage_sh.at[i, sl], tmp_v)
    def red(t, _):
      ds = pl.ds(t * L, L)
      acc_v[ds] = jnp.maximum(acc_v[ds], tmp_v[ds])
      return 0
    lax.fori_loop(0, RPT // L, red, 0)
  pltpu.sync_copy(acc_v, m_hbm.at[c, sl])
  pltpu.sync_copy(acc_v, stage_sh.at[0, sl])
  plsc.subcore_barrier()
  pltpu.sync_copy(stage_sh.at[0], m_v)

  def fill_val(slot, j):
    for k in range(EB // L):
      di = dst_v[j, pl.ds(k * L, L)]
      e = e_v[pl.ds(j * EB + k * L, L)]
      mv = plsc.load_gather(m_v, [di])
      val_v[slot, pl.ds(k * L, L)] = jnp.exp(e - mv)

  def pair(u, _):
    ja = 2 * u
    jb = 2 * u + 1

    @pl.when(u >= 1)
    def _():
      pltpu.make_async_copy(val_v.at[0], den_sh.at[dst_v.at[ja]], ssa).wait()
    fill_val(0, ja)
    pltpu.async_copy(val_v.at[0], den_sh.at[dst_v.at[ja]], ssa, add=True)

    @pl.when(u >= 1)
    def _():
      pltpu.make_async_copy(val_v.at[1], den_sh.at[dst_v.at[jb]], ssb).wait()
    fill_val(1, jb)
    pltpu.async_copy(val_v.at[1], den_sh.at[dst_v.at[jb]], ssb, add=True)
    return 0

  lax.fori_loop(0, NB // 2, pair, 0)
  pltpu.make_async_copy(val_v.at[0], den_sh.at[dst_v.at[NB - 2]], ssa).wait()
  pltpu.make_async_copy(val_v.at[1], den_sh.at[dst_v.at[NB - 1]], ssb).wait()

  plsc.subcore_barrier()
  pltpu.sync_copy(den_sh.at[sl], acc_v)
  pltpu.sync_copy(acc_v, den_hbm.at[c, sl])


@functools.partial(
    pl.kernel,
    out_type=jax.ShapeDtypeStruct((NC, NPAD), jnp.float32),
    mesh=_MESH,
    scratch_types=[
        pltpu.VMEM((NPAD,), jnp.float32),
        pltpu.VMEM((NPAD,), jnp.float32),
        pltpu.VMEM((NPAD,), jnp.float32),
        pltpu.VMEM((NPAD,), jnp.float32),
        pltpu.VMEM((NPAD,), jnp.float32),
        pltpu.VMEM((NPAD,), jnp.float32),
        pltpu.VMEM((NB, EB), jnp.int32),
        pltpu.VMEM((NB, EB), jnp.int32),
        pltpu.VMEM((2, EB), jnp.float32),
        pltpu.VMEM((RPT,), jnp.float32),
        pltpu.VMEM_SHARED((NPAD,), jnp.float32),
        pltpu.SemaphoreType.DMA,
        pltpu.SemaphoreType.DMA,
    ],
    compiler_params=pltpu.CompilerParams(needs_layout_passes=False),
)
def _gat_w_sc(s_hbm, d_hbm, m_hbm, den_hbm, src_hbm, dst_hbm, out_hbm,
              s_v, d_v, m_v, den_v, t_v, u_v, src_v, dst_v, val_v, row_v,
              acc_sh, ssa, ssb):
  c = lax.axis_index("c")
  s = lax.axis_index("s")
  pltpu.sync_copy(s_hbm, s_v)
  pltpu.sync_copy(d_hbm, d_v)
  pltpu.sync_copy(m_hbm.at[0], m_v)
  pltpu.sync_copy(m_hbm.at[1], t_v)
  pltpu.sync_copy(den_hbm.at[0], den_v)
  pltpu.sync_copy(den_hbm.at[1], u_v)

  def comb(t, _):
    ds = pl.ds(t * L, L)
    m0 = m_v[ds]
    m1 = t_v[ds]
    mm = jnp.maximum(m0, m1)
    den = (jnp.exp(m0 - mm) * den_v[ds] + jnp.exp(m1 - mm) * u_v[ds] + 1e-16)
    m_v[ds] = mm
    den_v[ds] = den
    return 0
  lax.fori_loop(0, NPAD // L, comb, 0)
  pltpu.sync_copy(src_hbm.at[c, s], src_v)
  pltpu.sync_copy(dst_hbm.at[c, s], dst_v)
  _fill(row_v, RPT, 0.0)
  pltpu.sync_copy(row_v, acc_sh.at[pl.ds(s * RPT, RPT)])
  plsc.subcore_barrier()

  def fill_val(slot, j):
    for k in range(EB // L):
      di, e = _edge_e(s_v, d_v, src_v, dst_v, j, k)
      mv = plsc.load_gather(m_v, [di])
      dv = plsc.load_gather(den_v, [di])
      val_v[slot, pl.ds(k * L, L)] = jnp.exp(e - mv) / dv

  def pair(u, _):
    ja = 2 * u
    jb = 2 * u + 1

    @pl.when(u >= 1)
    def _():
      pltpu.make_async_copy(val_v.at[0], acc_sh.at[src_v.at[ja]], ssa).wait()
    fill_val(0, ja)
    pltpu.async_copy(val_v.at[0], acc_sh.at[src_v.at[ja]], ssa, add=True)

    @pl.when(u >= 1)
    def _():
      pltpu.make_async_copy(val_v.at[1], acc_sh.at[src_v.at[jb]], ssb).wait()
    fill_val(1, jb)
    pltpu.async_copy(val_v.at[1], acc_sh.at[src_v.at[jb]], ssb, add=True)
    return 0

  lax.fori_loop(0, NB // 2, pair, 0)
  pltpu.make_async_copy(val_v.at[0], acc_sh.at[src_v.at[NB - 2]], ssa).wait()
  pltpu.make_async_copy(val_v.at[1], acc_sh.at[src_v.at[NB - 1]], ssb).wait()

  plsc.subcore_barrier()
  sl = pl.ds(s * RPT, RPT)
  pltpu.sync_copy(acc_sh.at[sl], row_v)
  pltpu.sync_copy(row_v, out_hbm.at[c, sl])


def _dis_of(deg2):
  deg = deg2[0] + deg2[1]
  return jnp.where(deg > 0, lax.rsqrt(jnp.maximum(deg, 1e-12)), 0.0)


def _t01_body(x_ref, wemb_ref, bemb_ref, w1_ref, deg_ref, o_ref):
  h0 = jax.nn.relu(
      jnp.dot(x_ref[...], wemb_ref[...], preferred_element_type=jnp.float32)
      + bemb_ref[...])
  dis = _dis_of(deg_ref[...])
  o_ref[...] = dis[:, None] * jnp.dot(
      h0, w1_ref[...], preferred_element_type=jnp.float32)


def _t2_body(q_ref, p_ref, deg_ref, b_ref, w_ref, o_ref):
  dis = _dis_of(deg_ref[...])
  h = jax.nn.relu(
      dis[:, None] * (q_ref[0] + q_ref[1] + p_ref[...]) + b_ref[...])
  o_ref[...] = dis[:, None] * jnp.dot(
      h, w_ref[...], preferred_element_type=jnp.float32)


def _t3_body(q_ref, p_ref, deg_ref, b_ref, wg_ref, asrc_ref, adst_ref,
             hg_ref, s_ref, d_ref):
  dis = _dis_of(deg_ref[...])
  h = jax.nn.relu(
      dis[:, None] * (q_ref[0] + q_ref[1] + p_ref[...]) + b_ref[...])
  hg = jnp.dot(h, wg_ref[...], preferred_element_type=jnp.float32)
  hg_ref[...] = hg
  s_ref[...] = jnp.dot(hg, asrc_ref[...],
                       preferred_element_type=jnp.float32).T
  d_ref[...] = jnp.dot(hg, adst_ref[...],
                       preferred_element_type=jnp.float32).T


def _t4_body(w_ref, hg_ref, bg_ref, o_ref):
  i = pl.program_id(0)
  rows = i * BLK + lax.broadcasted_iota(jnp.int32, (BLK,), 0)
  w = jnp.where(rows < N, w_ref[0] + w_ref[1], 0.0)
  part = jnp.dot(w[None, :], hg_ref[...], preferred_element_type=jnp.float32)

  @pl.when(i == 0)
  def _():
    o_ref[...] = jnp.zeros_like(o_ref)
  o_ref[...] += part

  @pl.when(i == GRID - 1)
  def _():
    o_ref[...] = o_ref[...] * (1.0 / N) + bg_ref[...]


def _row_spec(block=None):
  return pl.BlockSpec((BLK, D) if block is None else block, lambda i: (0, 0))


_SPEC_ROWS = pl.BlockSpec((BLK, D), lambda i: (i, 0))
_SPEC_W = pl.BlockSpec((D, D), lambda i: (0, 0))
_SPEC_B = pl.BlockSpec((1, D), lambda i: (0, 0))
_SPEC_DEG = pl.BlockSpec((NC, BLK), lambda i: (0, i))
_SPEC_Q = pl.BlockSpec((NC, BLK, D), lambda i: (0, i, 0))
_SPEC_VEC = pl.BlockSpec((D, 1), lambda i: (0, 0))
_SPEC_SD = pl.BlockSpec((1, BLK), lambda i: (0, i))


def _t01(xp, W_emb, b_emb, W1, deg2):
  return pl.pallas_call(
      _t01_body, grid=(GRID,),
      in_specs=[_SPEC_ROWS, _SPEC_W, _SPEC_B, _SPEC_W, _SPEC_DEG],
      out_specs=_SPEC_ROWS,
      out_shape=jax.ShapeDtypeStruct((NPAD, D), jnp.float32),
  )(xp, W_emb, b_emb, W1, deg2)


def _t2(q2, p, deg2, b, W):
  return pl.pallas_call(
      _t2_body, grid=(GRID,),
      in_specs=[_SPEC_Q, _SPEC_ROWS, _SPEC_DEG, _SPEC_B, _SPEC_W],
      out_specs=_SPEC_ROWS,
      out_shape=jax.ShapeDtypeStruct((NPAD, D), jnp.float32),
  )(q2, p, deg2, b, W)


def _t3(q2, p, deg2, b, Wg, a_src, a_dst):
  return pl.pallas_call(
      _t3_body, grid=(GRID,),
      in_specs=[_SPEC_Q, _SPEC_ROWS, _SPEC_DEG, _SPEC_B, _SPEC_W,
                _SPEC_VEC, _SPEC_VEC],
      out_specs=[_SPEC_ROWS, _SPEC_SD, _SPEC_SD],
      out_shape=[
          jax.ShapeDtypeStruct((NPAD, D), jnp.float32),
          jax.ShapeDtypeStruct((1, NPAD), jnp.float32),
          jax.ShapeDtypeStruct((1, NPAD), jnp.float32),
      ],
  )(q2, p, deg2, b, Wg, a_src, a_dst)


def _t4(w2, hg, bg):
  return pl.pallas_call(
      _t4_body, grid=(GRID,),
      in_specs=[_SPEC_DEG, _SPEC_ROWS, _SPEC_B],
      out_specs=_SPEC_B,
      out_shape=jax.ShapeDtypeStruct((1, D), jnp.float32),
  )(w2, hg, bg)


def kernel(x, edge_index, W_emb, b_emb, W1, b1, W2, b2, Wg, a_src, a_dst, bg):
  ei = edge_index.astype(jnp.int32)
  loop = jnp.arange(N, dtype=jnp.int32)

  e_real = ei.shape[1]
  pada = N + 200 + (jnp.arange(ETOT_S - e_real, dtype=jnp.int32) % 8)
  padb = N + 200 + (jnp.arange(ETOT - ETOT_S - N, dtype=jnp.int32) % 8)

  def padded(row):
    return jnp.concatenate([row, pada, loop, padb])

  sflat = padded(ei[0])
  dflat = padded(ei[1])
  src4 = sflat.reshape(NC, NS, NB, EB)
  dst4 = dflat.reshape(NC, NS, NB, EB)
  src5 = sflat[:ETOT_S].reshape(NC, NS, NBT, SB)
  dst5 = dflat[:ETOT_S].reshape(NC, NS, NBT, SB)
  xp = jnp.zeros((NPAD, D), jnp.float32).at[:N].set(x)
  b_emb2 = b_emb.reshape(1, D)
  b12 = b1.reshape(1, D)
  b22 = b2.reshape(1, D)
  bg2 = bg.reshape(1, D)
  a_src2 = a_src.reshape(D, 1)
  a_dst2 = a_dst.reshape(D, 1)

  deg2 = _deg_sc(dst4)
  p1 = _t01(xp, W_emb, b_emb2, W1, deg2)
  q1 = _spmm_sc(p1, src5, dst5)
  p2 = _t2(q1, p1, deg2, b12, W2)
  q2 = _spmm_sc(p2, src5, dst5)
  hg, s2, d2 = _t3(q2, p2, deg2, b22, Wg, a_src2, a_dst2)
  s1 = s2.reshape(NPAD)
  d1 = d2.reshape(NPAD)
  m2, den2 = _gat_maxdenom_sc(s1, d1, src4, dst4)
  w2 = _gat_w_sc(s1, d1, m2, den2, src4, dst4)
  return _t4(w2, hg, bg2)


# speedup vs baseline: 54.4726x; 1.0009x over previous
import functools

import jax
import jax.numpy as jnp
from jax import lax
from jax.experimental import pallas as pl
from jax.experimental.pallas import tpu as pltpu
from jax.experimental.pallas import tpu_sc as plsc

N = 10000
D = 128
NPAD = 10240
NC = 2
NS = 16
L = 16
EB = 128
NB = 82
SB = 96
NBT = 105
ETOT_S = NC * NS * NBT * SB
EPW = NB * EB
ETOT = NC * NS * EPW
RPT = NPAD // NS
BLK = 2048
GRID = NPAD // BLK
NEG = -1e30

_MESH = plsc.VectorSubcoreMesh(
    core_axis_name="c", subcore_axis_name="s", num_cores=NC, num_subcores=NS)


def _fill(ref, n, value):
  vec = jnp.full((L,), value, ref.dtype)
  def body(i, _):
    ref[pl.ds(i * L, L)] = vec
    return 0
  lax.fori_loop(0, n // L, body, 0)


def _zero_shared_slice(acc_sh, zbuf, s):
  nz = zbuf.shape[0]
  for i in range(RPT // nz):
    pltpu.sync_copy(zbuf, acc_sh.at[pl.ds(s * RPT + i * nz, nz)])
  rem = RPT % nz
  if rem:
    pltpu.sync_copy(zbuf.at[pl.ds(0, rem)],
                    acc_sh.at[pl.ds(s * RPT + (RPT // nz) * nz, rem)])


def _copyout_shared_slice(acc_sh, out_hbm, bounce, c, s):
  nz = bounce.shape[0]
  for i in range(RPT // nz):
    sl = pl.ds(s * RPT + i * nz, nz)
    pltpu.sync_copy(acc_sh.at[sl], bounce)
    pltpu.sync_copy(bounce, out_hbm.at[c, sl])
  rem = RPT % nz
  if rem:
    sl = pl.ds(s * RPT + (RPT // nz) * nz, rem)
    pltpu.sync_copy(acc_sh.at[sl], bounce.at[pl.ds(0, rem)])
    pltpu.sync_copy(bounce.at[pl.ds(0, rem)], out_hbm.at[c, sl])


@functools.partial(
    pl.kernel,
    out_type=jax.ShapeDtypeStruct((NC, NPAD), jnp.float32),
    mesh=_MESH,
    scratch_types=[
        pltpu.VMEM((NB, EB), jnp.int32),
        pltpu.VMEM((EB,), jnp.float32),
        pltpu.VMEM((RPT,), jnp.float32),
        pltpu.VMEM_SHARED((NPAD,), jnp.float32),
        pltpu.SemaphoreType.DMA,
    ],
)
def _deg_sc(dst_hbm, out_hbm, idx_v, ones_v, row_v, acc_sh, sadd):
  c = lax.axis_index("c")
  s = lax.axis_index("s")
  _fill(ones_v, EB, 1.0)
  _fill(row_v, RPT, 0.0)
  pltpu.sync_copy(row_v, acc_sh.at[pl.ds(s * RPT, RPT)])
  pltpu.sync_copy(dst_hbm.at[c, s], idx_v)
  plsc.subcore_barrier()

  def body(j, _):
    pltpu.async_copy(ones_v, acc_sh.at[idx_v.at[j]], sadd, add=True)
    return 0
  lax.fori_loop(0, NB, body, 0)

  def drain(j, _):
    pltpu.make_async_copy(ones_v, acc_sh.at[idx_v.at[j]], sadd).wait()
    return 0
  lax.fori_loop(0, NB, drain, 0)

  plsc.subcore_barrier()
  sl = pl.ds(s * RPT, RPT)
  pltpu.sync_copy(acc_sh.at[sl], row_v)
  pltpu.sync_copy(row_v, out_hbm.at[c, sl])


@functools.partial(
    pl.kernel,
    out_type=jax.ShapeDtypeStruct((NC, NPAD, D), jnp.float32),
    mesh=_MESH,
    scratch_types=[
        pltpu.VMEM((4, SB), jnp.int32),
        pltpu.VMEM((4, SB), jnp.int32),
        pltpu.VMEM((SB, D), jnp.float32),
        pltpu.VMEM((SB, D), jnp.float32),
        pltpu.VMEM((SB, D), jnp.float32),
        pltpu.VMEM_SHARED((NPAD, D), jnp.float32),
        pltpu.SemaphoreType.DMA,
        pltpu.SemaphoreType.DMA,
        pltpu.SemaphoreType.DMA,
        pltpu.SemaphoreType.DMA,
        pltpu.SemaphoreType.DMA,
        pltpu.SemaphoreType.DMA,
        pltpu.SemaphoreType.DMA,
    ],
)
def _spmm_sc(p_hbm, src_hbm, dst_hbm, out_hbm,
             srcb, dstb, buf0, buf1, buf2, acc_sh,
             sg0, sg1, sg2, ss0, ss1, ss2, sidx):
  c = lax.axis_index("c")
  s = lax.axis_index("s")
  bufs = (buf0, buf1, buf2)
  sgs = (sg0, sg1, sg2)
  sss = (ss0, ss1, ss2)
  zvec = jnp.zeros((L,), jnp.float32)
  def zrow(r, _):
    for k in range(D // L):
      buf0[r, pl.ds(k * L, L)] = zvec
    return 0
  lax.fori_loop(0, SB, zrow, 0)
  _zero_shared_slice(acc_sh, buf0, s)
  plsc.subcore_barrier()

  pltpu.sync_copy(src_hbm.at[c, s, 0], srcb.at[0])
  pltpu.sync_copy(dst_hbm.at[c, s, 0], dstb.at[0])
  pltpu.sync_copy(src_hbm.at[c, s, 1], srcb.at[1])
  pltpu.sync_copy(dst_hbm.at[c, s, 1], dstb.at[1])
  pltpu.async_copy(src_hbm.at[c, s, 2], srcb.at[2], sidx)
  pltpu.async_copy(dst_hbm.at[c, s, 2], dstb.at[2], sidx)
  pltpu.async_copy(p_hbm.at[srcb.at[0]], buf0, sg0)
  pltpu.async_copy(p_hbm.at[srcb.at[1]], buf1, sg1)

  def stage(t, b):
    bp = (b + 2) % 3
    i0 = lax.rem(t, 4)
    i2 = lax.rem(t + 2, 4)
    i3 = lax.rem(t + 3, 4)
    pltpu.make_async_copy(p_hbm.at[srcb.at[i0]], bufs[b], sgs[b]).wait()
    pltpu.async_copy(bufs[b], acc_sh.at[dstb.at[i0]], sss[b], add=True)

    @pl.when(t >= 1)
    def _():
      pltpu.make_async_copy(bufs[bp], acc_sh.at[dstb.at[i3]], sss[bp]).wait()

    @pl.when(t + 2 < NBT)
    def _():
      pltpu.make_async_copy(src_hbm.at[c, s, t + 2], srcb.at[i2], sidx).wait()
      pltpu.make_async_copy(dst_hbm.at[c, s, t + 2], dstb.at[i2], sidx).wait()

      @pl.when(t + 3 < NBT)
      def _():
        pltpu.async_copy(src_hbm.at[c, s, t + 3], srcb.at[i3], sidx)
        pltpu.async_copy(dst_hbm.at[c, s, t + 3], dstb.at[i3], sidx)
      pltpu.async_copy(p_hbm.at[srcb.at[i2]], bufs[bp], sgs[bp])

  def body(u, _):
    stage(3 * u, 0)
    stage(3 * u + 1, 1)
    stage(3 * u + 2, 2)
    return 0

  assert NBT % 3 == 0
  lax.fori_loop(0, NBT // 3, body, 0)

  lb = (NBT - 1) % 3
  pltpu.make_async_copy(
      bufs[lb], acc_sh.at[dstb.at[(NBT - 1) % 4]], sss[lb]).wait()

  plsc.subcore_barrier()
  _copyout_shared_slice(acc_sh, out_hbm, buf0, c, s)


def _edge_e(s_v, d_v, src_v, dst_v, j, k):
  si = src_v[j, pl.ds(k * L, L)]
  di = dst_v[j, pl.ds(k * L, L)]
  sv = plsc.load_gather(s_v, [si])
  dv = plsc.load_gather(d_v, [di])
  z = sv + dv
  return di, jnp.where(z >= 0, z, 0.2 * z)


@functools.partial(
    pl.kernel,
    out_type=[
        jax.ShapeDtypeStruct((NC, NPAD), jnp.float32),
        jax.ShapeDtypeStruct((NC, NPAD), jnp.float32),
        jax.ShapeDtypeStruct((NC, NS, EPW), jnp.float32),
    ],
    mesh=_MESH,
    scratch_types=[
        pltpu.VMEM((NPAD,), jnp.float32),
        pltpu.VMEM((NPAD,), jnp.float32),
        pltpu.VMEM((NPAD,), jnp.float32),
        pltpu.VMEM((NB, EB), jnp.int32),
        pltpu.VMEM((NB, EB), jnp.int32),
        pltpu.VMEM((RPT,), jnp.float32),
        pltpu.VMEM((RPT,), jnp.float32),
        pltpu.VMEM((2, EB), jnp.float32),
        pltpu.VMEM((EPW,), jnp.float32),
        pltpu.VMEM_SHARED((NS, NPAD), jnp.float32),
        pltpu.VMEM_SHARED((NPAD,), jnp.float32),
        pltpu.SemaphoreType.DMA,
        pltpu.SemaphoreType.DMA,
        pltpu.SemaphoreType.DMA,
        pltpu.SemaphoreType.DMA,
    ],
    compiler_params=pltpu.CompilerParams(needs_layout_passes=False),
)
def _gat_maxdenom_sc(s_hbm, d_hbm, src_hbm, dst_hbm, m_hbm, den_hbm, ex_hbm,
                     s_v, d_v, m_v, src_v, dst_v, acc_v, tmp_v, val_v,
                     e_v, stage_sh, den_sh, ssa, ssb, sea, seb):
  c = lax.axis_index("c")
  s = lax.axis_index("s")
  pltpu.sync_copy(s_hbm, s_v)
  pltpu.sync_copy(d_hbm, d_v)
  pltpu.sync_copy(src_hbm.at[c, s], src_v)
  pltpu.sync_copy(dst_hbm.at[c, s], dst_v)
  _fill(m_v, NPAD, NEG)

  def batch(j, _):
    for k in range(EB // L):
      di, e = _edge_e(s_v, d_v, src_v, dst_v, j, k)
      e_v[pl.ds(j * EB + k * L, L)] = e
      cur = plsc.load_gather(m_v, [di])
      need = e > cur

      def w_body(need):
        plsc.store_scatter(m_v, [di], e, mask=need)
        cur = plsc.load_gather(m_v, [di])
        return e > cur

      lax.while_loop(jnp.any, w_body, need)
    return 0

  lax.fori_loop(0, NB, batch, 0)

  pltpu.sync_copy(m_v, stage_sh.at[s])
  _fill(tmp_v, RPT, 0.0)
  sl = pl.ds(s * RPT, RPT)
  pltpu.sync_copy(tmp_v, den_sh.at[sl])
  plsc.subcore_barrier()
  pltpu.sync_copy(stage_sh.at[0, sl], acc_v)
  for i in range(1, NS):
    pltpu.sync_copy(stage_sh.at[i, sl], tmp_v)
    def red(t, _):
      ds = pl.ds(t * L, L)
      acc_v[ds] = jnp.maximum(acc_v[ds], tmp_v[ds])
      return 0
    lax.fori_loop(0, RPT // L, red, 0)
  pltpu.sync_copy(acc_v, m_hbm.at[c, sl])
  pltpu.sync_copy(acc_v, stage_sh.at[0, sl])
  plsc.subcore_barrier()
  pltpu.sync_copy(stage_sh.at[0], m_v)

  def fill_val(slot, j):
    for k in range(EB // L):
      di = dst_v[j, pl.ds(k * L, L)]
      e = e_v[pl.ds(j * EB + k * L, L)]
      mv = plsc.load_gather(m_v, [di])
      val_v[slot, pl.ds(k * L, L)] = jnp.exp(e - mv)

  def pair(u, _):
    ja = 2 * u
    jb = 2 * u + 1

    @pl.when(u >= 1)
    def _():
      pltpu.make_async_copy(val_v.at[0], den_sh.at[dst_v.at[ja]], ssa).wait()
      pltpu.make_async_copy(
          val_v.at[0], ex_hbm.at[c, s, pl.ds(ja * EB, EB)], sea).wait()
    fill_val(0, ja)
    pltpu.async_copy(val_v.at[0], den_sh.at[dst_v.at[ja]], ssa, add=True)
    pltpu.async_copy(val_v.at[0], ex_hbm.at[c, s, pl.ds(ja * EB, EB)], sea)

    @pl.when(u >= 1)
    def _():
      pltpu.make_async_copy(val_v.at[1], den_sh.at[dst_v.at[jb]], ssb).wait()
      pltpu.make_async_copy(
          val_v.at[1], ex_hbm.at[c, s, pl.ds(jb * EB, EB)], seb).wait()
    fill_val(1, jb)
    pltpu.async_copy(val_v.at[1], den_sh.at[dst_v.at[jb]], ssb, add=True)
    pltpu.async_copy(val_v.at[1], ex_hbm.at[c, s, pl.ds(jb * EB, EB)], seb)
    return 0

  lax.fori_loop(0, NB // 2, pair, 0)
  pltpu.make_async_copy(val_v.at[0], den_sh.at[dst_v.at[NB - 2]], ssa).wait()
  pltpu.make_async_copy(val_v.at[1], den_sh.at[dst_v.at[NB - 1]], ssb).wait()
  pltpu.make_async_copy(
      val_v.at[0], ex_hbm.at[c, s, pl.ds((NB - 2) * EB, EB)], sea).wait()
  pltpu.make_async_copy(
      val_v.at[1], ex_hbm.at[c, s, pl.ds((NB - 1) * EB, EB)], seb).wait()

  plsc.subcore_barrier()
  pltpu.sync_copy(den_sh.at[sl], acc_v)
  pltpu.sync_copy(acc_v, den_hbm.at[c, sl])


@functools.partial(
    pl.kernel,
    out_type=jax.ShapeDtypeStruct((NC, NPAD), jnp.float32),
    mesh=_MESH,
    scratch_types=[
        pltpu.VMEM((NPAD,), jnp.float32),
        pltpu.VMEM((NPAD,), jnp.float32),
        pltpu.VMEM((NPAD,), jnp.float32),
        pltpu.VMEM((NPAD,), jnp.float32),
        pltpu.VMEM((NB, EB), jnp.int32),
        pltpu.VMEM((NB, EB), jnp.int32),
        pltpu.VMEM((EPW,), jnp.float32),
        pltpu.VMEM((2, EB), jnp.float32),
        pltpu.VMEM((RPT,), jnp.float32),
        pltpu.VMEM_SHARED((NPAD,), jnp.float32),
        pltpu.SemaphoreType.DMA,
        pltpu.SemaphoreType.DMA,
    ],
    compiler_params=pltpu.CompilerParams(needs_layout_passes=False),
)
def _gat_w_sc(m_hbm, den_hbm, ex_hbm, src_hbm, dst_hbm, out_hbm,
              rr_v, t_v, d0_v, d1_v, src_v, dst_v, ex_v, val_v, row_v,
              acc_sh, ssa, ssb):
  c = lax.axis_index("c")
  s = lax.axis_index("s")
  pltpu.sync_copy(m_hbm.at[0], rr_v)
  pltpu.sync_copy(m_hbm.at[1], t_v)
  pltpu.sync_copy(den_hbm.at[0], d0_v)
  pltpu.sync_copy(den_hbm.at[1], d1_v)
  pltpu.sync_copy(ex_hbm.at[c, s], ex_v)
  pltpu.sync_copy(src_hbm.at[c, s], src_v)
  pltpu.sync_copy(dst_hbm.at[c, s], dst_v)

  def comb(t, _):
    ds = pl.ds(t * L, L)
    m0 = rr_v[ds]
    m1 = t_v[ds]
    mm = jnp.maximum(m0, m1)
    r0 = jnp.exp(m0 - mm)
    r1 = jnp.exp(m1 - mm)
    den = r0 * d0_v[ds] + r1 * d1_v[ds] + 1e-16
    rr_v[ds] = jnp.where(c == 0, r0, r1) / den
    return 0
  lax.fori_loop(0, NPAD // L, comb, 0)

  _fill(row_v, RPT, 0.0)
  pltpu.sync_copy(row_v, acc_sh.at[pl.ds(s * RPT, RPT)])
  plsc.subcore_barrier()

  def fill_val(slot, j):
    for k in range(EB // L):
      di = dst_v[j, pl.ds(k * L, L)]
      ex = ex_v[pl.ds(j * EB + k * L, L)]
      rr = plsc.load_gather(rr_v, [di])
      val_v[slot, pl.ds(k * L, L)] = ex * rr

  def pair(u, _):
    ja = 2 * u
    jb = 2 * u + 1

    @pl.when(u >= 1)
    def _():
      pltpu.make_async_copy(val_v.at[0], acc_sh.at[src_v.at[ja]], ssa).wait()
    fill_val(0, ja)
    pltpu.async_copy(val_v.at[0], acc_sh.at[src_v.at[ja]], ssa, add=True)

    @pl.when(u >= 1)
    def _():
      pltpu.make_async_copy(val_v.at[1], acc_sh.at[src_v.at[jb]], ssb).wait()
    fill_val(1, jb)
    pltpu.async_copy(val_v.at[1], acc_sh.at[src_v.at[jb]], ssb, add=True)
    return 0

  lax.fori_loop(0, NB // 2, pair, 0)
  pltpu.make_async_copy(val_v.at[0], acc_sh.at[src_v.at[NB - 2]], ssa).wait()
  pltpu.make_async_copy(val_v.at[1], acc_sh.at[src_v.at[NB - 1]], ssb).wait()

  plsc.subcore_barrier()
  sl = pl.ds(s * RPT, RPT)
  pltpu.sync_copy(acc_sh.at[sl], row_v)
  pltpu.sync_copy(row_v, out_hbm.at[c, sl])


def _dis_of(deg2):
  deg = deg2[0] + deg2[1]
  return jnp.where(deg > 0, lax.rsqrt(jnp.maximum(deg, 1e-12)), 0.0)


def _t01_body(x_ref, wemb_ref, bemb_ref, w1_ref, deg_ref, o_ref):
  h0 = jax.nn.relu(
      jnp.dot(x_ref[...], wemb_ref[...], preferred_element_type=jnp.float32)
      + bemb_ref[...])
  dis = _dis_of(deg_ref[...])
  o_ref[...] = dis[:, None] * jnp.dot(
      h0, w1_ref[...], preferred_element_type=jnp.float32)


def _t2_body(q_ref, p_ref, deg_ref, b_ref, w_ref, o_ref):
  dis = _dis_of(deg_ref[...])
  h = jax.nn.relu(
      dis[:, None] * (q_ref[0] + q_ref[1] + p_ref[...]) + b_ref[...])
  o_ref[...] = dis[:, None] * jnp.dot(
      h, w_ref[...], preferred_element_type=jnp.float32)


def _t3_body(q_ref, p_ref, deg_ref, b_ref, wg_ref, asrc_ref, adst_ref,
             hg_ref, s_ref, d_ref):
  dis = _dis_of(deg_ref[...])
  h = jax.nn.relu(
      dis[:, None] * (q_ref[0] + q_ref[1] + p_ref[...]) + b_ref[...])
  hg = jnp.dot(h, wg_ref[...], preferred_element_type=jnp.float32)
  hg_ref[...] = hg
  s_ref[...] = jnp.dot(hg, asrc_ref[...],
                       preferred_element_type=jnp.float32).T
  d_ref[...] = jnp.dot(hg, adst_ref[...],
                       preferred_element_type=jnp.float32).T


def _t4_body(w_ref, hg_ref, bg_ref, o_ref):
  i = pl.program_id(0)
  rows = i * BLK + lax.broadcasted_iota(jnp.int32, (BLK,), 0)
  w = jnp.where(rows < N, w_ref[0] + w_ref[1], 0.0)
  part = jnp.dot(w[None, :], hg_ref[...], preferred_element_type=jnp.float32)

  @pl.when(i == 0)
  def _():
    o_ref[...] = jnp.zeros_like(o_ref)
  o_ref[...] += part

  @pl.when(i == GRID - 1)
  def _():
    o_ref[...] = o_ref[...] * (1.0 / N) + bg_ref[...]


def _row_spec(block=None):
  return pl.BlockSpec((BLK, D) if block is None else block, lambda i: (0, 0))


_SPEC_ROWS = pl.BlockSpec((BLK, D), lambda i: (i, 0))
_SPEC_W = pl.BlockSpec((D, D), lambda i: (0, 0))
_SPEC_B = pl.BlockSpec((1, D), lambda i: (0, 0))
_SPEC_DEG = pl.BlockSpec((NC, BLK), lambda i: (0, i))
_SPEC_Q = pl.BlockSpec((NC, BLK, D), lambda i: (0, i, 0))
_SPEC_VEC = pl.BlockSpec((D, 1), lambda i: (0, 0))
_SPEC_SD = pl.BlockSpec((1, BLK), lambda i: (0, i))


def _t01(xp, W_emb, b_emb, W1, deg2):
  return pl.pallas_call(
      _t01_body, grid=(GRID,),
      in_specs=[_SPEC_ROWS, _SPEC_W, _SPEC_B, _SPEC_W, _SPEC_DEG],
      out_specs=_SPEC_ROWS,
      out_shape=jax.ShapeDtypeStruct((NPAD, D), jnp.float32),
  )(xp, W_emb, b_emb, W1, deg2)


def _t2(q2, p, deg2, b, W):
  return pl.pallas_call(
      _t2_body, grid=(GRID,),
      in_specs=[_SPEC_Q, _SPEC_ROWS, _SPEC_DEG, _SPEC_B, _SPEC_W],
      out_specs=_SPEC_ROWS,
      out_shape=jax.ShapeDtypeStruct((NPAD, D), jnp.float32),
  )(q2, p, deg2, b, W)


def _t3(q2, p, deg2, b, Wg, a_src, a_dst):
  return pl.pallas_call(
      _t3_body, grid=(GRID,),
      in_specs=[_SPEC_Q, _SPEC_ROWS, _SPEC_DEG, _SPEC_B, _SPEC_W,
                _SPEC_VEC, _SPEC_VEC],
      out_specs=[_SPEC_ROWS, _SPEC_SD, _SPEC_SD],
      out_shape=[
          jax.ShapeDtypeStruct((NPAD, D), jnp.float32),
          jax.ShapeDtypeStruct((1, NPAD), jnp.float32),
          jax.ShapeDtypeStruct((1, NPAD), jnp.float32),
      ],
  )(q2, p, deg2, b, Wg, a_src, a_dst)


def _t4(w2, hg, bg):
  return pl.pallas_call(
      _t4_body, grid=(GRID,),
      in_specs=[_SPEC_DEG, _SPEC_ROWS, _SPEC_B],
      out_specs=_SPEC_B,
      out_shape=jax.ShapeDtypeStruct((1, D), jnp.float32),
  )(w2, hg, bg)


def kernel(x, edge_index, W_emb, b_emb, W1, b1, W2, b2, Wg, a_src, a_dst, bg):
  ei = edge_index.astype(jnp.int32)
  loop = jnp.arange(N, dtype=jnp.int32)

  e_real = ei.shape[1]
  pada = N + 200 + (jnp.arange(ETOT_S - e_real, dtype=jnp.int32) % 8)
  padb = N + 200 + (jnp.arange(ETOT - ETOT_S - N, dtype=jnp.int32) % 8)

  def padded(row):
    return jnp.concatenate([row, pada, loop, padb])

  sflat = padded(ei[0])
  dflat = padded(ei[1])
  src4 = sflat.reshape(NC, NS, NB, EB)
  dst4 = dflat.reshape(NC, NS, NB, EB)
  src5 = sflat[:ETOT_S].reshape(NC, NS, NBT, SB)
  dst5 = dflat[:ETOT_S].reshape(NC, NS, NBT, SB)
  xp = jnp.zeros((NPAD, D), jnp.float32).at[:N].set(x)
  b_emb2 = b_emb.reshape(1, D)
  b12 = b1.reshape(1, D)
  b22 = b2.reshape(1, D)
  bg2 = bg.reshape(1, D)
  a_src2 = a_src.reshape(D, 1)
  a_dst2 = a_dst.reshape(D, 1)

  deg2 = _deg_sc(dst4)
  p1 = _t01(xp, W_emb, b_emb2, W1, deg2)
  q1 = _spmm_sc(p1, src5, dst5)
  p2 = _t2(q1, p1, deg2, b12, W2)
  q2 = _spmm_sc(p2, src5, dst5)
  hg, s2, d2 = _t3(q2, p2, deg2, b22, Wg, a_src2, a_dst2)
  s1 = s2.reshape(NPAD)
  d1 = d2.reshape(NPAD)
  m2, den2, ex3 = _gat_maxdenom_sc(s1, d1, src4, dst4)
  w2 = _gat_w_sc(m2, den2, ex3, src4, dst4)
  return _t4(w2, hg, bg2)


# speedup vs baseline: 54.7066x; 1.0043x over previous
import functools

import jax
import jax.numpy as jnp
from jax import lax
from jax.experimental import pallas as pl
from jax.experimental.pallas import tpu as pltpu
from jax.experimental.pallas import tpu_sc as plsc

N = 10000
D = 128
NPAD = 10240
NC = 2
NS = 16
L = 16
EB = 128
NB = 82
SB = 96
NBT = 105
ETOT_S = NC * NS * NBT * SB
EPW = NB * EB
ETOT = NC * NS * EPW
RPT = NPAD // NS
BLK = 2048
GRID = NPAD // BLK
NEG = -1e30

_MESH = plsc.VectorSubcoreMesh(
    core_axis_name="c", subcore_axis_name="s", num_cores=NC, num_subcores=NS)


def _fill(ref, n, value):
  vec = jnp.full((L,), value, ref.dtype)
  def body(i, _):
    ref[pl.ds(i * L, L)] = vec
    return 0
  lax.fori_loop(0, n // L, body, 0)


def _zero_shared_slice(acc_sh, zbuf, s):
  nz = zbuf.shape[0]
  for i in range(RPT // nz):
    pltpu.sync_copy(zbuf, acc_sh.at[pl.ds(s * RPT + i * nz, nz)])
  rem = RPT % nz
  if rem:
    pltpu.sync_copy(zbuf.at[pl.ds(0, rem)],
                    acc_sh.at[pl.ds(s * RPT + (RPT // nz) * nz, rem)])


def _copyout_shared_slice(acc_sh, out_hbm, bounce, c, s):
  nz = bounce.shape[0]
  for i in range(RPT // nz):
    sl = pl.ds(s * RPT + i * nz, nz)
    pltpu.sync_copy(acc_sh.at[sl], bounce)
    pltpu.sync_copy(bounce, out_hbm.at[c, sl])
  rem = RPT % nz
  if rem:
    sl = pl.ds(s * RPT + (RPT // nz) * nz, rem)
    pltpu.sync_copy(acc_sh.at[sl], bounce.at[pl.ds(0, rem)])
    pltpu.sync_copy(bounce.at[pl.ds(0, rem)], out_hbm.at[c, sl])


@functools.partial(
    pl.kernel,
    out_type=jax.ShapeDtypeStruct((NC, NPAD), jnp.float32),
    mesh=_MESH,
    scratch_types=[
        pltpu.VMEM((NBT, SB), jnp.int32),
        pltpu.VMEM((SB,), jnp.float32),
        pltpu.VMEM((RPT,), jnp.float32),
        pltpu.VMEM_SHARED((NPAD,), jnp.float32),
        pltpu.SemaphoreType.DMA,
    ],
)
def _deg_sc(dst_hbm, out_hbm, idx_v, ones_v, row_v, acc_sh, sadd):
  c = lax.axis_index("c")
  s = lax.axis_index("s")
  _fill(ones_v, SB, 1.0)
  _fill(row_v, RPT, 0.0)
  pltpu.sync_copy(row_v, acc_sh.at[pl.ds(s * RPT, RPT)])
  pltpu.sync_copy(dst_hbm.at[c, s], idx_v)
  plsc.subcore_barrier()

  def body(j, _):
    pltpu.async_copy(ones_v, acc_sh.at[idx_v.at[j]], sadd, add=True)
    return 0
  lax.fori_loop(0, NBT, body, 0)

  def drain(j, _):
    pltpu.make_async_copy(ones_v, acc_sh.at[idx_v.at[j]], sadd).wait()
    return 0
  lax.fori_loop(0, NBT, drain, 0)

  plsc.subcore_barrier()
  sl = pl.ds(s * RPT, RPT)
  pltpu.sync_copy(acc_sh.at[sl], row_v)
  pltpu.sync_copy(row_v, out_hbm.at[c, sl])


@functools.partial(
    pl.kernel,
    out_type=jax.ShapeDtypeStruct((NC, NPAD, D), jnp.float32),
    mesh=_MESH,
    scratch_types=[
        pltpu.VMEM((4, SB), jnp.int32),
        pltpu.VMEM((4, SB), jnp.int32),
        pltpu.VMEM((SB, D), jnp.float32),
        pltpu.VMEM((SB, D), jnp.float32),
        pltpu.VMEM((SB, D), jnp.float32),
        pltpu.VMEM_SHARED((NPAD, D), jnp.float32),
        pltpu.SemaphoreType.DMA,
        pltpu.SemaphoreType.DMA,
        pltpu.SemaphoreType.DMA,
        pltpu.SemaphoreType.DMA,
        pltpu.SemaphoreType.DMA,
        pltpu.SemaphoreType.DMA,
        pltpu.SemaphoreType.DMA,
    ],
)
def _spmm_sc(p_hbm, src_hbm, dst_hbm, out_hbm,
             srcb, dstb, buf0, buf1, buf2, acc_sh,
             sg0, sg1, sg2, ss0, ss1, ss2, sidx):
  c = lax.axis_index("c")
  s = lax.axis_index("s")
  bufs = (buf0, buf1, buf2)
  sgs = (sg0, sg1, sg2)
  sss = (ss0, ss1, ss2)
  zvec = jnp.zeros((L,), jnp.float32)
  def zrow(r, _):
    for k in range(D // L):
      buf0[r, pl.ds(k * L, L)] = zvec
    return 0
  lax.fori_loop(0, SB, zrow, 0)
  _zero_shared_slice(acc_sh, buf0, s)
  plsc.subcore_barrier()

  pltpu.sync_copy(src_hbm.at[c, s, 0], srcb.at[0])
  pltpu.sync_copy(dst_hbm.at[c, s, 0], dstb.at[0])
  pltpu.sync_copy(src_hbm.at[c, s, 1], srcb.at[1])
  pltpu.sync_copy(dst_hbm.at[c, s, 1], dstb.at[1])
  pltpu.async_copy(src_hbm.at[c, s, 2], srcb.at[2], sidx)
  pltpu.async_copy(dst_hbm.at[c, s, 2], dstb.at[2], sidx)
  pltpu.async_copy(p_hbm.at[srcb.at[0]], buf0, sg0)
  pltpu.async_copy(p_hbm.at[srcb.at[1]], buf1, sg1)

  def stage(t, b):
    bp = (b + 2) % 3
    i0 = lax.rem(t, 4)
    i2 = lax.rem(t + 2, 4)
    i3 = lax.rem(t + 3, 4)
    pltpu.make_async_copy(p_hbm.at[srcb.at[i0]], bufs[b], sgs[b]).wait()
    pltpu.async_copy(bufs[b], acc_sh.at[dstb.at[i0]], sss[b], add=True)

    @pl.when(t >= 1)
    def _():
      pltpu.make_async_copy(bufs[bp], acc_sh.at[dstb.at[i3]], sss[bp]).wait()

    @pl.when(t + 2 < NBT)
    def _():
      pltpu.make_async_copy(src_hbm.at[c, s, t + 2], srcb.at[i2], sidx).wait()
      pltpu.make_async_copy(dst_hbm.at[c, s, t + 2], dstb.at[i2], sidx).wait()

      @pl.when(t + 3 < NBT)
      def _():
        pltpu.async_copy(src_hbm.at[c, s, t + 3], srcb.at[i3], sidx)
        pltpu.async_copy(dst_hbm.at[c, s, t + 3], dstb.at[i3], sidx)
      pltpu.async_copy(p_hbm.at[srcb.at[i2]], bufs[bp], sgs[bp])

  def body(u, _):
    stage(3 * u, 0)
    stage(3 * u + 1, 1)
    stage(3 * u + 2, 2)
    return 0

  assert NBT % 3 == 0
  lax.fori_loop(0, NBT // 3, body, 0)

  lb = (NBT - 1) % 3
  pltpu.make_async_copy(
      bufs[lb], acc_sh.at[dstb.at[(NBT - 1) % 4]], sss[lb]).wait()

  plsc.subcore_barrier()
  _copyout_shared_slice(acc_sh, out_hbm, buf0, c, s)


def _edge_e(s_v, d_v, src_v, dst_v, j, k):
  si = src_v[j, pl.ds(k * L, L)]
  di = dst_v[j, pl.ds(k * L, L)]
  sv = plsc.load_gather(s_v, [si])
  dv = plsc.load_gather(d_v, [di])
  z = sv + dv
  return di, jnp.where(z >= 0, z, 0.2 * z)


@functools.partial(
    pl.kernel,
    out_type=[
        jax.ShapeDtypeStruct((NC, NPAD), jnp.float32),
        jax.ShapeDtypeStruct((NC, NPAD), jnp.float32),
        jax.ShapeDtypeStruct((NC, NS, EPW), jnp.float32),
    ],
    mesh=_MESH,
    scratch_types=[
        pltpu.VMEM((NPAD,), jnp.float32),
        pltpu.VMEM((NPAD,), jnp.float32),
        pltpu.VMEM((NPAD,), jnp.float32),
        pltpu.VMEM((NB, EB), jnp.int32),
        pltpu.VMEM((NB, EB), jnp.int32),
        pltpu.VMEM((RPT,), jnp.float32),
        pltpu.VMEM((RPT,), jnp.float32),
        pltpu.VMEM((2, EB), jnp.float32),
        pltpu.VMEM((EPW,), jnp.float32),
        pltpu.VMEM_SHARED((NS, NPAD), jnp.float32),
        pltpu.VMEM_SHARED((NPAD,), jnp.float32),
        pltpu.SemaphoreType.DMA,
        pltpu.SemaphoreType.DMA,
        pltpu.SemaphoreType.DMA,
        pltpu.SemaphoreType.DMA,
    ],
    compiler_params=pltpu.CompilerParams(needs_layout_passes=False),
)
def _gat_maxdenom_sc(s_hbm, d_hbm, src_hbm, dst_hbm, m_hbm, den_hbm, ex_hbm,
                     s_v, d_v, m_v, src_v, dst_v, acc_v, tmp_v, val_v,
                     e_v, stage_sh, den_sh, ssa, ssb, sea, seb):
  c = lax.axis_index("c")
  s = lax.axis_index("s")
  pltpu.sync_copy(s_hbm, s_v)
  pltpu.sync_copy(d_hbm, d_v)
  pltpu.sync_copy(src_hbm.at[c, s], src_v)
  pltpu.sync_copy(dst_hbm.at[c, s], dst_v)
  _fill(m_v, NPAD, NEG)

  def batch(j, _):
    for k in range(EB // L):
      di, e = _edge_e(s_v, d_v, src_v, dst_v, j, k)
      e_v[pl.ds(j * EB + k * L, L)] = e
      cur = plsc.load_gather(m_v, [di])
      need = e > cur

      def w_body(need):
        plsc.store_scatter(m_v, [di], e, mask=need)
        cur = plsc.load_gather(m_v, [di])
        return e > cur

      lax.while_loop(jnp.any, w_body, need)
    return 0

  lax.fori_loop(0, NB, batch, 0)

  pltpu.sync_copy(m_v, stage_sh.at[s])
  _fill(tmp_v, RPT, 0.0)
  sl = pl.ds(s * RPT, RPT)
  pltpu.sync_copy(tmp_v, den_sh.at[sl])
  plsc.subcore_barrier()
  pltpu.sync_copy(stage_sh.at[0, sl], acc_v)
  for i in range(1, NS):
    pltpu.sync_copy(stage_sh.at[i, sl], tmp_v)
    def red(t, _):
      ds = pl.ds(t * L, L)
      acc_v[ds] = jnp.maximum(acc_v[ds], tmp_v[ds])
      return 0
    lax.fori_loop(0, RPT // L, red, 0)
  pltpu.sync_copy(acc_v, m_hbm.at[c, sl])
  pltpu.sync_copy(acc_v, stage_sh.at[0, sl])
  plsc.subcore_barrier()
  pltpu.sync_copy(stage_sh.at[0], m_v)

  def fill_val(slot, j):
    for k in range(EB // L):
      di = dst_v[j, pl.ds(k * L, L)]
      e = e_v[pl.ds(j * EB + k * L, L)]
      mv = plsc.load_gather(m_v, [di])
      val_v[slot, pl.ds(k * L, L)] = jnp.exp(e - mv)

  def pair(u, _):
    ja = 2 * u
    jb = 2 * u + 1

    @pl.when(u >= 1)
    def _():
      pltpu.make_async_copy(val_v.at[0], den_sh.at[dst_v.at[ja]], ssa).wait()
      pltpu.make_async_copy(
          val_v.at[0], ex_hbm.at[c, s, pl.ds(ja * EB, EB)], sea).wait()
    fill_val(0, ja)
    pltpu.async_copy(val_v.at[0], den_sh.at[dst_v.at[ja]], ssa, add=True)
    pltpu.async_copy(val_v.at[0], ex_hbm.at[c, s, pl.ds(ja * EB, EB)], sea)

    @pl.when(u >= 1)
    def _():
      pltpu.make_async_copy(val_v.at[1], den_sh.at[dst_v.at[jb]], ssb).wait()
      pltpu.make_async_copy(
          val_v.at[1], ex_hbm.at[c, s, pl.ds(jb * EB, EB)], seb).wait()
    fill_val(1, jb)
    pltpu.async_copy(val_v.at[1], den_sh.at[dst_v.at[jb]], ssb, add=True)
    pltpu.async_copy(val_v.at[1], ex_hbm.at[c, s, pl.ds(jb * EB, EB)], seb)
    return 0

  lax.fori_loop(0, NB // 2, pair, 0)
  pltpu.make_async_copy(val_v.at[0], den_sh.at[dst_v.at[NB - 2]], ssa).wait()
  pltpu.make_async_copy(val_v.at[1], den_sh.at[dst_v.at[NB - 1]], ssb).wait()
  pltpu.make_async_copy(
      val_v.at[0], ex_hbm.at[c, s, pl.ds((NB - 2) * EB, EB)], sea).wait()
  pltpu.make_async_copy(
      val_v.at[1], ex_hbm.at[c, s, pl.ds((NB - 1) * EB, EB)], seb).wait()

  plsc.subcore_barrier()
  pltpu.sync_copy(den_sh.at[sl], acc_v)
  pltpu.sync_copy(acc_v, den_hbm.at[c, sl])


@functools.partial(
    pl.kernel,
    out_type=jax.ShapeDtypeStruct((NC, NPAD), jnp.float32),
    mesh=_MESH,
    scratch_types=[
        pltpu.VMEM((NPAD,), jnp.float32),
        pltpu.VMEM((NPAD,), jnp.float32),
        pltpu.VMEM((NPAD,), jnp.float32),
        pltpu.VMEM((NPAD,), jnp.float32),
        pltpu.VMEM((NB, EB), jnp.int32),
        pltpu.VMEM((NB, EB), jnp.int32),
        pltpu.VMEM((EPW,), jnp.float32),
        pltpu.VMEM((2, EB), jnp.float32),
        pltpu.VMEM((RPT,), jnp.float32),
        pltpu.VMEM_SHARED((NPAD,), jnp.float32),
        pltpu.SemaphoreType.DMA,
        pltpu.SemaphoreType.DMA,
    ],
    compiler_params=pltpu.CompilerParams(needs_layout_passes=False),
)
def _gat_w_sc(m_hbm, den_hbm, ex_hbm, src_hbm, dst_hbm, out_hbm,
              rr_v, t_v, d0_v, d1_v, src_v, dst_v, ex_v, val_v, row_v,
              acc_sh, ssa, ssb):
  c = lax.axis_index("c")
  s = lax.axis_index("s")
  pltpu.sync_copy(m_hbm.at[0], rr_v)
  pltpu.sync_copy(m_hbm.at[1], t_v)
  pltpu.sync_copy(den_hbm.at[0], d0_v)
  pltpu.sync_copy(den_hbm.at[1], d1_v)
  pltpu.sync_copy(ex_hbm.at[c, s], ex_v)
  pltpu.sync_copy(src_hbm.at[c, s], src_v)
  pltpu.sync_copy(dst_hbm.at[c, s], dst_v)

  def comb(t, _):
    ds = pl.ds(t * L, L)
    m0 = rr_v[ds]
    m1 = t_v[ds]
    mm = jnp.maximum(m0, m1)
    r0 = jnp.exp(m0 - mm)
    r1 = jnp.exp(m1 - mm)
    den = r0 * d0_v[ds] + r1 * d1_v[ds] + 1e-16
    rr_v[ds] = jnp.where(c == 0, r0, r1) / den
    return 0
  lax.fori_loop(0, NPAD // L, comb, 0)

  _fill(row_v, RPT, 0.0)
  pltpu.sync_copy(row_v, acc_sh.at[pl.ds(s * RPT, RPT)])
  plsc.subcore_barrier()

  def fill_val(slot, j):
    for k in range(EB // L):
      di = dst_v[j, pl.ds(k * L, L)]
      ex = ex_v[pl.ds(j * EB + k * L, L)]
      rr = plsc.load_gather(rr_v, [di])
      val_v[slot, pl.ds(k * L, L)] = ex * rr

  def pair(u, _):
    ja = 2 * u
    jb = 2 * u + 1

    @pl.when(u >= 1)
    def _():
      pltpu.make_async_copy(val_v.at[0], acc_sh.at[src_v.at[ja]], ssa).wait()
    fill_val(0, ja)
    pltpu.async_copy(val_v.at[0], acc_sh.at[src_v.at[ja]], ssa, add=True)

    @pl.when(u >= 1)
    def _():
      pltpu.make_async_copy(val_v.at[1], acc_sh.at[src_v.at[jb]], ssb).wait()
    fill_val(1, jb)
    pltpu.async_copy(val_v.at[1], acc_sh.at[src_v.at[jb]], ssb, add=True)
    return 0

  lax.fori_loop(0, NB // 2, pair, 0)
  pltpu.make_async_copy(val_v.at[0], acc_sh.at[src_v.at[NB - 2]], ssa).wait()
  pltpu.make_async_copy(val_v.at[1], acc_sh.at[src_v.at[NB - 1]], ssb).wait()

  plsc.subcore_barrier()
  sl = pl.ds(s * RPT, RPT)
  pltpu.sync_copy(acc_sh.at[sl], row_v)
  pltpu.sync_copy(row_v, out_hbm.at[c, sl])


def _dis_of(deg2):
  deg = deg2[0] + deg2[1] + 1.0
  return jnp.where(deg > 0, lax.rsqrt(jnp.maximum(deg, 1e-12)), 0.0)


def _t01_body(x_ref, wemb_ref, bemb_ref, w1_ref, deg_ref, o_ref):
  h0 = jax.nn.relu(
      jnp.dot(x_ref[...], wemb_ref[...], preferred_element_type=jnp.float32)
      + bemb_ref[...])
  dis = _dis_of(deg_ref[...])
  o_ref[...] = dis[:, None] * jnp.dot(
      h0, w1_ref[...], preferred_element_type=jnp.float32)


def _t2_body(q_ref, p_ref, deg_ref, b_ref, w_ref, o_ref):
  dis = _dis_of(deg_ref[...])
  h = jax.nn.relu(
      dis[:, None] * (q_ref[0] + q_ref[1] + p_ref[...]) + b_ref[...])
  o_ref[...] = dis[:, None] * jnp.dot(
      h, w_ref[...], preferred_element_type=jnp.float32)


def _t3_body(q_ref, p_ref, deg_ref, b_ref, wg_ref, asrc_ref, adst_ref,
             hg_ref, s_ref, d_ref):
  dis = _dis_of(deg_ref[...])
  h = jax.nn.relu(
      dis[:, None] * (q_ref[0] + q_ref[1] + p_ref[...]) + b_ref[...])
  hg = jnp.dot(h, wg_ref[...], preferred_element_type=jnp.float32)
  hg_ref[...] = hg
  s_ref[...] = jnp.dot(hg, asrc_ref[...],
                       preferred_element_type=jnp.float32).T
  d_ref[...] = jnp.dot(hg, adst_ref[...],
                       preferred_element_type=jnp.float32).T


def _t4_body(w_ref, hg_ref, bg_ref, o_ref):
  i = pl.program_id(0)
  rows = i * BLK + lax.broadcasted_iota(jnp.int32, (BLK,), 0)
  w = jnp.where(rows < N, w_ref[0] + w_ref[1], 0.0)
  part = jnp.dot(w[None, :], hg_ref[...], preferred_element_type=jnp.float32)

  @pl.when(i == 0)
  def _():
    o_ref[...] = jnp.zeros_like(o_ref)
  o_ref[...] += part

  @pl.when(i == GRID - 1)
  def _():
    o_ref[...] = o_ref[...] * (1.0 / N) + bg_ref[...]


def _row_spec(block=None):
  return pl.BlockSpec((BLK, D) if block is None else block, lambda i: (0, 0))


_SPEC_ROWS = pl.BlockSpec((BLK, D), lambda i: (i, 0))
_SPEC_W = pl.BlockSpec((D, D), lambda i: (0, 0))
_SPEC_B = pl.BlockSpec((1, D), lambda i: (0, 0))
_SPEC_DEG = pl.BlockSpec((NC, BLK), lambda i: (0, i))
_SPEC_Q = pl.BlockSpec((NC, BLK, D), lambda i: (0, i, 0))
_SPEC_VEC = pl.BlockSpec((D, 1), lambda i: (0, 0))
_SPEC_SD = pl.BlockSpec((1, BLK), lambda i: (0, i))


def _t01(xp, W_emb, b_emb, W1, deg2):
  return pl.pallas_call(
      _t01_body, grid=(GRID,),
      in_specs=[_SPEC_ROWS, _SPEC_W, _SPEC_B, _SPEC_W, _SPEC_DEG],
      out_specs=_SPEC_ROWS,
      out_shape=jax.ShapeDtypeStruct((NPAD, D), jnp.float32),
  )(xp, W_emb, b_emb, W1, deg2)


def _t2(q2, p, deg2, b, W):
  return pl.pallas_call(
      _t2_body, grid=(GRID,),
      in_specs=[_SPEC_Q, _SPEC_ROWS, _SPEC_DEG, _SPEC_B, _SPEC_W],
      out_specs=_SPEC_ROWS,
      out_shape=jax.ShapeDtypeStruct((NPAD, D), jnp.float32),
  )(q2, p, deg2, b, W)


def _t3(q2, p, deg2, b, Wg, a_src, a_dst):
  return pl.pallas_call(
      _t3_body, grid=(GRID,),
      in_specs=[_SPEC_Q, _SPEC_ROWS, _SPEC_DEG, _SPEC_B, _SPEC_W,
                _SPEC_VEC, _SPEC_VEC],
      out_specs=[_SPEC_ROWS, _SPEC_SD, _SPEC_SD],
      out_shape=[
          jax.ShapeDtypeStruct((NPAD, D), jnp.float32),
          jax.ShapeDtypeStruct((1, NPAD), jnp.float32),
          jax.ShapeDtypeStruct((1, NPAD), jnp.float32),
      ],
  )(q2, p, deg2, b, Wg, a_src, a_dst)


def _t4(w2, hg, bg):
  return pl.pallas_call(
      _t4_body, grid=(GRID,),
      in_specs=[_SPEC_DEG, _SPEC_ROWS, _SPEC_B],
      out_specs=_SPEC_B,
      out_shape=jax.ShapeDtypeStruct((1, D), jnp.float32),
  )(w2, hg, bg)


def kernel(x, edge_index, W_emb, b_emb, W1, b1, W2, b2, Wg, a_src, a_dst, bg):
  ei = edge_index.astype(jnp.int32)
  loop = jnp.arange(N, dtype=jnp.int32)

  e_real = ei.shape[1]
  pada = N + 200 + (jnp.arange(ETOT_S - e_real, dtype=jnp.int32) % 8)
  padb = N + 200 + (jnp.arange(ETOT - e_real - N, dtype=jnp.int32) % 8)
  src5 = jnp.concatenate([ei[0], pada]).reshape(NC, NS, NBT, SB)
  dst5 = jnp.concatenate([ei[1], pada]).reshape(NC, NS, NBT, SB)
  src4 = jnp.concatenate([ei[0], loop, padb]).reshape(NC, NS, NB, EB)
  dst4 = jnp.concatenate([ei[1], loop, padb]).reshape(NC, NS, NB, EB)
  xp = jnp.zeros((NPAD, D), jnp.float32).at[:N].set(x)
  b_emb2 = b_emb.reshape(1, D)
  b12 = b1.reshape(1, D)
  b22 = b2.reshape(1, D)
  bg2 = bg.reshape(1, D)
  a_src2 = a_src.reshape(D, 1)
  a_dst2 = a_dst.reshape(D, 1)

  deg2 = _deg_sc(dst5)
  p1 = _t01(xp, W_emb, b_emb2, W1, deg2)
  q1 = _spmm_sc(p1, src5, dst5)
  p2 = _t2(q1, p1, deg2, b12, W2)
  q2 = _spmm_sc(p2, src5, dst5)
  hg, s2, d2 = _t3(q2, p2, deg2, b22, Wg, a_src2, a_dst2)
  s1 = s2.reshape(NPAD)
  d1 = d2.reshape(NPAD)
  m2, den2, ex3 = _gat_maxdenom_sc(s1, d1, src4, dst4)
  w2 = _gat_w_sc(m2, den2, ex3, src4, dst4)
  return _t4(w2, hg, bg2)


# speedup vs baseline: 54.8854x; 1.0033x over previous
import functools

import jax
import jax.numpy as jnp
from jax import lax
from jax.experimental import pallas as pl
from jax.experimental.pallas import tpu as pltpu
from jax.experimental.pallas import tpu_sc as plsc

N = 10000
D = 128
NPAD = 10240
NC = 2
NS = 16
L = 16
EB = 128
NB = 82
SB = 96
NBD = 125
SBD = 80
NBT = 105
ETOT_S = NC * NS * NBT * SB
EPW = NB * EB
ETOT = NC * NS * EPW
RPT = NPAD // NS
BLK = 2048
GRID = NPAD // BLK
NEG = -1e30

_MESH = plsc.VectorSubcoreMesh(
    core_axis_name="c", subcore_axis_name="s", num_cores=NC, num_subcores=NS)


def _fill(ref, n, value):
  vec = jnp.full((L,), value, ref.dtype)
  def body(i, _):
    ref[pl.ds(i * L, L)] = vec
    return 0
  lax.fori_loop(0, n // L, body, 0)


def _zero_shared_slice(acc_sh, zbuf, s):
  nz = zbuf.shape[0]
  for i in range(RPT // nz):
    pltpu.sync_copy(zbuf, acc_sh.at[pl.ds(s * RPT + i * nz, nz)])
  rem = RPT % nz
  if rem:
    pltpu.sync_copy(zbuf.at[pl.ds(0, rem)],
                    acc_sh.at[pl.ds(s * RPT + (RPT // nz) * nz, rem)])


def _copyout_shared_slice(acc_sh, out_hbm, bounce, c, s):
  nz = bounce.shape[0]
  for i in range(RPT // nz):
    sl = pl.ds(s * RPT + i * nz, nz)
    pltpu.sync_copy(acc_sh.at[sl], bounce)
    pltpu.sync_copy(bounce, out_hbm.at[c, sl])
  rem = RPT % nz
  if rem:
    sl = pl.ds(s * RPT + (RPT // nz) * nz, rem)
    pltpu.sync_copy(acc_sh.at[sl], bounce.at[pl.ds(0, rem)])
    pltpu.sync_copy(bounce.at[pl.ds(0, rem)], out_hbm.at[c, sl])


@functools.partial(
    pl.kernel,
    out_type=jax.ShapeDtypeStruct((NC, NPAD), jnp.float32),
    mesh=_MESH,
    scratch_types=[
        pltpu.VMEM((NBD, SBD), jnp.int32),
        pltpu.VMEM((SBD,), jnp.float32),
        pltpu.VMEM((RPT,), jnp.float32),
        pltpu.VMEM_SHARED((NPAD,), jnp.float32),
        pltpu.SemaphoreType.DMA,
    ],
)
def _deg_sc(dst_hbm, out_hbm, idx_v, ones_v, row_v, acc_sh, sadd):
  c = lax.axis_index("c")
  s = lax.axis_index("s")
  _fill(ones_v, SBD, 1.0)
  _fill(row_v, RPT, 0.0)
  pltpu.sync_copy(row_v, acc_sh.at[pl.ds(s * RPT, RPT)])
  pltpu.sync_copy(dst_hbm.at[c, s], idx_v)
  plsc.subcore_barrier()

  def body(j, _):
    pltpu.async_copy(ones_v, acc_sh.at[idx_v.at[j]], sadd, add=True)
    return 0
  lax.fori_loop(0, NBD, body, 0)

  def drain(j, _):
    pltpu.make_async_copy(ones_v, acc_sh.at[idx_v.at[j]], sadd).wait()
    return 0
  lax.fori_loop(0, NBD, drain, 0)

  plsc.subcore_barrier()
  sl = pl.ds(s * RPT, RPT)
  pltpu.sync_copy(acc_sh.at[sl], row_v)
  pltpu.sync_copy(row_v, out_hbm.at[c, sl])


@functools.partial(
    pl.kernel,
    out_type=jax.ShapeDtypeStruct((NC, NPAD, D), jnp.float32),
    mesh=_MESH,
    scratch_types=[
        pltpu.VMEM((4, SB), jnp.int32),
        pltpu.VMEM((4, SB), jnp.int32),
        pltpu.VMEM((SB, D), jnp.float32),
        pltpu.VMEM((SB, D), jnp.float32),
        pltpu.VMEM((SB, D), jnp.float32),
        pltpu.VMEM_SHARED((NPAD, D), jnp.float32),
        pltpu.SemaphoreType.DMA,
        pltpu.SemaphoreType.DMA,
        pltpu.SemaphoreType.DMA,
        pltpu.SemaphoreType.DMA,
        pltpu.SemaphoreType.DMA,
        pltpu.SemaphoreType.DMA,
        pltpu.SemaphoreType.DMA,
    ],
)
def _spmm_sc(p_hbm, src_hbm, dst_hbm, out_hbm,
             srcb, dstb, buf0, buf1, buf2, acc_sh,
             sg0, sg1, sg2, ss0, ss1, ss2, sidx):
  c = lax.axis_index("c")
  s = lax.axis_index("s")
  bufs = (buf0, buf1, buf2)
  sgs = (sg0, sg1, sg2)
  sss = (ss0, ss1, ss2)
  zvec = jnp.zeros((L,), jnp.float32)
  def zrow(r, _):
    for k in range(D // L):
      buf0[r, pl.ds(k * L, L)] = zvec
    return 0
  lax.fori_loop(0, SB, zrow, 0)
  _zero_shared_slice(acc_sh, buf0, s)
  plsc.subcore_barrier()

  pltpu.sync_copy(src_hbm.at[c, s, 0], srcb.at[0])
  pltpu.sync_copy(dst_hbm.at[c, s, 0], dstb.at[0])
  pltpu.sync_copy(src_hbm.at[c, s, 1], srcb.at[1])
  pltpu.sync_copy(dst_hbm.at[c, s, 1], dstb.at[1])
  pltpu.async_copy(src_hbm.at[c, s, 2], srcb.at[2], sidx)
  pltpu.async_copy(dst_hbm.at[c, s, 2], dstb.at[2], sidx)
  pltpu.async_copy(p_hbm.at[srcb.at[0]], buf0, sg0)
  pltpu.async_copy(p_hbm.at[srcb.at[1]], buf1, sg1)

  def stage(t, b):
    bp = (b + 2) % 3
    i0 = lax.rem(t, 4)
    i2 = lax.rem(t + 2, 4)
    i3 = lax.rem(t + 3, 4)
    pltpu.make_async_copy(p_hbm.at[srcb.at[i0]], bufs[b], sgs[b]).wait()
    pltpu.async_copy(bufs[b], acc_sh.at[dstb.at[i0]], sss[b], add=True)

    @pl.when(t >= 1)
    def _():
      pltpu.make_async_copy(bufs[bp], acc_sh.at[dstb.at[i3]], sss[bp]).wait()

    @pl.when(t + 2 < NBT)
    def _():
      pltpu.make_async_copy(src_hbm.at[c, s, t + 2], srcb.at[i2], sidx).wait()
      pltpu.make_async_copy(dst_hbm.at[c, s, t + 2], dstb.at[i2], sidx).wait()

      @pl.when(t + 3 < NBT)
      def _():
        pltpu.async_copy(src_hbm.at[c, s, t + 3], srcb.at[i3], sidx)
        pltpu.async_copy(dst_hbm.at[c, s, t + 3], dstb.at[i3], sidx)
      pltpu.async_copy(p_hbm.at[srcb.at[i2]], bufs[bp], sgs[bp])

  def body(u, _):
    stage(3 * u, 0)
    stage(3 * u + 1, 1)
    stage(3 * u + 2, 2)
    return 0

  assert NBT % 3 == 0
  lax.fori_loop(0, NBT // 3, body, 0)

  lb = (NBT - 1) % 3
  pltpu.make_async_copy(
      bufs[lb], acc_sh.at[dstb.at[(NBT - 1) % 4]], sss[lb]).wait()

  plsc.subcore_barrier()
  _copyout_shared_slice(acc_sh, out_hbm, buf0, c, s)


def _edge_e(s_v, d_v, src_v, dst_v, j, k):
  si = src_v[j, pl.ds(k * L, L)]
  di = dst_v[j, pl.ds(k * L, L)]
  sv = plsc.load_gather(s_v, [si])
  dv = plsc.load_gather(d_v, [di])
  z = sv + dv
  return di, jnp.where(z >= 0, z, 0.2 * z)


@functools.partial(
    pl.kernel,
    out_type=[
        jax.ShapeDtypeStruct((NC, NPAD), jnp.float32),
        jax.ShapeDtypeStruct((NC, NPAD), jnp.float32),
        jax.ShapeDtypeStruct((NC, NS, EPW), jnp.float32),
    ],
    mesh=_MESH,
    scratch_types=[
        pltpu.VMEM((NPAD,), jnp.float32),
        pltpu.VMEM((NPAD,), jnp.float32),
        pltpu.VMEM((NPAD,), jnp.float32),
        pltpu.VMEM((NB, EB), jnp.int32),
        pltpu.VMEM((NB, EB), jnp.int32),
        pltpu.VMEM((RPT,), jnp.float32),
        pltpu.VMEM((RPT,), jnp.float32),
        pltpu.VMEM((2, EB), jnp.float32),
        pltpu.VMEM((EPW,), jnp.float32),
        pltpu.VMEM_SHARED((NS, NPAD), jnp.float32),
        pltpu.VMEM_SHARED((NPAD,), jnp.float32),
        pltpu.SemaphoreType.DMA,
        pltpu.SemaphoreType.DMA,
        pltpu.SemaphoreType.DMA,
        pltpu.SemaphoreType.DMA,
    ],
    compiler_params=pltpu.CompilerParams(needs_layout_passes=False),
)
def _gat_maxdenom_sc(s_hbm, d_hbm, src_hbm, dst_hbm, m_hbm, den_hbm, ex_hbm,
                     s_v, d_v, m_v, src_v, dst_v, acc_v, tmp_v, val_v,
                     e_v, stage_sh, den_sh, ssa, ssb, sea, seb):
  c = lax.axis_index("c")
  s = lax.axis_index("s")
  pltpu.sync_copy(s_hbm, s_v)
  pltpu.sync_copy(d_hbm, d_v)
  pltpu.sync_copy(src_hbm.at[c, s], src_v)
  pltpu.sync_copy(dst_hbm.at[c, s], dst_v)
  _fill(m_v, NPAD, NEG)

  def batch(j, _):
    for k in range(EB // L):
      di, e = _edge_e(s_v, d_v, src_v, dst_v, j, k)
      e_v[pl.ds(j * EB + k * L, L)] = e
      cur = plsc.load_gather(m_v, [di])
      need = e > cur

      def w_body(need):
        plsc.store_scatter(m_v, [di], e, mask=need)
        cur = plsc.load_gather(m_v, [di])
        return e > cur

      lax.while_loop(jnp.any, w_body, need)
    return 0

  lax.fori_loop(0, NB, batch, 0)

  pltpu.sync_copy(m_v, stage_sh.at[s])
  _fill(tmp_v, RPT, 0.0)
  sl = pl.ds(s * RPT, RPT)
  pltpu.sync_copy(tmp_v, den_sh.at[sl])
  plsc.subcore_barrier()
  pltpu.sync_copy(stage_sh.at[0, sl], acc_v)
  for i in range(1, NS):
    pltpu.sync_copy(stage_sh.at[i, sl], tmp_v)
    def red(t, _):
      ds = pl.ds(t * L, L)
      acc_v[ds] = jnp.maximum(acc_v[ds], tmp_v[ds])
      return 0
    lax.fori_loop(0, RPT // L, red, 0)
  pltpu.sync_copy(acc_v, m_hbm.at[c, sl])
  pltpu.sync_copy(acc_v, stage_sh.at[0, sl])
  plsc.subcore_barrier()
  pltpu.sync_copy(stage_sh.at[0], m_v)

  def fill_val(slot, j):
    for k in range(EB // L):
      di = dst_v[j, pl.ds(k * L, L)]
      e = e_v[pl.ds(j * EB + k * L, L)]
      mv = plsc.load_gather(m_v, [di])
      val_v[slot, pl.ds(k * L, L)] = jnp.exp(e - mv)

  def pair(u, _):
    ja = 2 * u
    jb = 2 * u + 1

    @pl.when(u >= 1)
    def _():
      pltpu.make_async_copy(val_v.at[0], den_sh.at[dst_v.at[ja]], ssa).wait()
      pltpu.make_async_copy(
          val_v.at[0], ex_hbm.at[c, s, pl.ds(ja * EB, EB)], sea).wait()
    fill_val(0, ja)
    pltpu.async_copy(val_v.at[0], den_sh.at[dst_v.at[ja]], ssa, add=True)
    pltpu.async_copy(val_v.at[0], ex_hbm.at[c, s, pl.ds(ja * EB, EB)], sea)

    @pl.when(u >= 1)
    def _():
      pltpu.make_async_copy(val_v.at[1], den_sh.at[dst_v.at[jb]], ssb).wait()
      pltpu.make_async_copy(
          val_v.at[1], ex_hbm.at[c, s, pl.ds(jb * EB, EB)], seb).wait()
    fill_val(1, jb)
    pltpu.async_copy(val_v.at[1], den_sh.at[dst_v.at[jb]], ssb, add=True)
    pltpu.async_copy(val_v.at[1], ex_hbm.at[c, s, pl.ds(jb * EB, EB)], seb)
    return 0

  lax.fori_loop(0, NB // 2, pair, 0)
  pltpu.make_async_copy(val_v.at[0], den_sh.at[dst_v.at[NB - 2]], ssa).wait()
  pltpu.make_async_copy(val_v.at[1], den_sh.at[dst_v.at[NB - 1]], ssb).wait()
  pltpu.make_async_copy(
      val_v.at[0], ex_hbm.at[c, s, pl.ds((NB - 2) * EB, EB)], sea).wait()
  pltpu.make_async_copy(
      val_v.at[1], ex_hbm.at[c, s, pl.ds((NB - 1) * EB, EB)], seb).wait()

  plsc.subcore_barrier()
  pltpu.sync_copy(den_sh.at[sl], acc_v)
  pltpu.sync_copy(acc_v, den_hbm.at[c, sl])


@functools.partial(
    pl.kernel,
    out_type=jax.ShapeDtypeStruct((NC, NPAD), jnp.float32),
    mesh=_MESH,
    scratch_types=[
        pltpu.VMEM((NPAD,), jnp.float32),
        pltpu.VMEM((NPAD,), jnp.float32),
        pltpu.VMEM((NPAD,), jnp.float32),
        pltpu.VMEM((NPAD,), jnp.float32),
        pltpu.VMEM((NB, EB), jnp.int32),
        pltpu.VMEM((NB, EB), jnp.int32),
        pltpu.VMEM((EPW,), jnp.float32),
        pltpu.VMEM((2, EB), jnp.float32),
        pltpu.VMEM((RPT,), jnp.float32),
        pltpu.VMEM_SHARED((NPAD,), jnp.float32),
        pltpu.SemaphoreType.DMA,
        pltpu.SemaphoreType.DMA,
    ],
    compiler_params=pltpu.CompilerParams(needs_layout_passes=False),
)
def _gat_w_sc(m_hbm, den_hbm, ex_hbm, src_hbm, dst_hbm, out_hbm,
              rr_v, t_v, d0_v, d1_v, src_v, dst_v, ex_v, val_v, row_v,
              acc_sh, ssa, ssb):
  c = lax.axis_index("c")
  s = lax.axis_index("s")
  pltpu.sync_copy(m_hbm.at[0], rr_v)
  pltpu.sync_copy(m_hbm.at[1], t_v)
  pltpu.sync_copy(den_hbm.at[0], d0_v)
  pltpu.sync_copy(den_hbm.at[1], d1_v)
  pltpu.sync_copy(ex_hbm.at[c, s], ex_v)
  pltpu.sync_copy(src_hbm.at[c, s], src_v)
  pltpu.sync_copy(dst_hbm.at[c, s], dst_v)

  def comb(t, _):
    ds = pl.ds(t * L, L)
    m0 = rr_v[ds]
    m1 = t_v[ds]
    mm = jnp.maximum(m0, m1)
    r0 = jnp.exp(m0 - mm)
    r1 = jnp.exp(m1 - mm)
    den = r0 * d0_v[ds] + r1 * d1_v[ds] + 1e-16
    rr_v[ds] = jnp.where(c == 0, r0, r1) / den
    return 0
  lax.fori_loop(0, NPAD // L, comb, 0)

  _fill(row_v, RPT, 0.0)
  pltpu.sync_copy(row_v, acc_sh.at[pl.ds(s * RPT, RPT)])
  plsc.subcore_barrier()

  def fill_val(slot, j):
    for k in range(EB // L):
      di = dst_v[j, pl.ds(k * L, L)]
      ex = ex_v[pl.ds(j * EB + k * L, L)]
      rr = plsc.load_gather(rr_v, [di])
      val_v[slot, pl.ds(k * L, L)] = ex * rr

  def pair(u, _):
    ja = 2 * u
    jb = 2 * u + 1

    @pl.when(u >= 1)
    def _():
      pltpu.make_async_copy(val_v.at[0], acc_sh.at[src_v.at[ja]], ssa).wait()
    fill_val(0, ja)
    pltpu.async_copy(val_v.at[0], acc_sh.at[src_v.at[ja]], ssa, add=True)

    @pl.when(u >= 1)
    def _():
      pltpu.make_async_copy(val_v.at[1], acc_sh.at[src_v.at[jb]], ssb).wait()
    fill_val(1, jb)
    pltpu.async_copy(val_v.at[1], acc_sh.at[src_v.at[jb]], ssb, add=True)
    return 0

  lax.fori_loop(0, NB // 2, pair, 0)
  pltpu.make_async_copy(val_v.at[0], acc_sh.at[src_v.at[NB - 2]], ssa).wait()
  pltpu.make_async_copy(val_v.at[1], acc_sh.at[src_v.at[NB - 1]], ssb).wait()

  plsc.subcore_barrier()
  sl = pl.ds(s * RPT, RPT)
  pltpu.sync_copy(acc_sh.at[sl], row_v)
  pltpu.sync_copy(row_v, out_hbm.at[c, sl])


def _dis_of(deg2):
  deg = deg2[0] + deg2[1] + 1.0
  return jnp.where(deg > 0, lax.rsqrt(jnp.maximum(deg, 1e-12)), 0.0)


def _t01_body(x_ref, wemb_ref, bemb_ref, w1_ref, deg_ref, o_ref):
  h0 = jax.nn.relu(
      jnp.dot(x_ref[...], wemb_ref[...], preferred_element_type=jnp.float32)
      + bemb_ref[...])
  dis = _dis_of(deg_ref[...])
  o_ref[...] = dis[:, None] * jnp.dot(
      h0, w1_ref[...], preferred_element_type=jnp.float32)


def _t2_body(q_ref, p_ref, deg_ref, b_ref, w_ref, o_ref):
  dis = _dis_of(deg_ref[...])
  h = jax.nn.relu(
      dis[:, None] * (q_ref[0] + q_ref[1] + p_ref[...]) + b_ref[...])
  o_ref[...] = dis[:, None] * jnp.dot(
      h, w_ref[...], preferred_element_type=jnp.float32)


def _t3_body(q_ref, p_ref, deg_ref, b_ref, wg_ref, asrc_ref, adst_ref,
             hg_ref, s_ref, d_ref):
  dis = _dis_of(deg_ref[...])
  h = jax.nn.relu(
      dis[:, None] * (q_ref[0] + q_ref[1] + p_ref[...]) + b_ref[...])
  hg = jnp.dot(h, wg_ref[...], preferred_element_type=jnp.float32)
  hg_ref[...] = hg
  s_ref[...] = jnp.dot(hg, asrc_ref[...],
                       preferred_element_type=jnp.float32).T
  d_ref[...] = jnp.dot(hg, adst_ref[...],
                       preferred_element_type=jnp.float32).T


def _t4_body(w_ref, hg_ref, bg_ref, o_ref):
  i = pl.program_id(0)
  rows = i * BLK + lax.broadcasted_iota(jnp.int32, (BLK,), 0)
  w = jnp.where(rows < N, w_ref[0] + w_ref[1], 0.0)
  part = jnp.dot(w[None, :], hg_ref[...], preferred_element_type=jnp.float32)

  @pl.when(i == 0)
  def _():
    o_ref[...] = jnp.zeros_like(o_ref)
  o_ref[...] += part

  @pl.when(i == GRID - 1)
  def _():
    o_ref[...] = o_ref[...] * (1.0 / N) + bg_ref[...]


def _row_spec(block=None):
  return pl.BlockSpec((BLK, D) if block is None else block, lambda i: (0, 0))


_SPEC_ROWS = pl.BlockSpec((BLK, D), lambda i: (i, 0))
_SPEC_W = pl.BlockSpec((D, D), lambda i: (0, 0))
_SPEC_B = pl.BlockSpec((1, D), lambda i: (0, 0))
_SPEC_DEG = pl.BlockSpec((NC, BLK), lambda i: (0, i))
_SPEC_Q = pl.BlockSpec((NC, BLK, D), lambda i: (0, i, 0))
_SPEC_VEC = pl.BlockSpec((D, 1), lambda i: (0, 0))
_SPEC_SD = pl.BlockSpec((1, BLK), lambda i: (0, i))


def _t01(xp, W_emb, b_emb, W1, deg2):
  return pl.pallas_call(
      _t01_body, grid=(GRID,),
      in_specs=[_SPEC_ROWS, _SPEC_W, _SPEC_B, _SPEC_W, _SPEC_DEG],
      out_specs=_SPEC_ROWS,
      out_shape=jax.ShapeDtypeStruct((NPAD, D), jnp.float32),
  )(xp, W_emb, b_emb, W1, deg2)


def _t2(q2, p, deg2, b, W):
  return pl.pallas_call(
      _t2_body, grid=(GRID,),
      in_specs=[_SPEC_Q, _SPEC_ROWS, _SPEC_DEG, _SPEC_B, _SPEC_W],
      out_specs=_SPEC_ROWS,
      out_shape=jax.ShapeDtypeStruct((NPAD, D), jnp.float32),
  )(q2, p, deg2, b, W)


def _t3(q2, p, deg2, b, Wg, a_src, a_dst):
  return pl.pallas_call(
      _t3_body, grid=(GRID,),
      in_specs=[_SPEC_Q, _SPEC_ROWS, _SPEC_DEG, _SPEC_B, _SPEC_W,
                _SPEC_VEC, _SPEC_VEC],
      out_specs=[_SPEC_ROWS, _SPEC_SD, _SPEC_SD],
      out_shape=[
          jax.ShapeDtypeStruct((NPAD, D), jnp.float32),
          jax.ShapeDtypeStruct((1, NPAD), jnp.float32),
          jax.ShapeDtypeStruct((1, NPAD), jnp.float32),
      ],
  )(q2, p, deg2, b, Wg, a_src, a_dst)


def _t4(w2, hg, bg):
  return pl.pallas_call(
      _t4_body, grid=(GRID,),
      in_specs=[_SPEC_DEG, _SPEC_ROWS, _SPEC_B],
      out_specs=_SPEC_B,
      out_shape=jax.ShapeDtypeStruct((1, D), jnp.float32),
  )(w2, hg, bg)


def kernel(x, edge_index, W_emb, b_emb, W1, b1, W2, b2, Wg, a_src, a_dst, bg):
  ei = edge_index.astype(jnp.int32)
  loop = jnp.arange(N, dtype=jnp.int32)

  e_real = ei.shape[1]
  pada = N + 200 + (jnp.arange(ETOT_S - e_real, dtype=jnp.int32) % 8)
  padb = N + 200 + (jnp.arange(ETOT - e_real - N, dtype=jnp.int32) % 8)
  src5 = jnp.concatenate([ei[0], pada]).reshape(NC, NS, NBT, SB)
  dst5 = jnp.concatenate([ei[1], pada]).reshape(NC, NS, NBT, SB)
  src4 = jnp.concatenate([ei[0], loop, padb]).reshape(NC, NS, NB, EB)
  dst4 = jnp.concatenate([ei[1], loop, padb]).reshape(NC, NS, NB, EB)
  xp = jnp.zeros((NPAD, D), jnp.float32).at[:N].set(x)
  b_emb2 = b_emb.reshape(1, D)
  b12 = b1.reshape(1, D)
  b22 = b2.reshape(1, D)
  bg2 = bg.reshape(1, D)
  a_src2 = a_src.reshape(D, 1)
  a_dst2 = a_dst.reshape(D, 1)

  dstdeg = ei[1].reshape(NC, NS, NBD, SBD)
  deg2 = _deg_sc(dstdeg)
  p1 = _t01(xp, W_emb, b_emb2, W1, deg2)
  q1 = _spmm_sc(p1, src5, dst5)
  p2 = _t2(q1, p1, deg2, b12, W2)
  q2 = _spmm_sc(p2, src5, dst5)
  hg, s2, d2 = _t3(q2, p2, deg2, b22, Wg, a_src2, a_dst2)
  s1 = s2.reshape(NPAD)
  d1 = d2.reshape(NPAD)
  m2, den2, ex3 = _gat_maxdenom_sc(s1, d1, src4, dst4)
  w2 = _gat_w_sc(m2, den2, ex3, src4, dst4)
  return _t4(w2, hg, bg2)
